# speedup vs baseline: 1.7077x; 1.7077x over previous
import functools

import jax
import jax.numpy as jnp
from jax.experimental import pallas as pl
from jax.experimental.pallas import tpu as pltpu

N = 10000
NSP = 500
C = 256
HIDE = 256
NCLS = 16
EM = 160000
EA = 8000
HEAD = 2
DH = HIDE // HEAD
NSF = 5
NPF = 2
RHP = 5
GAMA = 0.9

ROWS_BLK = 2000


def _prelin_proj_body(x_ref, w1_ref, b1_ref, wcat_ref, bcat_ref,
                      h0_ref, qcat_ref, vcat_ref):
    x = x_ref[...]
    h0 = jnp.dot(x, w1_ref[...], preferred_element_type=jnp.float32) + b1_ref[...]
    h0_ref[...] = h0
    cat = jnp.dot(h0, wcat_ref[...], preferred_element_type=jnp.float32) + bcat_ref[...]
    q = cat[:, :HIDE]
    v = cat[:, HIDE:]
    q2 = q.reshape(q.shape[0], HEAD, DH)
    m = jnp.mean(q2, axis=-1, keepdims=True)
    var = jnp.mean((q2 - m) ** 2, axis=-1, keepdims=True)
    qn = ((q2 - m) / jnp.sqrt(var + 1e-5)).reshape(q.shape)
    qcat_ref[...] = qn
    vcat_ref[...] = v


def _prelin_proj(x, w1t, b1, wcatt, bcat):
    grid = (N // ROWS_BLK,)
    return pl.pallas_call(
        _prelin_proj_body,
        grid=grid,
        in_specs=[
            pl.BlockSpec((ROWS_BLK, C), lambda i: (i, 0)),
            pl.BlockSpec((C, HIDE), lambda i: (0, 0)),
            pl.BlockSpec((1, HIDE), lambda i: (0, 0)),
            pl.BlockSpec((HIDE, 2 * HIDE), lambda i: (0, 0)),
            pl.BlockSpec((1, 2 * HIDE), lambda i: (0, 0)),
        ],
        out_specs=[
            pl.BlockSpec((ROWS_BLK, HIDE), lambda i: (i, 0)),
            pl.BlockSpec((ROWS_BLK, HIDE), lambda i: (i, 0)),
            pl.BlockSpec((ROWS_BLK, HIDE), lambda i: (i, 0)),
        ],
        out_shape=[
            jax.ShapeDtypeStruct((N, HIDE), jnp.float32),
            jax.ShapeDtypeStruct((N, HIDE), jnp.float32),
            jax.ShapeDtypeStruct((N, HIDE), jnp.float32),
        ],
    )(x, w1t, b1, wcatt, bcat)


def _leaky(x):
    return jnp.where(x >= 0, x, 0.01 * x)


def kernel(x, mask_index, mask_value, q_assign, a_index, a_value,
           prelin_W, prelin_b, bn0_g, bn0_b,
           sf_W, sf_b, sf_g, sf_be,
           pf_Wv, pf_bv, pf_Wq, pf_bq, pf_g, pf_be,
           out_W, out_b):
    bnscale = 1.0 / jnp.sqrt(1.0 + 1e-5)
    w1 = prelin_W * (bn0_g * bnscale)[:, None]
    b1 = prelin_b * (bn0_g * bnscale) + bn0_b

    src = mask_index[0]
    dst = mask_index[1]

    h = None
    for j in range(NPF):
        wcat = jnp.concatenate([pf_Wq[j, 0], pf_Wq[j, 1], pf_Wv[j, 0], pf_Wv[j, 1]], axis=0)
        bcat = jnp.concatenate([pf_bq[j, 0], pf_bq[j, 1], pf_bv[j, 0], pf_bv[j, 1]], axis=0)
        if j == 0:
            h0, qcat, vcat = _prelin_proj(x, w1.T, b1[None, :], wcat.T, bcat[None, :])
            x_pre = h0
        else:
            cat = h @ wcat.T + bcat
            q2 = cat[:, :HIDE].reshape(N, HEAD, DH)
            m = jnp.mean(q2, axis=-1, keepdims=True)
            var = jnp.mean((q2 - m) ** 2, axis=-1, keepdims=True)
            qcat = ((q2 - m) / jnp.sqrt(var + 1e-5)).reshape(N, HIDE)
            vcat = cat[:, HIDE:]
        heads = []
        for hh in range(HEAD):
            qh = qcat[:, hh * DH:(hh + 1) * DH]
            vh = vcat[:, hh * DH:(hh + 1) * DH]
            att = jnp.sum(qh[src] * qh[dst], axis=1) / float(HIDE)
            e = jnp.exp(att)
            s = jax.ops.segment_sum(e, src, num_segments=N)
            w = e / s[src]
            agg = vh[dst] * w[:, None]
            heads.append(jax.ops.segment_sum(agg, src, num_segments=N))
        out = jnp.concatenate(heads, axis=1)
        h = _leaky(out * (pf_g[j] * bnscale) + pf_be[j])
    pf_out = h

    deg = jax.ops.segment_sum(jnp.ones((N,), jnp.float32), q_assign, num_segments=NSP)
    inv = jnp.where(deg > 0, 1.0 / deg, 0.0)
    hp = jax.ops.segment_sum(x_pre, q_assign, num_segments=NSP) * inv[:, None]
    A = jnp.zeros((NSP, NSP), jnp.float32).at[a_index[1], a_index[0]].add(a_value)
    for i in range(NSF):
        hlin = hp @ sf_W[i].T + sf_b[i]
        x_start = A @ hlin
        xc = hp
        for _ in range(RHP):
            xc = (A @ xc + x_start) / (2.0 + GAMA)
        hp = _leaky(xc * (sf_g[i] * bnscale) + sf_be[i])
    sf_out = hp[q_assign]

    H1 = pf_out + sf_out
    Y = jax.nn.softmax(H1 @ out_W.T + out_b, axis=-1)
    return Y


# speedup vs baseline: 2.6040x; 1.5248x over previous
import functools

import jax
import jax.numpy as jnp
from jax import lax
from jax.experimental import pallas as pl
from jax.experimental.pallas import tpu as pltpu
from jax.experimental.pallas import tpu_sc as plsc

N = 10000
NSP = 500
C = 256
HIDE = 256
NCLS = 16
EM = 160000
EA = 8000
HEAD = 2
DH = HIDE // HEAD
NSF = 5
NPF = 2
RHP = 5
GAMA = 0.9

ROWS_BLK = 2000


def _prelin_proj_body(x_ref, w1_ref, b1_ref, wcat_ref, bcat_ref,
                      h0_ref, qcat_ref, vcat_ref):
    x = x_ref[...]
    h0 = jnp.dot(x, w1_ref[...], preferred_element_type=jnp.float32) + b1_ref[...]
    h0_ref[...] = h0
    cat = jnp.dot(h0, wcat_ref[...], preferred_element_type=jnp.float32) + bcat_ref[...]
    q = cat[:, :HIDE]
    v = cat[:, HIDE:]
    q2 = q.reshape(q.shape[0], HEAD, DH)
    m = jnp.mean(q2, axis=-1, keepdims=True)
    var = jnp.mean((q2 - m) ** 2, axis=-1, keepdims=True)
    qn = ((q2 - m) / jnp.sqrt(var + 1e-5)).reshape(q.shape)
    qcat_ref[...] = qn
    vcat_ref[...] = v


def _prelin_proj(x, w1t, b1, wcatt, bcat):
    grid = (N // ROWS_BLK,)
    return pl.pallas_call(
        _prelin_proj_body,
        grid=grid,
        in_specs=[
            pl.BlockSpec((ROWS_BLK, C), lambda i: (i, 0)),
            pl.BlockSpec((C, HIDE), lambda i: (0, 0)),
            pl.BlockSpec((1, HIDE), lambda i: (0, 0)),
            pl.BlockSpec((HIDE, 2 * HIDE), lambda i: (0, 0)),
            pl.BlockSpec((1, 2 * HIDE), lambda i: (0, 0)),
        ],
        out_specs=[
            pl.BlockSpec((ROWS_BLK, HIDE), lambda i: (i, 0)),
            pl.BlockSpec((ROWS_BLK, HIDE), lambda i: (i, 0)),
            pl.BlockSpec((ROWS_BLK, HIDE), lambda i: (i, 0)),
        ],
        out_shape=[
            jax.ShapeDtypeStruct((N, HIDE), jnp.float32),
            jax.ShapeDtypeStruct((N, HIDE), jnp.float32),
            jax.ShapeDtypeStruct((N, HIDE), jnp.float32),
        ],
    )(x, w1t, b1, wcatt, bcat)


def _leaky(x):
    return jnp.where(x >= 0, x, 0.01 * x)


NC = 2
NS = 16
NW = NC * NS
KCH = 128
NCHUNKS = EM // KCH
KPW = -(-NCHUNKS // NW)
NSP_S = 10240
SSTR = NSP_S // NS

_sc_mesh = plsc.VectorSubcoreMesh(core_axis_name="c", subcore_axis_name="s")


@functools.partial(
    pl.kernel,
    out_type=[
        jax.ShapeDtypeStruct((2, EM), jnp.float32),
        jax.ShapeDtypeStruct((NC, 2, NSP_S), jnp.float32),
    ],
    mesh=_sc_mesh,
    compiler_params=pltpu.CompilerParams(
        use_tc_tiling_on_sc=False, needs_layout_passes=False),
    scratch_types=[
        pltpu.VMEM((KCH,), jnp.int32),
        pltpu.VMEM((KCH,), jnp.int32),
        pltpu.VMEM((KCH, HIDE), jnp.float32),
        pltpu.VMEM((KCH, HIDE), jnp.float32),
        pltpu.VMEM((KCH,), jnp.float32),
        pltpu.VMEM((KCH,), jnp.float32),
        pltpu.VMEM((SSTR,), jnp.float32),
        pltpu.VMEM_SHARED((NSP_S,), jnp.float32),
        pltpu.VMEM_SHARED((NSP_S,), jnp.float32),
        pltpu.SemaphoreType.DMA,
        pltpu.SemaphoreType.DMA,
    ],
)
def _edge_a(qcat_hbm, src_hbm, dst_hbm, e_hbm, s_hbm,
            src_v, dst_v, qsrc_v, qdst_v, e0_v, e1_v, buf_v,
            s0_sh, s1_sh, sem1, sem2):
    cid = lax.axis_index("c")
    sid = lax.axis_index("s")
    wid = cid * NS + sid

    def _zero(i, _):
        buf_v[pl.ds(i * 16, 16)] = jnp.zeros((16,), jnp.float32)
        return 0
    lax.fori_loop(0, SSTR // 16, _zero, 0)
    pltpu.sync_copy(buf_v, s0_sh.at[pl.ds(sid * SSTR, SSTR)])
    pltpu.sync_copy(buf_v, s1_sh.at[pl.ds(sid * SSTR, SSTR)])
    plsc.subcore_barrier()

    def _chunk(k, _):
        cidx = wid + NW * k

        @pl.when(cidx < NCHUNKS)
        def _():
            base = cidx * KCH
            pltpu.sync_copy(src_hbm.at[pl.ds(base, KCH)], src_v)
            pltpu.sync_copy(dst_hbm.at[pl.ds(base, KCH)], dst_v)
            cp1 = pltpu.async_copy(qcat_hbm.at[src_v], qsrc_v, sem1)
            cp2 = pltpu.async_copy(qcat_hbm.at[dst_v], qdst_v, sem2)
            cp1.wait()
            cp2.wait()

            lastlane = lax.iota(jnp.int32, 16) == 15

            def _edge(ei, _):
                z = jnp.zeros((16,), jnp.float32)
                a0 = z
                a1 = z
                for c in range(DH // 16):
                    a0 = a0 + qsrc_v[ei, pl.ds(c * 16, 16)] * qdst_v[ei, pl.ds(c * 16, 16)]
                    a1 = a1 + qsrc_v[ei, pl.ds(DH + c * 16, 16)] * qdst_v[ei, pl.ds(DH + c * 16, 16)]
                eidx = jnp.full((16,), ei, jnp.int32)
                plsc.store_scatter(e0_v, [eidx], plsc.cumsum(a0), mask=lastlane)
                plsc.store_scatter(e1_v, [eidx], plsc.cumsum(a1), mask=lastlane)
                return 0

            lax.fori_loop(0, KCH, _edge, 0, unroll=2)

            def _vexp(g, _):
                e0_v[pl.ds(g * 16, 16)] = jnp.exp(e0_v[pl.ds(g * 16, 16)] * (1.0 / HIDE))
                e1_v[pl.ds(g * 16, 16)] = jnp.exp(e1_v[pl.ds(g * 16, 16)] * (1.0 / HIDE))
                return 0

            lax.fori_loop(0, KCH // 16, _vexp, 0)
            pltpu.sync_copy(e0_v, e_hbm.at[0, pl.ds(base, KCH)])
            pltpu.sync_copy(e1_v, e_hbm.at[1, pl.ds(base, KCH)])
            pltpu.sync_copy(e0_v, s0_sh.at[src_v], add=True)
            pltpu.sync_copy(e1_v, s1_sh.at[src_v], add=True)

        return 0

    lax.fori_loop(0, KPW, _chunk, 0)
    plsc.subcore_barrier()

    pltpu.sync_copy(s0_sh.at[pl.ds(sid * SSTR, SSTR)], buf_v)
    pltpu.sync_copy(buf_v, s_hbm.at[cid, 0, pl.ds(sid * SSTR, SSTR)])
    pltpu.sync_copy(s1_sh.at[pl.ds(sid * SSTR, SSTR)], buf_v)
    pltpu.sync_copy(buf_v, s_hbm.at[cid, 1, pl.ds(sid * SSTR, SSTR)])


def kernel(x, mask_index, mask_value, q_assign, a_index, a_value,
           prelin_W, prelin_b, bn0_g, bn0_b,
           sf_W, sf_b, sf_g, sf_be,
           pf_Wv, pf_bv, pf_Wq, pf_bq, pf_g, pf_be,
           out_W, out_b):
    bnscale = 1.0 / jnp.sqrt(1.0 + 1e-5)
    w1 = prelin_W * (bn0_g * bnscale)[:, None]
    b1 = prelin_b * (bn0_g * bnscale) + bn0_b

    src = mask_index[0]
    dst = mask_index[1]

    h = None
    for j in range(NPF):
        wcat = jnp.concatenate([pf_Wq[j, 0], pf_Wq[j, 1], pf_Wv[j, 0], pf_Wv[j, 1]], axis=0)
        bcat = jnp.concatenate([pf_bq[j, 0], pf_bq[j, 1], pf_bv[j, 0], pf_bv[j, 1]], axis=0)
        if j == 0:
            h0, qcat, vcat = _prelin_proj(x, w1.T, b1[None, :], wcat.T, bcat[None, :])
            x_pre = h0
        else:
            cat = h @ wcat.T + bcat
            q2 = cat[:, :HIDE].reshape(N, HEAD, DH)
            m = jnp.mean(q2, axis=-1, keepdims=True)
            var = jnp.mean((q2 - m) ** 2, axis=-1, keepdims=True)
            qcat = ((q2 - m) / jnp.sqrt(var + 1e-5)).reshape(N, HIDE)
            vcat = cat[:, HIDE:]
        e2, sP = _edge_a(qcat, src, dst)
        s2 = (sP[0] + sP[1])[:, :N]
        heads = []
        for hh in range(HEAD):
            vh = vcat[:, hh * DH:(hh + 1) * DH]
            w = e2[hh] / s2[hh][src]
            agg = vh[dst] * w[:, None]
            heads.append(jax.ops.segment_sum(agg, src, num_segments=N))
        out = jnp.concatenate(heads, axis=1)
        h = _leaky(out * (pf_g[j] * bnscale) + pf_be[j])
    pf_out = h

    deg = jax.ops.segment_sum(jnp.ones((N,), jnp.float32), q_assign, num_segments=NSP)
    inv = jnp.where(deg > 0, 1.0 / deg, 0.0)
    hp = jax.ops.segment_sum(x_pre, q_assign, num_segments=NSP) * inv[:, None]
    A = jnp.zeros((NSP, NSP), jnp.float32).at[a_index[1], a_index[0]].add(a_value)
    for i in range(NSF):
        hlin = hp @ sf_W[i].T + sf_b[i]
        x_start = A @ hlin
        xc = hp
        for _ in range(RHP):
            xc = (A @ xc + x_start) / (2.0 + GAMA)
        hp = _leaky(xc * (sf_g[i] * bnscale) + sf_be[i])
    sf_out = hp[q_assign]

    H1 = pf_out + sf_out
    Y = jax.nn.softmax(H1 @ out_W.T + out_b, axis=-1)
    return Y


# speedup vs baseline: 7.6340x; 2.9316x over previous
import functools

import jax
import jax.numpy as jnp
from jax import lax
from jax.experimental import pallas as pl
from jax.experimental.pallas import tpu as pltpu
from jax.experimental.pallas import tpu_sc as plsc

N = 10000
NSP = 500
C = 256
HIDE = 256
NCLS = 16
EM = 160000
EA = 8000
HEAD = 2
DH = HIDE // HEAD
NSF = 5
NPF = 2
RHP = 5
GAMA = 0.9

ROWS_BLK = 2000


def _prelin_proj_body(x_ref, w1_ref, b1_ref, wcat_ref, bcat_ref,
                      h0_ref, qcat_ref, vcat_ref):
    x = x_ref[...]
    h0 = jnp.dot(x, w1_ref[...], preferred_element_type=jnp.float32) + b1_ref[...]
    h0_ref[...] = h0
    cat = jnp.dot(h0, wcat_ref[...], preferred_element_type=jnp.float32) + bcat_ref[...]
    q = cat[:, :HIDE]
    v = cat[:, HIDE:]
    q2 = q.reshape(q.shape[0], HEAD, DH)
    m = jnp.mean(q2, axis=-1, keepdims=True)
    var = jnp.mean((q2 - m) ** 2, axis=-1, keepdims=True)
    qn = ((q2 - m) / jnp.sqrt(var + 1e-5)).reshape(q.shape)
    qcat_ref[...] = qn
    vcat_ref[...] = v


def _prelin_proj(x, w1t, b1, wcatt, bcat):
    grid = (N // ROWS_BLK,)
    return pl.pallas_call(
        _prelin_proj_body,
        grid=grid,
        in_specs=[
            pl.BlockSpec((ROWS_BLK, C), lambda i: (i, 0)),
            pl.BlockSpec((C, HIDE), lambda i: (0, 0)),
            pl.BlockSpec((1, HIDE), lambda i: (0, 0)),
            pl.BlockSpec((HIDE, 2 * HIDE), lambda i: (0, 0)),
            pl.BlockSpec((1, 2 * HIDE), lambda i: (0, 0)),
        ],
        out_specs=[
            pl.BlockSpec((ROWS_BLK, HIDE), lambda i: (i, 0)),
            pl.BlockSpec((ROWS_BLK, HIDE), lambda i: (i, 0)),
            pl.BlockSpec((ROWS_BLK, HIDE), lambda i: (i, 0)),
        ],
        out_shape=[
            jax.ShapeDtypeStruct((N, HIDE), jnp.float32),
            jax.ShapeDtypeStruct((N, HIDE), jnp.float32),
            jax.ShapeDtypeStruct((N, HIDE), jnp.float32),
        ],
    )(x, w1t, b1, wcatt, bcat)


def _leaky(x):
    return jnp.where(x >= 0, x, 0.01 * x)


NC = 2
NS = 16
NW = NC * NS
KCH = 128
NCHUNKS = EM // KCH
KPW = -(-NCHUNKS // NW)
NSP_S = 10240
SSTR = NSP_S // NS

_sc_mesh = plsc.VectorSubcoreMesh(core_axis_name="c", subcore_axis_name="s")


@functools.partial(
    pl.kernel,
    out_type=[
        jax.ShapeDtypeStruct((2, EM), jnp.float32),
        jax.ShapeDtypeStruct((NC, 2, NSP_S), jnp.float32),
    ],
    mesh=_sc_mesh,
    compiler_params=pltpu.CompilerParams(
        use_tc_tiling_on_sc=False, needs_layout_passes=False),
    scratch_types=[
        pltpu.VMEM((KCH,), jnp.int32),
        pltpu.VMEM((KCH,), jnp.int32),
        pltpu.VMEM((KCH, HIDE), jnp.float32),
        pltpu.VMEM((KCH, HIDE), jnp.float32),
        pltpu.VMEM((KCH,), jnp.float32),
        pltpu.VMEM((KCH,), jnp.float32),
        pltpu.VMEM((SSTR,), jnp.float32),
        pltpu.VMEM_SHARED((NSP_S,), jnp.float32),
        pltpu.VMEM_SHARED((NSP_S,), jnp.float32),
        pltpu.SemaphoreType.DMA,
        pltpu.SemaphoreType.DMA,
    ],
)
def _edge_a(qcat_hbm, src_hbm, dst_hbm, e_hbm, s_hbm,
            src_v, dst_v, qsrc_v, qdst_v, e0_v, e1_v, buf_v,
            s0_sh, s1_sh, sem1, sem2):
    cid = lax.axis_index("c")
    sid = lax.axis_index("s")
    wid = cid * NS + sid

    def _zero(i, _):
        buf_v[pl.ds(i * 16, 16)] = jnp.zeros((16,), jnp.float32)
        return 0
    lax.fori_loop(0, SSTR // 16, _zero, 0)
    pltpu.sync_copy(buf_v, s0_sh.at[pl.ds(sid * SSTR, SSTR)])
    pltpu.sync_copy(buf_v, s1_sh.at[pl.ds(sid * SSTR, SSTR)])
    plsc.subcore_barrier()

    def _chunk(k, _):
        cidx = wid + NW * k

        @pl.when(cidx < NCHUNKS)
        def _():
            base = cidx * KCH
            pltpu.sync_copy(src_hbm.at[pl.ds(base, KCH)], src_v)
            pltpu.sync_copy(dst_hbm.at[pl.ds(base, KCH)], dst_v)
            cp1 = pltpu.async_copy(qcat_hbm.at[src_v], qsrc_v, sem1)
            cp2 = pltpu.async_copy(qcat_hbm.at[dst_v], qdst_v, sem2)
            cp1.wait()
            cp2.wait()

            lastlane = lax.iota(jnp.int32, 16) == 15

            def _edge(ei, _):
                z = jnp.zeros((16,), jnp.float32)
                a0 = z
                a1 = z
                for c in range(DH // 16):
                    a0 = a0 + qsrc_v[ei, pl.ds(c * 16, 16)] * qdst_v[ei, pl.ds(c * 16, 16)]
                    a1 = a1 + qsrc_v[ei, pl.ds(DH + c * 16, 16)] * qdst_v[ei, pl.ds(DH + c * 16, 16)]
                eidx = jnp.full((16,), ei, jnp.int32)
                plsc.store_scatter(e0_v, [eidx], plsc.cumsum(a0), mask=lastlane)
                plsc.store_scatter(e1_v, [eidx], plsc.cumsum(a1), mask=lastlane)
                return 0

            lax.fori_loop(0, KCH, _edge, 0, unroll=2)

            def _vexp(g, _):
                e0_v[pl.ds(g * 16, 16)] = jnp.exp(e0_v[pl.ds(g * 16, 16)] * (1.0 / HIDE))
                e1_v[pl.ds(g * 16, 16)] = jnp.exp(e1_v[pl.ds(g * 16, 16)] * (1.0 / HIDE))
                return 0

            lax.fori_loop(0, KCH // 16, _vexp, 0)
            pltpu.sync_copy(e0_v, e_hbm.at[0, pl.ds(base, KCH)])
            pltpu.sync_copy(e1_v, e_hbm.at[1, pl.ds(base, KCH)])
            pltpu.sync_copy(e0_v, s0_sh.at[src_v], add=True)
            pltpu.sync_copy(e1_v, s1_sh.at[src_v], add=True)

        return 0

    lax.fori_loop(0, KPW, _chunk, 0)
    plsc.subcore_barrier()

    pltpu.sync_copy(s0_sh.at[pl.ds(sid * SSTR, SSTR)], buf_v)
    pltpu.sync_copy(buf_v, s_hbm.at[cid, 0, pl.ds(sid * SSTR, SSTR)])
    pltpu.sync_copy(s1_sh.at[pl.ds(sid * SSTR, SSTR)], buf_v)
    pltpu.sync_copy(buf_v, s_hbm.at[cid, 1, pl.ds(sid * SSTR, SSTR)])


RPW = N // NS
RPP = 125


@functools.partial(
    pl.kernel,
    out_type=jax.ShapeDtypeStruct((NC, HEAD, N, DH), jnp.float32),
    mesh=_sc_mesh,
    compiler_params=pltpu.CompilerParams(
        use_tc_tiling_on_sc=False, needs_layout_passes=False),
    scratch_types=[
        pltpu.VMEM((KCH,), jnp.int32),
        pltpu.VMEM((KCH,), jnp.int32),
        pltpu.VMEM((KCH, DH), jnp.float32),
        pltpu.VMEM((KCH,), jnp.float32),
        pltpu.VMEM((KCH,), jnp.float32),
        pltpu.VMEM((NSP_S,), jnp.float32),
        pltpu.VMEM((NSP_S,), jnp.float32),
        pltpu.VMEM_SHARED((N, DH), jnp.float32),
        pltpu.SemaphoreType.DMA,
    ],
)
def _edge_b(v0_hbm, v1_hbm, e_hbm, s_hbm, src_hbm, dst_hbm, out_hbm,
            src_v, dst_v, vrow_v, e_v, w_v, sA_v, sB_v, out_sh, sem1):
    cid = lax.axis_index("c")
    sid = lax.axis_index("s")
    wid = cid * NS + sid

    for hh in range(HEAD):
        v_hbm = v0_hbm if hh == 0 else v1_hbm
        pltpu.sync_copy(s_hbm.at[0, hh], sA_v)
        pltpu.sync_copy(s_hbm.at[1, hh], sB_v)

        def _sadd(i, _):
            sA_v[pl.ds(i * 16, 16)] = sA_v[pl.ds(i * 16, 16)] + sB_v[pl.ds(i * 16, 16)]
            return 0
        lax.fori_loop(0, NSP_S // 16, _sadd, 0)

        def _zrow2(i, _):
            def _zc(c, _):
                vrow_v[i, pl.ds(c * 16, 16)] = jnp.zeros((16,), jnp.float32)
                return 0
            lax.fori_loop(0, DH // 16, _zc, 0)
            return 0
        lax.fori_loop(0, KCH, _zrow2, 0)
        for p in range(RPW // RPP):
            pltpu.sync_copy(vrow_v.at[pl.ds(0, RPP), :],
                            out_sh.at[pl.ds(sid * RPW + p * RPP, RPP), :])
        plsc.subcore_barrier()

        def _chunk(k, _):
            cidx = wid + NW * k

            @pl.when(cidx < NCHUNKS)
            def _():
                base = cidx * KCH
                pltpu.sync_copy(src_hbm.at[pl.ds(base, KCH)], src_v)
                pltpu.sync_copy(dst_hbm.at[pl.ds(base, KCH)], dst_v)
                pltpu.sync_copy(e_hbm.at[hh, pl.ds(base, KCH)], e_v)
                pltpu.async_copy(v_hbm.at[dst_v], vrow_v, sem1).wait()

                def _wgrp(g, _):
                    sl = pl.ds(g * 16, 16)
                    sg = plsc.load_gather(sA_v, [src_v[sl]])
                    w_v[sl] = e_v[sl] / sg
                    return 0
                lax.fori_loop(0, KCH // 16, _wgrp, 0)

                def _scale(ei, _):
                    wb = plsc.load_gather(w_v, [jnp.full((16,), ei, jnp.int32)])
                    for c in range(DH // 16):
                        vrow_v[ei, pl.ds(c * 16, 16)] = vrow_v[ei, pl.ds(c * 16, 16)] * wb
                    return 0
                lax.fori_loop(0, KCH, _scale, 0, unroll=2)

                pltpu.sync_copy(vrow_v, out_sh.at[src_v], add=True)
            return 0

        lax.fori_loop(0, KPW, _chunk, 0)
        plsc.subcore_barrier()

        for p in range(RPW // RPP):
            r0 = sid * RPW + p * RPP
            pltpu.sync_copy(out_sh.at[pl.ds(r0, RPP), :], vrow_v.at[pl.ds(0, RPP), :])
            pltpu.sync_copy(vrow_v.at[pl.ds(0, RPP), :],
                            out_hbm.at[cid, hh, pl.ds(r0, RPP), :])
        if hh == 0:
            plsc.subcore_barrier()


def kernel(x, mask_index, mask_value, q_assign, a_index, a_value,
           prelin_W, prelin_b, bn0_g, bn0_b,
           sf_W, sf_b, sf_g, sf_be,
           pf_Wv, pf_bv, pf_Wq, pf_bq, pf_g, pf_be,
           out_W, out_b):
    bnscale = 1.0 / jnp.sqrt(1.0 + 1e-5)
    w1 = prelin_W * (bn0_g * bnscale)[:, None]
    b1 = prelin_b * (bn0_g * bnscale) + bn0_b

    src = mask_index[0]
    dst = mask_index[1]

    h = None
    for j in range(NPF):
        wcat = jnp.concatenate([pf_Wq[j, 0], pf_Wq[j, 1], pf_Wv[j, 0], pf_Wv[j, 1]], axis=0)
        bcat = jnp.concatenate([pf_bq[j, 0], pf_bq[j, 1], pf_bv[j, 0], pf_bv[j, 1]], axis=0)
        if j == 0:
            h0, qcat, vcat = _prelin_proj(x, w1.T, b1[None, :], wcat.T, bcat[None, :])
            x_pre = h0
        else:
            cat = h @ wcat.T + bcat
            q2 = cat[:, :HIDE].reshape(N, HEAD, DH)
            m = jnp.mean(q2, axis=-1, keepdims=True)
            var = jnp.mean((q2 - m) ** 2, axis=-1, keepdims=True)
            qcat = ((q2 - m) / jnp.sqrt(var + 1e-5)).reshape(N, HIDE)
            vcat = cat[:, HIDE:]
        e2, sP = _edge_a(qcat, src, dst)
        outP = _edge_b(vcat[:, :DH], vcat[:, DH:], e2, sP, src, dst)
        out = jnp.concatenate([outP[0, 0] + outP[1, 0],
                               outP[0, 1] + outP[1, 1]], axis=1)
        h = _leaky(out * (pf_g[j] * bnscale) + pf_be[j])
    pf_out = h

    deg = jax.ops.segment_sum(jnp.ones((N,), jnp.float32), q_assign, num_segments=NSP)
    inv = jnp.where(deg > 0, 1.0 / deg, 0.0)
    hp = jax.ops.segment_sum(x_pre, q_assign, num_segments=NSP) * inv[:, None]
    A = jnp.zeros((NSP, NSP), jnp.float32).at[a_index[1], a_index[0]].add(a_value)
    for i in range(NSF):
        hlin = hp @ sf_W[i].T + sf_b[i]
        x_start = A @ hlin
        xc = hp
        for _ in range(RHP):
            xc = (A @ xc + x_start) / (2.0 + GAMA)
        hp = _leaky(xc * (sf_g[i] * bnscale) + sf_be[i])
    sf_out = hp[q_assign]

    H1 = pf_out + sf_out
    Y = jax.nn.softmax(H1 @ out_W.T + out_b, axis=-1)
    return Y


# speedup vs baseline: 8.7736x; 1.1493x over previous
import functools

import jax
import jax.numpy as jnp
from jax import lax
from jax.experimental import pallas as pl
from jax.experimental.pallas import tpu as pltpu
from jax.experimental.pallas import tpu_sc as plsc

N = 10000
NSP = 500
C = 256
HIDE = 256
NCLS = 16
EM = 160000
EA = 8000
HEAD = 2
DH = HIDE // HEAD
NSF = 5
NPF = 2
RHP = 5
GAMA = 0.9

ROWS_BLK = 2000


def _prelin_proj_body(x_ref, w1_ref, b1_ref, wcat_ref, bcat_ref,
                      h0_ref, qcat_ref, vcat_ref):
    x = x_ref[...]
    h0 = jnp.dot(x, w1_ref[...], preferred_element_type=jnp.float32) + b1_ref[...]
    h0_ref[...] = h0
    cat = jnp.dot(h0, wcat_ref[...], preferred_element_type=jnp.float32) + bcat_ref[...]
    q = cat[:, :HIDE]
    v = cat[:, HIDE:]
    q2 = q.reshape(q.shape[0], HEAD, DH)
    m = jnp.mean(q2, axis=-1, keepdims=True)
    var = jnp.mean((q2 - m) ** 2, axis=-1, keepdims=True)
    qn = ((q2 - m) / jnp.sqrt(var + 1e-5)).reshape(q.shape)
    qcat_ref[...] = qn
    vcat_ref[...] = v


def _prelin_proj(x, w1t, b1, wcatt, bcat):
    grid = (N // ROWS_BLK,)
    return pl.pallas_call(
        _prelin_proj_body,
        grid=grid,
        in_specs=[
            pl.BlockSpec((ROWS_BLK, C), lambda i: (i, 0)),
            pl.BlockSpec((C, HIDE), lambda i: (0, 0)),
            pl.BlockSpec((1, HIDE), lambda i: (0, 0)),
            pl.BlockSpec((HIDE, 2 * HIDE), lambda i: (0, 0)),
            pl.BlockSpec((1, 2 * HIDE), lambda i: (0, 0)),
        ],
        out_specs=[
            pl.BlockSpec((ROWS_BLK, HIDE), lambda i: (i, 0)),
            pl.BlockSpec((ROWS_BLK, HIDE), lambda i: (i, 0)),
            pl.BlockSpec((ROWS_BLK, HIDE), lambda i: (i, 0)),
        ],
        out_shape=[
            jax.ShapeDtypeStruct((N, HIDE), jnp.float32),
            jax.ShapeDtypeStruct((N, HIDE), jnp.float32),
            jax.ShapeDtypeStruct((N, HIDE), jnp.float32),
        ],
    )(x, w1t, b1, wcatt, bcat)


def _leaky(x):
    return jnp.where(x >= 0, x, 0.01 * x)


NC = 2
NS = 16
NW = NC * NS
KCH = 128
NCHUNKS = EM // KCH
KPW = -(-NCHUNKS // NW)
NSP_S = 10240
SSTR = NSP_S // NS

_sc_mesh = plsc.VectorSubcoreMesh(core_axis_name="c", subcore_axis_name="s")


@functools.partial(
    pl.kernel,
    out_type=[
        jax.ShapeDtypeStruct((2, EM), jnp.float32),
        jax.ShapeDtypeStruct((NC, 2, NSP_S), jnp.float32),
    ],
    mesh=_sc_mesh,
    compiler_params=pltpu.CompilerParams(
        use_tc_tiling_on_sc=False, needs_layout_passes=False),
    scratch_types=[
        pltpu.VMEM((KCH,), jnp.int32),
        pltpu.VMEM((KCH,), jnp.int32),
        pltpu.VMEM((KCH, HIDE), jnp.float32),
        pltpu.VMEM((KCH, HIDE), jnp.float32),
        pltpu.VMEM((KCH,), jnp.float32),
        pltpu.VMEM((KCH,), jnp.float32),
        pltpu.VMEM((SSTR,), jnp.float32),
        pltpu.VMEM_SHARED((NSP_S,), jnp.float32),
        pltpu.VMEM_SHARED((NSP_S,), jnp.float32),
        pltpu.SemaphoreType.DMA,
        pltpu.SemaphoreType.DMA,
    ],
)
def _edge_a(qcat_hbm, src_hbm, dst_hbm, e_hbm, s_hbm,
            src_v, dst_v, qsrc_v, qdst_v, e0_v, e1_v, buf_v,
            s0_sh, s1_sh, sem1, sem2):
    cid = lax.axis_index("c")
    sid = lax.axis_index("s")
    wid = cid * NS + sid

    def _zero(i, _):
        buf_v[pl.ds(i * 16, 16)] = jnp.zeros((16,), jnp.float32)
        return 0
    lax.fori_loop(0, SSTR // 16, _zero, 0)
    pltpu.sync_copy(buf_v, s0_sh.at[pl.ds(sid * SSTR, SSTR)])
    pltpu.sync_copy(buf_v, s1_sh.at[pl.ds(sid * SSTR, SSTR)])
    plsc.subcore_barrier()

    def _chunk(k, _):
        cidx = wid + NW * k

        @pl.when(cidx < NCHUNKS)
        def _():
            base = cidx * KCH
            pltpu.sync_copy(src_hbm.at[pl.ds(base, KCH)], src_v)
            pltpu.sync_copy(dst_hbm.at[pl.ds(base, KCH)], dst_v)
            cp1 = pltpu.async_copy(qcat_hbm.at[src_v], qsrc_v, sem1)
            cp2 = pltpu.async_copy(qcat_hbm.at[dst_v], qdst_v, sem2)
            cp1.wait()
            cp2.wait()

            lastlane = lax.iota(jnp.int32, 16) == 15

            def _edge(ei, _):
                z = jnp.zeros((16,), jnp.float32)
                a0 = z
                a1 = z
                for c in range(DH // 16):
                    a0 = a0 + qsrc_v[ei, pl.ds(c * 16, 16)] * qdst_v[ei, pl.ds(c * 16, 16)]
                    a1 = a1 + qsrc_v[ei, pl.ds(DH + c * 16, 16)] * qdst_v[ei, pl.ds(DH + c * 16, 16)]
                eidx = jnp.full((16,), ei, jnp.int32)
                plsc.store_scatter(e0_v, [eidx], plsc.cumsum(a0), mask=lastlane)
                plsc.store_scatter(e1_v, [eidx], plsc.cumsum(a1), mask=lastlane)
                return 0

            lax.fori_loop(0, KCH, _edge, 0, unroll=2)

            def _vexp(g, _):
                e0_v[pl.ds(g * 16, 16)] = jnp.exp(e0_v[pl.ds(g * 16, 16)] * (1.0 / HIDE))
                e1_v[pl.ds(g * 16, 16)] = jnp.exp(e1_v[pl.ds(g * 16, 16)] * (1.0 / HIDE))
                return 0

            lax.fori_loop(0, KCH // 16, _vexp, 0)
            pltpu.sync_copy(e0_v, e_hbm.at[0, pl.ds(base, KCH)])
            pltpu.sync_copy(e1_v, e_hbm.at[1, pl.ds(base, KCH)])
            pltpu.sync_copy(e0_v, s0_sh.at[src_v], add=True)
            pltpu.sync_copy(e1_v, s1_sh.at[src_v], add=True)

        return 0

    lax.fori_loop(0, KPW, _chunk, 0)
    plsc.subcore_barrier()

    pltpu.sync_copy(s0_sh.at[pl.ds(sid * SSTR, SSTR)], buf_v)
    pltpu.sync_copy(buf_v, s_hbm.at[cid, 0, pl.ds(sid * SSTR, SSTR)])
    pltpu.sync_copy(s1_sh.at[pl.ds(sid * SSTR, SSTR)], buf_v)
    pltpu.sync_copy(buf_v, s_hbm.at[cid, 1, pl.ds(sid * SSTR, SSTR)])


NSP_P = 512


def _mid_body(o_ref, g_ref, b_ref, wcat_ref, bcat_ref, qcat_ref, vcat_ref):
    o = o_ref[...]
    h = jnp.concatenate([o[0] + o[2], o[1] + o[3]], axis=1)
    h = h * g_ref[...] + b_ref[...]
    h = jnp.where(h >= 0, h, 0.01 * h)
    cat = jnp.dot(h, wcat_ref[...], preferred_element_type=jnp.float32) + bcat_ref[...]
    q = cat[:, :HIDE]
    q2 = q.reshape(q.shape[0], HEAD, DH)
    m = jnp.mean(q2, axis=-1, keepdims=True)
    var = jnp.mean((q2 - m) ** 2, axis=-1, keepdims=True)
    qcat_ref[...] = ((q2 - m) / jnp.sqrt(var + 1e-5)).reshape(q.shape)
    vcat_ref[...] = cat[:, HIDE:]


def _mid(outPr, g, b, wcatt, bcat):
    return pl.pallas_call(
        _mid_body,
        grid=(N // ROWS_BLK,),
        in_specs=[
            pl.BlockSpec((4, ROWS_BLK, DH), lambda i: (0, i, 0)),
            pl.BlockSpec((1, HIDE), lambda i: (0, 0)),
            pl.BlockSpec((1, HIDE), lambda i: (0, 0)),
            pl.BlockSpec((HIDE, 2 * HIDE), lambda i: (0, 0)),
            pl.BlockSpec((1, 2 * HIDE), lambda i: (0, 0)),
        ],
        out_specs=[
            pl.BlockSpec((ROWS_BLK, HIDE), lambda i: (i, 0)),
            pl.BlockSpec((ROWS_BLK, HIDE), lambda i: (i, 0)),
        ],
        out_shape=[
            jax.ShapeDtypeStruct((N, HIDE), jnp.float32),
            jax.ShapeDtypeStruct((N, HIDE), jnp.float32),
        ],
    )(outPr, g, b, wcatt, bcat)


def _sf_body(hpP_ref, degP_ref, AP_ref, wt_ref, b_ref, g_ref, be_ref, out_ref):
    degP = degP_ref[...]
    deg = degP[0] + degP[1]
    inv = jnp.where(deg > 0, 1.0 / deg, 0.0)
    hp = (hpP_ref[0] + hpP_ref[1]) * inv[:, None]
    A = AP_ref[0] + AP_ref[1]
    for i in range(NSF):
        hlin = jnp.dot(hp, wt_ref[i], preferred_element_type=jnp.float32) + b_ref[i]
        x_start = jnp.dot(A, hlin, preferred_element_type=jnp.float32)
        xc = hp
        for _ in range(RHP):
            xc = (jnp.dot(A, xc, preferred_element_type=jnp.float32) + x_start) * (1.0 / (2.0 + GAMA))
        hp = xc * g_ref[i] + be_ref[i]
        hp = jnp.where(hp >= 0, hp, 0.01 * hp)
    out_ref[...] = hp


def _sf_dense(hpP, degP, AP, wt, b, g, be):
    return pl.pallas_call(
        _sf_body,
        out_shape=jax.ShapeDtypeStruct((NSP_P, HIDE), jnp.float32),
    )(hpP, degP, AP, wt, b, g, be)


def _fin_body(o_ref, g_ref, b_ref, qa_ref, hp_ref, wout_ref, bout_ref, y_ref):
    o = o_ref[...]
    h = jnp.concatenate([o[0] + o[2], o[1] + o[3]], axis=1)
    h = h * g_ref[...] + b_ref[...]
    pf = jnp.where(h >= 0, h, 0.01 * h)
    qa = qa_ref[...]
    oh = (qa == jax.lax.broadcasted_iota(jnp.int32, (qa.shape[0], NSP_P), 1)
          ).astype(jnp.float32)
    sf = jnp.dot(oh, hp_ref[...], preferred_element_type=jnp.float32)
    H1 = pf + sf
    logits = jnp.dot(H1, wout_ref[...], preferred_element_type=jnp.float32) + bout_ref[...]
    mx = jnp.max(logits, axis=-1, keepdims=True)
    e = jnp.exp(logits - mx)
    y_ref[...] = e / jnp.sum(e, axis=-1, keepdims=True)


def _final(outPr, g, b, qa2, hp, wout, bout):
    return pl.pallas_call(
        _fin_body,
        grid=(N // ROWS_BLK,),
        in_specs=[
            pl.BlockSpec((4, ROWS_BLK, DH), lambda i: (0, i, 0)),
            pl.BlockSpec((1, HIDE), lambda i: (0, 0)),
            pl.BlockSpec((1, HIDE), lambda i: (0, 0)),
            pl.BlockSpec((ROWS_BLK, 1), lambda i: (i, 0)),
            pl.BlockSpec((NSP_P, HIDE), lambda i: (0, 0)),
            pl.BlockSpec((HIDE, 128), lambda i: (0, 0)),
            pl.BlockSpec((1, 128), lambda i: (0, 0)),
        ],
        out_specs=pl.BlockSpec((ROWS_BLK, 128), lambda i: (i, 0)),
        out_shape=jax.ShapeDtypeStruct((N, 128), jnp.float32),
    )(outPr, g, b, qa2, hp, wout, bout)


RPW = N // NS
RPP = 125


@functools.partial(
    pl.kernel,
    out_type=jax.ShapeDtypeStruct((NC, HEAD, N, DH), jnp.float32),
    mesh=_sc_mesh,
    compiler_params=pltpu.CompilerParams(
        use_tc_tiling_on_sc=False, needs_layout_passes=False),
    scratch_types=[
        pltpu.VMEM((KCH,), jnp.int32),
        pltpu.VMEM((KCH,), jnp.int32),
        pltpu.VMEM((KCH, DH), jnp.float32),
        pltpu.VMEM((KCH,), jnp.float32),
        pltpu.VMEM((KCH,), jnp.float32),
        pltpu.VMEM((NSP_S,), jnp.float32),
        pltpu.VMEM((NSP_S,), jnp.float32),
        pltpu.VMEM_SHARED((N, DH), jnp.float32),
        pltpu.SemaphoreType.DMA,
    ],
)
def _edge_b(v0_hbm, v1_hbm, e_hbm, s_hbm, src_hbm, dst_hbm, out_hbm,
            src_v, dst_v, vrow_v, e_v, w_v, sA_v, sB_v, out_sh, sem1):
    cid = lax.axis_index("c")
    sid = lax.axis_index("s")
    wid = cid * NS + sid

    for hh in range(HEAD):
        v_hbm = v0_hbm if hh == 0 else v1_hbm
        pltpu.sync_copy(s_hbm.at[0, hh], sA_v)
        pltpu.sync_copy(s_hbm.at[1, hh], sB_v)

        def _sadd(i, _):
            sA_v[pl.ds(i * 16, 16)] = sA_v[pl.ds(i * 16, 16)] + sB_v[pl.ds(i * 16, 16)]
            return 0
        lax.fori_loop(0, NSP_S // 16, _sadd, 0)

        def _zrow2(i, _):
            def _zc(c, _):
                vrow_v[i, pl.ds(c * 16, 16)] = jnp.zeros((16,), jnp.float32)
                return 0
            lax.fori_loop(0, DH // 16, _zc, 0)
            return 0
        lax.fori_loop(0, KCH, _zrow2, 0)
        for p in range(RPW // RPP):
            pltpu.sync_copy(vrow_v.at[pl.ds(0, RPP), :],
                            out_sh.at[pl.ds(sid * RPW + p * RPP, RPP), :])
        plsc.subcore_barrier()

        def _chunk(k, _):
            cidx = wid + NW * k

            @pl.when(cidx < NCHUNKS)
            def _():
                base = cidx * KCH
                pltpu.sync_copy(src_hbm.at[pl.ds(base, KCH)], src_v)
                pltpu.sync_copy(dst_hbm.at[pl.ds(base, KCH)], dst_v)
                pltpu.sync_copy(e_hbm.at[hh, pl.ds(base, KCH)], e_v)
                pltpu.async_copy(v_hbm.at[dst_v], vrow_v, sem1).wait()

                def _wgrp(g, _):
                    sl = pl.ds(g * 16, 16)
                    sg = plsc.load_gather(sA_v, [src_v[sl]])
                    w_v[sl] = e_v[sl] / sg
                    return 0
                lax.fori_loop(0, KCH // 16, _wgrp, 0)

                def _scale(ei, _):
                    wb = plsc.load_gather(w_v, [jnp.full((16,), ei, jnp.int32)])
                    for c in range(DH // 16):
                        vrow_v[ei, pl.ds(c * 16, 16)] = vrow_v[ei, pl.ds(c * 16, 16)] * wb
                    return 0
                lax.fori_loop(0, KCH, _scale, 0, unroll=2)

                pltpu.sync_copy(vrow_v, out_sh.at[src_v], add=True)
            return 0

        lax.fori_loop(0, KPW, _chunk, 0)
        plsc.subcore_barrier()

        for p in range(RPW // RPP):
            r0 = sid * RPW + p * RPP
            pltpu.sync_copy(out_sh.at[pl.ds(r0, RPP), :], vrow_v.at[pl.ds(0, RPP), :])
            pltpu.sync_copy(vrow_v.at[pl.ds(0, RPP), :],
                            out_hbm.at[cid, hh, pl.ds(r0, RPP), :])
        if hh == 0:
            plsc.subcore_barrier()


NCH_P = N // KCH
PTAIL = N - NCH_P * KCH
KPW_P = -(-NCH_P // NW)
SPSTR = NSP_P // NS


@functools.partial(
    pl.kernel,
    out_type=[
        jax.ShapeDtypeStruct((NC, NSP_P, HIDE), jnp.float32),
        jax.ShapeDtypeStruct((NC, NSP_P), jnp.float32),
    ],
    mesh=_sc_mesh,
    compiler_params=pltpu.CompilerParams(
        use_tc_tiling_on_sc=False, needs_layout_passes=False),
    scratch_types=[
        pltpu.VMEM((KCH,), jnp.int32),
        pltpu.VMEM((KCH, HIDE), jnp.float32),
        pltpu.VMEM((KCH,), jnp.float32),
        pltpu.VMEM((PTAIL,), jnp.int32),
        pltpu.VMEM((PTAIL, HIDE), jnp.float32),
        pltpu.VMEM((PTAIL,), jnp.float32),
        pltpu.VMEM((SPSTR,), jnp.float32),
        pltpu.VMEM_SHARED((NSP_P, HIDE), jnp.float32),
        pltpu.VMEM_SHARED((NSP_P,), jnp.float32),
    ],
)
def _pool(x_hbm, qa_hbm, hp_hbm, deg_hbm,
          qa_v, rows_v, ones_v, qat_v, rowst_v, onest_v, buf_v,
          hp_sh, deg_sh):
    cid = lax.axis_index("c")
    sid = lax.axis_index("s")
    wid = cid * NS + sid

    def _fill(ref, n, val):
        def _f(i, _):
            ref[pl.ds(i * 16, 16)] = jnp.full((16,), val, jnp.float32)
            return 0
        lax.fori_loop(0, n // 16, _f, 0)

    _fill(ones_v, KCH, 1.0)
    _fill(onest_v, PTAIL, 1.0)
    _fill(buf_v, SPSTR, 0.0)

    def _zrow(i, _):
        def _zc(c, _):
            rows_v[i, pl.ds(c * 16, 16)] = jnp.zeros((16,), jnp.float32)
            return 0
        lax.fori_loop(0, HIDE // 16, _zc, 0)
        return 0
    lax.fori_loop(0, SPSTR, _zrow, 0)
    pltpu.sync_copy(rows_v.at[pl.ds(0, SPSTR), :],
                    hp_sh.at[pl.ds(sid * SPSTR, SPSTR), :])
    pltpu.sync_copy(buf_v, deg_sh.at[pl.ds(sid * SPSTR, SPSTR)])
    plsc.subcore_barrier()

    def _chunk(k, _):
        cidx = wid + NW * k

        @pl.when(cidx < NCH_P)
        def _():
            base = cidx * KCH
            pltpu.sync_copy(qa_hbm.at[pl.ds(base, KCH)], qa_v)
            pltpu.sync_copy(x_hbm.at[pl.ds(base, KCH), :], rows_v)
            pltpu.sync_copy(rows_v, hp_sh.at[qa_v], add=True)
            pltpu.sync_copy(ones_v, deg_sh.at[qa_v], add=True)
        return 0

    lax.fori_loop(0, KPW_P, _chunk, 0)

    @pl.when(wid == 0)
    def _tail():
        base = NCH_P * KCH
        pltpu.sync_copy(qa_hbm.at[pl.ds(base, PTAIL)], qat_v)
        pltpu.sync_copy(x_hbm.at[pl.ds(base, PTAIL), :], rowst_v)
        pltpu.sync_copy(rowst_v, hp_sh.at[qat_v], add=True)
        pltpu.sync_copy(onest_v, deg_sh.at[qat_v], add=True)

    plsc.subcore_barrier()
    r0 = sid * SPSTR
    pltpu.sync_copy(hp_sh.at[pl.ds(r0, SPSTR), :], rows_v.at[pl.ds(0, SPSTR), :])
    pltpu.sync_copy(rows_v.at[pl.ds(0, SPSTR), :], hp_hbm.at[cid, pl.ds(r0, SPSTR), :])
    pltpu.sync_copy(deg_sh.at[pl.ds(r0, SPSTR)], buf_v)
    pltpu.sync_copy(buf_v, deg_hbm.at[cid, pl.ds(r0, SPSTR)])


NCH_A = EA // KCH
ATAIL = EA - NCH_A * KCH
KPW_A = -(-NCH_A // NW)
AFL = NSP_P * NSP_P
ASTR = AFL // NS


@functools.partial(
    pl.kernel,
    out_type=jax.ShapeDtypeStruct((NC, AFL), jnp.float32),
    mesh=_sc_mesh,
    compiler_params=pltpu.CompilerParams(
        use_tc_tiling_on_sc=False, needs_layout_passes=False),
    scratch_types=[
        pltpu.VMEM((KCH,), jnp.int32),
        pltpu.VMEM((KCH,), jnp.int32),
        pltpu.VMEM((KCH,), jnp.float32),
        pltpu.VMEM((KCH,), jnp.int32),
        pltpu.VMEM((ATAIL,), jnp.int32),
        pltpu.VMEM((ATAIL,), jnp.int32),
        pltpu.VMEM((ATAIL,), jnp.float32),
        pltpu.VMEM((ATAIL,), jnp.int32),
        pltpu.VMEM((ASTR,), jnp.float32),
        pltpu.VMEM_SHARED((AFL,), jnp.float32),
    ],
)
def _abuild(asrc_hbm, adst_hbm, aval_hbm, A_hbm,
            s_v, d_v, v_v, fidx_v, st_v, dt_v, vt_v, fidxt_v, zbuf_v, A_sh):
    cid = lax.axis_index("c")
    sid = lax.axis_index("s")
    wid = cid * NS + sid

    def _f(i, _):
        zbuf_v[pl.ds(i * 16, 16)] = jnp.zeros((16,), jnp.float32)
        return 0
    lax.fori_loop(0, ASTR // 16, _f, 0)
    pltpu.sync_copy(zbuf_v, A_sh.at[pl.ds(sid * ASTR, ASTR)])
    plsc.subcore_barrier()

    def _chunk(k, _):
        cidx = wid + NW * k

        @pl.when(cidx < NCH_A)
        def _():
            base = cidx * KCH
            pltpu.sync_copy(asrc_hbm.at[pl.ds(base, KCH)], s_v)
            pltpu.sync_copy(adst_hbm.at[pl.ds(base, KCH)], d_v)
            pltpu.sync_copy(aval_hbm.at[pl.ds(base, KCH)], v_v)

            def _fi(g, _):
                sl = pl.ds(g * 16, 16)
                fidx_v[sl] = d_v[sl] * NSP_P + s_v[sl]
                return 0
            lax.fori_loop(0, KCH // 16, _fi, 0)
            pltpu.sync_copy(v_v, A_sh.at[fidx_v], add=True)
        return 0

    lax.fori_loop(0, KPW_A, _chunk, 0)

    @pl.when(wid == 0)
    def _tail():
        base = NCH_A * KCH
        pltpu.sync_copy(asrc_hbm.at[pl.ds(base, ATAIL)], st_v)
        pltpu.sync_copy(adst_hbm.at[pl.ds(base, ATAIL)], dt_v)
        pltpu.sync_copy(aval_hbm.at[pl.ds(base, ATAIL)], vt_v)

        def _fi(g, _):
            sl = pl.ds(g * 16, 16)
            fidxt_v[sl] = dt_v[sl] * NSP_P + st_v[sl]
            return 0
        lax.fori_loop(0, ATAIL // 16, _fi, 0)
        pltpu.sync_copy(vt_v, A_sh.at[fidxt_v], add=True)

    plsc.subcore_barrier()
    pltpu.sync_copy(A_sh.at[pl.ds(sid * ASTR, ASTR)], zbuf_v)
    pltpu.sync_copy(zbuf_v, A_hbm.at[cid, pl.ds(sid * ASTR, ASTR)])


def kernel(x, mask_index, mask_value, q_assign, a_index, a_value,
           prelin_W, prelin_b, bn0_g, bn0_b,
           sf_W, sf_b, sf_g, sf_be,
           pf_Wv, pf_bv, pf_Wq, pf_bq, pf_g, pf_be,
           out_W, out_b):
    bnscale = 1.0 / jnp.sqrt(1.0 + 1e-5)
    w1 = prelin_W * (bn0_g * bnscale)[:, None]
    b1 = prelin_b * (bn0_g * bnscale) + bn0_b

    src = mask_index[0]
    dst = mask_index[1]

    outPr = None
    for j in range(NPF):
        wcat = jnp.concatenate([pf_Wq[j, 0], pf_Wq[j, 1], pf_Wv[j, 0], pf_Wv[j, 1]], axis=0)
        bcat = jnp.concatenate([pf_bq[j, 0], pf_bq[j, 1], pf_bv[j, 0], pf_bv[j, 1]], axis=0)
        if j == 0:
            h0, qcat, vcat = _prelin_proj(x, w1.T, b1[None, :], wcat.T, bcat[None, :])
            x_pre = h0
        else:
            g0 = (pf_g[0] * bnscale)[None, :]
            b0 = pf_be[0][None, :]
            qcat, vcat = _mid(outPr, g0, b0, wcat.T, bcat[None, :])
        e2, sP = _edge_a(qcat, src, dst)
        outP = _edge_b(vcat[:, :DH], vcat[:, DH:], e2, sP, src, dst)
        outPr = outP.reshape(NC * HEAD, N, DH)

    hpP, degP = _pool(x_pre, q_assign)
    APf = _abuild(a_index[0], a_index[1], a_value)
    AP = APf.reshape(NC, NSP_P, NSP_P)
    hp_fin = _sf_dense(hpP, degP, AP, jnp.transpose(sf_W, (0, 2, 1)),
                       sf_b, sf_g * bnscale, sf_be)

    g1 = (pf_g[1] * bnscale)[None, :]
    b1f = pf_be[1][None, :]
    woutp = jnp.zeros((HIDE, 128), jnp.float32).at[:, :NCLS].set(out_W.T)
    boutp = jnp.full((128,), -1e30, jnp.float32).at[:NCLS].set(out_b)
    ypad = _final(outPr, g1, b1f, q_assign.reshape(N, 1), hp_fin,
                  woutp, boutp[None, :])
    return ypad[:, :NCLS]


# speedup vs baseline: 10.2683x; 1.1704x over previous
import functools

import jax
import jax.numpy as jnp
from jax import lax
from jax.experimental import pallas as pl
from jax.experimental.pallas import tpu as pltpu
from jax.experimental.pallas import tpu_sc as plsc

N = 10000
NSP = 500
C = 256
HIDE = 256
NCLS = 16
EM = 160000
EA = 8000
HEAD = 2
DH = HIDE // HEAD
NSF = 5
NPF = 2
RHP = 5
GAMA = 0.9

ROWS_BLK = 2000


def _prelin_proj_body(x_ref, w1_ref, b1_ref, wcat_ref, bcat_ref,
                      h0_ref, qcat_ref, vcat_ref):
    x = x_ref[...]
    h0 = jnp.dot(x, w1_ref[...], preferred_element_type=jnp.float32) + b1_ref[...]
    h0_ref[...] = h0
    cat = jnp.dot(h0, wcat_ref[...], preferred_element_type=jnp.float32) + bcat_ref[...]
    q = cat[:, :HIDE]
    v = cat[:, HIDE:]
    q2 = q.reshape(q.shape[0], HEAD, DH)
    m = jnp.mean(q2, axis=-1, keepdims=True)
    var = jnp.mean((q2 - m) ** 2, axis=-1, keepdims=True)
    qn = ((q2 - m) / jnp.sqrt(var + 1e-5)).reshape(q.shape)
    qcat_ref[...] = qn
    vcat_ref[...] = v


def _prelin_proj(x, w1t, b1, wcatt, bcat):
    grid = (N // ROWS_BLK,)
    return pl.pallas_call(
        _prelin_proj_body,
        grid=grid,
        in_specs=[
            pl.BlockSpec((ROWS_BLK, C), lambda i: (i, 0)),
            pl.BlockSpec((C, HIDE), lambda i: (0, 0)),
            pl.BlockSpec((1, HIDE), lambda i: (0, 0)),
            pl.BlockSpec((HIDE, 2 * HIDE), lambda i: (0, 0)),
            pl.BlockSpec((1, 2 * HIDE), lambda i: (0, 0)),
        ],
        out_specs=[
            pl.BlockSpec((ROWS_BLK, HIDE), lambda i: (i, 0)),
            pl.BlockSpec((ROWS_BLK, HIDE), lambda i: (i, 0)),
            pl.BlockSpec((ROWS_BLK, HIDE), lambda i: (i, 0)),
        ],
        out_shape=[
            jax.ShapeDtypeStruct((N, HIDE), jnp.float32),
            jax.ShapeDtypeStruct((N, HIDE), jnp.float32),
            jax.ShapeDtypeStruct((N, HIDE), jnp.float32),
        ],
    )(x, w1t, b1, wcatt, bcat)


def _leaky(x):
    return jnp.where(x >= 0, x, 0.01 * x)


NC = 2
NS = 16
NW = NC * NS
KCH = 128
NCHUNKS = EM // KCH
KPW = -(-NCHUNKS // NW)
NSP_S = 10240
SSTR = NSP_S // NS

_sc_mesh = plsc.VectorSubcoreMesh(core_axis_name="c", subcore_axis_name="s")


KA = 64
NCHA = EM // KA
KPA = -(-NCHA // NW)
KPA4 = ((KPA + 3) // 4) * 4


@functools.partial(
    pl.kernel,
    out_type=[
        jax.ShapeDtypeStruct((2, EM), jnp.float32),
        jax.ShapeDtypeStruct((NC, 2, NSP_S), jnp.float32),
    ],
    mesh=_sc_mesh,
    compiler_params=pltpu.CompilerParams(
        use_tc_tiling_on_sc=False, needs_layout_passes=False),
    scratch_types=[
        pltpu.VMEM((2, KA), jnp.int32),
        pltpu.VMEM((2, KA), jnp.int32),
        pltpu.VMEM((2, KA, HIDE), jnp.float32),
        pltpu.VMEM((2, KA, HIDE), jnp.float32),
        pltpu.VMEM((2, KA), jnp.float32),
        pltpu.VMEM((2, KA), jnp.float32),
        pltpu.VMEM((SSTR,), jnp.float32),
        pltpu.VMEM_SHARED((NSP_S,), jnp.float32),
        pltpu.VMEM_SHARED((NSP_S,), jnp.float32),
        [pltpu.SemaphoreType.DMA] * 2,
        [pltpu.SemaphoreType.DMA] * 2,
    ],
)
def _edge_a(qcat_hbm, src_hbm, dst_hbm, e_hbm, s_hbm,
            src_v, dst_v, qsrc_v, qdst_v, e0_v, e1_v, buf_v,
            s0_sh, s1_sh, si, sg):
    cid = lax.axis_index("c")
    sid = lax.axis_index("s")
    wid = cid * NS + sid

    def _zero(i, _):
        buf_v[pl.ds(i * 16, 16)] = jnp.zeros((16,), jnp.float32)
        return 0
    lax.fori_loop(0, SSTR // 16, _zero, 0)
    pltpu.sync_copy(buf_v, s0_sh.at[pl.ds(sid * SSTR, SSTR)])
    pltpu.sync_copy(buf_v, s1_sh.at[pl.ds(sid * SSTR, SSTR)])
    plsc.subcore_barrier()

    def _issue_idx(c, b4):
        @pl.when(c < NCHA)
        def _():
            base = c * KA
            pltpu.async_copy(src_hbm.at[pl.ds(base, KA)], src_v.at[b4], si[b4])
            pltpu.async_copy(dst_hbm.at[pl.ds(base, KA)], dst_v.at[b4], si[b4])

    def _issue_gather(c, b4, b2):
        @pl.when(c < NCHA)
        def _():
            base = c * KA
            pltpu.make_async_copy(src_hbm.at[pl.ds(base, KA)], src_v.at[b4], si[b4]).wait()
            pltpu.make_async_copy(dst_hbm.at[pl.ds(base, KA)], dst_v.at[b4], si[b4]).wait()
            pltpu.async_copy(qcat_hbm.at[src_v.at[b4]], qsrc_v.at[b2], sg[b2])
            pltpu.async_copy(qcat_hbm.at[dst_v.at[b4]], qdst_v.at[b2], sg[b2])

    def _compute(c, b4, b2):
        @pl.when(c < NCHA)
        def _():
            base = c * KA
            pltpu.make_async_copy(qcat_hbm.at[src_v.at[b4]], qsrc_v.at[b2], sg[b2]).wait()
            pltpu.make_async_copy(qcat_hbm.at[dst_v.at[b4]], qdst_v.at[b2], sg[b2]).wait()
            lastlane = lax.iota(jnp.int32, 16) == 15

            def _edge(ei, _):
                z = jnp.zeros((16,), jnp.float32)
                a0 = z
                a1 = z
                for cc in range(DH // 16):
                    a0 = a0 + qsrc_v[b2, ei, pl.ds(cc * 16, 16)] * qdst_v[b2, ei, pl.ds(cc * 16, 16)]
                    a1 = a1 + qsrc_v[b2, ei, pl.ds(DH + cc * 16, 16)] * qdst_v[b2, ei, pl.ds(DH + cc * 16, 16)]
                eidx = jnp.full((16,), ei, jnp.int32)
                plsc.store_scatter(e0_v.at[b4], [eidx], plsc.cumsum(a0), mask=lastlane)
                plsc.store_scatter(e1_v.at[b4], [eidx], plsc.cumsum(a1), mask=lastlane)
                return 0

            lax.fori_loop(0, KA, _edge, 0, unroll=2)

            def _vexp(g, _):
                sl = pl.ds(g * 16, 16)
                e0_v[b4, sl] = jnp.exp(e0_v[b4, sl] * (1.0 / HIDE))
                e1_v[b4, sl] = jnp.exp(e1_v[b4, sl] * (1.0 / HIDE))
                return 0

            lax.fori_loop(0, KA // 16, _vexp, 0)
            pltpu.sync_copy(e0_v.at[b4], e_hbm.at[0, pl.ds(base, KA)])
            pltpu.sync_copy(e1_v.at[b4], e_hbm.at[1, pl.ds(base, KA)])
            pltpu.sync_copy(e0_v.at[b4], s0_sh.at[src_v.at[b4]], add=True)
            pltpu.sync_copy(e1_v.at[b4], s1_sh.at[src_v.at[b4]], add=True)

    _issue_idx(wid, 0)
    _issue_idx(wid + NW, 1)
    _issue_gather(wid, 0, 0)

    def _pair(k2, _):
        for b in range(2):
            k = 2 * k2 + b
            c = wid + NW * k
            _issue_gather(c + NW, 1 - b, 1 - b)
            _compute(c, b, b)
            _issue_idx(c + 2 * NW, b)
        return 0

    lax.fori_loop(0, KPA4 // 2, _pair, 0)
    plsc.subcore_barrier()

    pltpu.sync_copy(s0_sh.at[pl.ds(sid * SSTR, SSTR)], buf_v)
    pltpu.sync_copy(buf_v, s_hbm.at[cid, 0, pl.ds(sid * SSTR, SSTR)])
    pltpu.sync_copy(s1_sh.at[pl.ds(sid * SSTR, SSTR)], buf_v)
    pltpu.sync_copy(buf_v, s_hbm.at[cid, 1, pl.ds(sid * SSTR, SSTR)])


NSP_P = 512


def _mid_body(o_ref, g_ref, b_ref, wcat_ref, bcat_ref, qcat_ref, vcat_ref):
    o = o_ref[...]
    h = jnp.concatenate([o[0] + o[2], o[1] + o[3]], axis=1)
    h = h * g_ref[...] + b_ref[...]
    h = jnp.where(h >= 0, h, 0.01 * h)
    cat = jnp.dot(h, wcat_ref[...], preferred_element_type=jnp.float32) + bcat_ref[...]
    q = cat[:, :HIDE]
    q2 = q.reshape(q.shape[0], HEAD, DH)
    m = jnp.mean(q2, axis=-1, keepdims=True)
    var = jnp.mean((q2 - m) ** 2, axis=-1, keepdims=True)
    qcat_ref[...] = ((q2 - m) / jnp.sqrt(var + 1e-5)).reshape(q.shape)
    vcat_ref[...] = cat[:, HIDE:]


def _mid(outPr, g, b, wcatt, bcat):
    return pl.pallas_call(
        _mid_body,
        grid=(N // ROWS_BLK,),
        in_specs=[
            pl.BlockSpec((4, ROWS_BLK, DH), lambda i: (0, i, 0)),
            pl.BlockSpec((1, HIDE), lambda i: (0, 0)),
            pl.BlockSpec((1, HIDE), lambda i: (0, 0)),
            pl.BlockSpec((HIDE, 2 * HIDE), lambda i: (0, 0)),
            pl.BlockSpec((1, 2 * HIDE), lambda i: (0, 0)),
        ],
        out_specs=[
            pl.BlockSpec((ROWS_BLK, HIDE), lambda i: (i, 0)),
            pl.BlockSpec((ROWS_BLK, HIDE), lambda i: (i, 0)),
        ],
        out_shape=[
            jax.ShapeDtypeStruct((N, HIDE), jnp.float32),
            jax.ShapeDtypeStruct((N, HIDE), jnp.float32),
        ],
    )(outPr, g, b, wcatt, bcat)


def _sf_body(hpP_ref, degP_ref, AP_ref, wt_ref, b_ref, g_ref, be_ref, out_ref):
    degP = degP_ref[...]
    deg = degP[0] + degP[1]
    inv = jnp.where(deg > 0, 1.0 / deg, 0.0)
    hp = (hpP_ref[0] + hpP_ref[1]) * inv[:, None]
    A = AP_ref[0] + AP_ref[1]
    for i in range(NSF):
        hlin = jnp.dot(hp, wt_ref[i], preferred_element_type=jnp.float32) + b_ref[i]
        x_start = jnp.dot(A, hlin, preferred_element_type=jnp.float32)
        xc = hp
        for _ in range(RHP):
            xc = (jnp.dot(A, xc, preferred_element_type=jnp.float32) + x_start) * (1.0 / (2.0 + GAMA))
        hp = xc * g_ref[i] + be_ref[i]
        hp = jnp.where(hp >= 0, hp, 0.01 * hp)
    out_ref[...] = hp


def _sf_dense(hpP, degP, AP, wt, b, g, be):
    return pl.pallas_call(
        _sf_body,
        out_shape=jax.ShapeDtypeStruct((NSP_P, HIDE), jnp.float32),
    )(hpP, degP, AP, wt, b, g, be)


def _fin_body(o_ref, g_ref, b_ref, qa_ref, hp_ref, wout_ref, bout_ref, y_ref):
    o = o_ref[...]
    h = jnp.concatenate([o[0] + o[2], o[1] + o[3]], axis=1)
    h = h * g_ref[...] + b_ref[...]
    pf = jnp.where(h >= 0, h, 0.01 * h)
    qa = qa_ref[...]
    oh = (qa == jax.lax.broadcasted_iota(jnp.int32, (qa.shape[0], NSP_P), 1)
          ).astype(jnp.float32)
    sf = jnp.dot(oh, hp_ref[...], preferred_element_type=jnp.float32)
    H1 = pf + sf
    logits = jnp.dot(H1, wout_ref[...], preferred_element_type=jnp.float32) + bout_ref[...]
    mx = jnp.max(logits, axis=-1, keepdims=True)
    e = jnp.exp(logits - mx)
    y_ref[...] = e / jnp.sum(e, axis=-1, keepdims=True)


def _final(outPr, g, b, qa2, hp, wout, bout):
    return pl.pallas_call(
        _fin_body,
        grid=(N // ROWS_BLK,),
        in_specs=[
            pl.BlockSpec((4, ROWS_BLK, DH), lambda i: (0, i, 0)),
            pl.BlockSpec((1, HIDE), lambda i: (0, 0)),
            pl.BlockSpec((1, HIDE), lambda i: (0, 0)),
            pl.BlockSpec((ROWS_BLK, 1), lambda i: (i, 0)),
            pl.BlockSpec((NSP_P, HIDE), lambda i: (0, 0)),
            pl.BlockSpec((HIDE, 128), lambda i: (0, 0)),
            pl.BlockSpec((1, 128), lambda i: (0, 0)),
        ],
        out_specs=pl.BlockSpec((ROWS_BLK, 128), lambda i: (i, 0)),
        out_shape=jax.ShapeDtypeStruct((N, 128), jnp.float32),
    )(outPr, g, b, qa2, hp, wout, bout)


RPW = N // NS
RPP = 125


@functools.partial(
    pl.kernel,
    out_type=jax.ShapeDtypeStruct((NC, HEAD, N, DH), jnp.float32),
    mesh=_sc_mesh,
    compiler_params=pltpu.CompilerParams(
        use_tc_tiling_on_sc=False, needs_layout_passes=False),
    scratch_types=[
        pltpu.VMEM((KCH,), jnp.int32),
        pltpu.VMEM((KCH,), jnp.int32),
        pltpu.VMEM((KCH, DH), jnp.float32),
        pltpu.VMEM((KCH,), jnp.float32),
        pltpu.VMEM((KCH,), jnp.float32),
        pltpu.VMEM((NSP_S,), jnp.float32),
        pltpu.VMEM((NSP_S,), jnp.float32),
        pltpu.VMEM_SHARED((N, DH), jnp.float32),
        pltpu.SemaphoreType.DMA,
    ],
)
def _edge_b(v0_hbm, v1_hbm, e_hbm, s_hbm, src_hbm, dst_hbm, out_hbm,
            src_v, dst_v, vrow_v, e_v, w_v, sA_v, sB_v, out_sh, sem1):
    cid = lax.axis_index("c")
    sid = lax.axis_index("s")
    wid = cid * NS + sid

    for hh in range(HEAD):
        v_hbm = v0_hbm if hh == 0 else v1_hbm
        pltpu.sync_copy(s_hbm.at[0, hh], sA_v)
        pltpu.sync_copy(s_hbm.at[1, hh], sB_v)

        def _sadd(i, _):
            sA_v[pl.ds(i * 16, 16)] = sA_v[pl.ds(i * 16, 16)] + sB_v[pl.ds(i * 16, 16)]
            return 0
        lax.fori_loop(0, NSP_S // 16, _sadd, 0)

        def _zrow2(i, _):
            def _zc(c, _):
                vrow_v[i, pl.ds(c * 16, 16)] = jnp.zeros((16,), jnp.float32)
                return 0
            lax.fori_loop(0, DH // 16, _zc, 0)
            return 0
        lax.fori_loop(0, KCH, _zrow2, 0)
        for p in range(RPW // RPP):
            pltpu.sync_copy(vrow_v.at[pl.ds(0, RPP), :],
                            out_sh.at[pl.ds(sid * RPW + p * RPP, RPP), :])
        plsc.subcore_barrier()

        def _chunk(k, _):
            cidx = wid + NW * k

            @pl.when(cidx < NCHUNKS)
            def _():
                base = cidx * KCH
                pltpu.sync_copy(src_hbm.at[pl.ds(base, KCH)], src_v)
                pltpu.sync_copy(dst_hbm.at[pl.ds(base, KCH)], dst_v)
                pltpu.sync_copy(e_hbm.at[hh, pl.ds(base, KCH)], e_v)
                pltpu.async_copy(v_hbm.at[dst_v], vrow_v, sem1).wait()

                def _wgrp(g, _):
                    sl = pl.ds(g * 16, 16)
                    sg = plsc.load_gather(sA_v, [src_v[sl]])
                    w_v[sl] = e_v[sl] / sg
                    return 0
                lax.fori_loop(0, KCH // 16, _wgrp, 0)

                def _scale(ei, _):
                    wb = plsc.load_gather(w_v, [jnp.full((16,), ei, jnp.int32)])
                    for c in range(DH // 16):
                        vrow_v[ei, pl.ds(c * 16, 16)] = vrow_v[ei, pl.ds(c * 16, 16)] * wb
                    return 0
                lax.fori_loop(0, KCH, _scale, 0, unroll=2)

                pltpu.sync_copy(vrow_v, out_sh.at[src_v], add=True)
            return 0

        lax.fori_loop(0, KPW, _chunk, 0)
        plsc.subcore_barrier()

        for p in range(RPW // RPP):
            r0 = sid * RPW + p * RPP
            pltpu.sync_copy(out_sh.at[pl.ds(r0, RPP), :], vrow_v.at[pl.ds(0, RPP), :])
            pltpu.sync_copy(vrow_v.at[pl.ds(0, RPP), :],
                            out_hbm.at[cid, hh, pl.ds(r0, RPP), :])
        if hh == 0:
            plsc.subcore_barrier()


NCH_P = N // KCH
PTAIL = N - NCH_P * KCH
KPW_P = -(-NCH_P // NW)
SPSTR = NSP_P // NS


@functools.partial(
    pl.kernel,
    out_type=[
        jax.ShapeDtypeStruct((NC, NSP_P, HIDE), jnp.float32),
        jax.ShapeDtypeStruct((NC, NSP_P), jnp.float32),
    ],
    mesh=_sc_mesh,
    compiler_params=pltpu.CompilerParams(
        use_tc_tiling_on_sc=False, needs_layout_passes=False),
    scratch_types=[
        pltpu.VMEM((KCH,), jnp.int32),
        pltpu.VMEM((KCH, HIDE), jnp.float32),
        pltpu.VMEM((KCH,), jnp.float32),
        pltpu.VMEM((PTAIL,), jnp.int32),
        pltpu.VMEM((PTAIL, HIDE), jnp.float32),
        pltpu.VMEM((PTAIL,), jnp.float32),
        pltpu.VMEM((SPSTR,), jnp.float32),
        pltpu.VMEM_SHARED((NSP_P, HIDE), jnp.float32),
        pltpu.VMEM_SHARED((NSP_P,), jnp.float32),
    ],
)
def _pool(x_hbm, qa_hbm, hp_hbm, deg_hbm,
          qa_v, rows_v, ones_v, qat_v, rowst_v, onest_v, buf_v,
          hp_sh, deg_sh):
    cid = lax.axis_index("c")
    sid = lax.axis_index("s")
    wid = cid * NS + sid

    def _fill(ref, n, val):
        def _f(i, _):
            ref[pl.ds(i * 16, 16)] = jnp.full((16,), val, jnp.float32)
            return 0
        lax.fori_loop(0, n // 16, _f, 0)

    _fill(ones_v, KCH, 1.0)
    _fill(onest_v, PTAIL, 1.0)
    _fill(buf_v, SPSTR, 0.0)

    def _zrow(i, _):
        def _zc(c, _):
            rows_v[i, pl.ds(c * 16, 16)] = jnp.zeros((16,), jnp.float32)
            return 0
        lax.fori_loop(0, HIDE // 16, _zc, 0)
        return 0
    lax.fori_loop(0, SPSTR, _zrow, 0)
    pltpu.sync_copy(rows_v.at[pl.ds(0, SPSTR), :],
                    hp_sh.at[pl.ds(sid * SPSTR, SPSTR), :])
    pltpu.sync_copy(buf_v, deg_sh.at[pl.ds(sid * SPSTR, SPSTR)])
    plsc.subcore_barrier()

    def _chunk(k, _):
        cidx = wid + NW * k

        @pl.when(cidx < NCH_P)
        def _():
            base = cidx * KCH
            pltpu.sync_copy(qa_hbm.at[pl.ds(base, KCH)], qa_v)
            pltpu.sync_copy(x_hbm.at[pl.ds(base, KCH), :], rows_v)
            pltpu.sync_copy(rows_v, hp_sh.at[qa_v], add=True)
            pltpu.sync_copy(ones_v, deg_sh.at[qa_v], add=True)
        return 0

    lax.fori_loop(0, KPW_P, _chunk, 0)

    @pl.when(wid == 0)
    def _tail():
        base = NCH_P * KCH
        pltpu.sync_copy(qa_hbm.at[pl.ds(base, PTAIL)], qat_v)
        pltpu.sync_copy(x_hbm.at[pl.ds(base, PTAIL), :], rowst_v)
        pltpu.sync_copy(rowst_v, hp_sh.at[qat_v], add=True)
        pltpu.sync_copy(onest_v, deg_sh.at[qat_v], add=True)

    plsc.subcore_barrier()
    r0 = sid * SPSTR
    pltpu.sync_copy(hp_sh.at[pl.ds(r0, SPSTR), :], rows_v.at[pl.ds(0, SPSTR), :])
    pltpu.sync_copy(rows_v.at[pl.ds(0, SPSTR), :], hp_hbm.at[cid, pl.ds(r0, SPSTR), :])
    pltpu.sync_copy(deg_sh.at[pl.ds(r0, SPSTR)], buf_v)
    pltpu.sync_copy(buf_v, deg_hbm.at[cid, pl.ds(r0, SPSTR)])


NCH_A = EA // KCH
ATAIL = EA - NCH_A * KCH
KPW_A = -(-NCH_A // NW)
AFL = NSP_P * NSP_P
ASTR = AFL // NS


@functools.partial(
    pl.kernel,
    out_type=jax.ShapeDtypeStruct((NC, AFL), jnp.float32),
    mesh=_sc_mesh,
    compiler_params=pltpu.CompilerParams(
        use_tc_tiling_on_sc=False, needs_layout_passes=False),
    scratch_types=[
        pltpu.VMEM((KCH,), jnp.int32),
        pltpu.VMEM((KCH,), jnp.int32),
        pltpu.VMEM((KCH,), jnp.float32),
        pltpu.VMEM((KCH,), jnp.int32),
        pltpu.VMEM((ATAIL,), jnp.int32),
        pltpu.VMEM((ATAIL,), jnp.int32),
        pltpu.VMEM((ATAIL,), jnp.float32),
        pltpu.VMEM((ATAIL,), jnp.int32),
        pltpu.VMEM((ASTR,), jnp.float32),
        pltpu.VMEM_SHARED((AFL,), jnp.float32),
    ],
)
def _abuild(asrc_hbm, adst_hbm, aval_hbm, A_hbm,
            s_v, d_v, v_v, fidx_v, st_v, dt_v, vt_v, fidxt_v, zbuf_v, A_sh):
    cid = lax.axis_index("c")
    sid = lax.axis_index("s")
    wid = cid * NS + sid

    def _f(i, _):
        zbuf_v[pl.ds(i * 16, 16)] = jnp.zeros((16,), jnp.float32)
        return 0
    lax.fori_loop(0, ASTR // 16, _f, 0)
    pltpu.sync_copy(zbuf_v, A_sh.at[pl.ds(sid * ASTR, ASTR)])
    plsc.subcore_barrier()

    def _chunk(k, _):
        cidx = wid + NW * k

        @pl.when(cidx < NCH_A)
        def _():
            base = cidx * KCH
            pltpu.sync_copy(asrc_hbm.at[pl.ds(base, KCH)], s_v)
            pltpu.sync_copy(adst_hbm.at[pl.ds(base, KCH)], d_v)
            pltpu.sync_copy(aval_hbm.at[pl.ds(base, KCH)], v_v)

            def _fi(g, _):
                sl = pl.ds(g * 16, 16)
                fidx_v[sl] = d_v[sl] * NSP_P + s_v[sl]
                return 0
            lax.fori_loop(0, KCH // 16, _fi, 0)
            pltpu.sync_copy(v_v, A_sh.at[fidx_v], add=True)
        return 0

    lax.fori_loop(0, KPW_A, _chunk, 0)

    @pl.when(wid == 0)
    def _tail():
        base = NCH_A * KCH
        pltpu.sync_copy(asrc_hbm.at[pl.ds(base, ATAIL)], st_v)
        pltpu.sync_copy(adst_hbm.at[pl.ds(base, ATAIL)], dt_v)
        pltpu.sync_copy(aval_hbm.at[pl.ds(base, ATAIL)], vt_v)

        def _fi(g, _):
            sl = pl.ds(g * 16, 16)
            fidxt_v[sl] = dt_v[sl] * NSP_P + st_v[sl]
            return 0
        lax.fori_loop(0, ATAIL // 16, _fi, 0)
        pltpu.sync_copy(vt_v, A_sh.at[fidxt_v], add=True)

    plsc.subcore_barrier()
    pltpu.sync_copy(A_sh.at[pl.ds(sid * ASTR, ASTR)], zbuf_v)
    pltpu.sync_copy(zbuf_v, A_hbm.at[cid, pl.ds(sid * ASTR, ASTR)])


def kernel(x, mask_index, mask_value, q_assign, a_index, a_value,
           prelin_W, prelin_b, bn0_g, bn0_b,
           sf_W, sf_b, sf_g, sf_be,
           pf_Wv, pf_bv, pf_Wq, pf_bq, pf_g, pf_be,
           out_W, out_b):
    bnscale = 1.0 / jnp.sqrt(1.0 + 1e-5)
    w1 = prelin_W * (bn0_g * bnscale)[:, None]
    b1 = prelin_b * (bn0_g * bnscale) + bn0_b

    src = mask_index[0]
    dst = mask_index[1]

    outPr = None
    for j in range(NPF):
        wcat = jnp.concatenate([pf_Wq[j, 0], pf_Wq[j, 1], pf_Wv[j, 0], pf_Wv[j, 1]], axis=0)
        bcat = jnp.concatenate([pf_bq[j, 0], pf_bq[j, 1], pf_bv[j, 0], pf_bv[j, 1]], axis=0)
        if j == 0:
            h0, qcat, vcat = _prelin_proj(x, w1.T, b1[None, :], wcat.T, bcat[None, :])
            x_pre = h0
        else:
            g0 = (pf_g[0] * bnscale)[None, :]
            b0 = pf_be[0][None, :]
            qcat, vcat = _mid(outPr, g0, b0, wcat.T, bcat[None, :])
        e2, sP = _edge_a(qcat, src, dst)
        outP = _edge_b(vcat[:, :DH], vcat[:, DH:], e2, sP, src, dst)
        outPr = outP.reshape(NC * HEAD, N, DH)

    hpP, degP = _pool(x_pre, q_assign)
    APf = _abuild(a_index[0], a_index[1], a_value)
    AP = APf.reshape(NC, NSP_P, NSP_P)
    hp_fin = _sf_dense(hpP, degP, AP, jnp.transpose(sf_W, (0, 2, 1)),
                       sf_b, sf_g * bnscale, sf_be)

    g1 = (pf_g[1] * bnscale)[None, :]
    b1f = pf_be[1][None, :]
    woutp = jnp.zeros((HIDE, 128), jnp.float32).at[:, :NCLS].set(out_W.T)
    boutp = jnp.full((128,), -1e30, jnp.float32).at[:NCLS].set(out_b)
    ypad = _final(outPr, g1, b1f, q_assign.reshape(N, 1), hp_fin,
                  woutp, boutp[None, :])
    return ypad[:, :NCLS]


# speedup vs baseline: 11.2471x; 1.0953x over previous
import functools

import jax
import jax.numpy as jnp
from jax import lax
from jax.experimental import pallas as pl
from jax.experimental.pallas import tpu as pltpu
from jax.experimental.pallas import tpu_sc as plsc

N = 10000
NSP = 500
C = 256
HIDE = 256
NCLS = 16
EM = 160000
EA = 8000
HEAD = 2
DH = HIDE // HEAD
NSF = 5
NPF = 2
RHP = 5
GAMA = 0.9

ROWS_BLK = 2000


def _prelin_proj_body(x_ref, w1_ref, b1_ref, wcat_ref, bcat_ref,
                      h0_ref, qcat_ref, vcat_ref):
    x = x_ref[...]
    h0 = jnp.dot(x, w1_ref[...], preferred_element_type=jnp.float32) + b1_ref[...]
    h0_ref[...] = h0
    cat = jnp.dot(h0, wcat_ref[...], preferred_element_type=jnp.float32) + bcat_ref[...]
    q = cat[:, :HIDE]
    v = cat[:, HIDE:]
    q2 = q.reshape(q.shape[0], HEAD, DH)
    m = jnp.mean(q2, axis=-1, keepdims=True)
    var = jnp.mean((q2 - m) ** 2, axis=-1, keepdims=True)
    qn = ((q2 - m) / jnp.sqrt(var + 1e-5)).reshape(q.shape)
    qcat_ref[...] = qn
    vcat_ref[...] = v


def _prelin_proj(x, w1t, b1, wcatt, bcat):
    grid = (N // ROWS_BLK,)
    return pl.pallas_call(
        _prelin_proj_body,
        grid=grid,
        in_specs=[
            pl.BlockSpec((ROWS_BLK, C), lambda i: (i, 0)),
            pl.BlockSpec((C, HIDE), lambda i: (0, 0)),
            pl.BlockSpec((1, HIDE), lambda i: (0, 0)),
            pl.BlockSpec((HIDE, 2 * HIDE), lambda i: (0, 0)),
            pl.BlockSpec((1, 2 * HIDE), lambda i: (0, 0)),
        ],
        out_specs=[
            pl.BlockSpec((ROWS_BLK, HIDE), lambda i: (i, 0)),
            pl.BlockSpec((ROWS_BLK, HIDE), lambda i: (i, 0)),
            pl.BlockSpec((ROWS_BLK, HIDE), lambda i: (i, 0)),
        ],
        out_shape=[
            jax.ShapeDtypeStruct((N, HIDE), jnp.float32),
            jax.ShapeDtypeStruct((N, HIDE), jnp.float32),
            jax.ShapeDtypeStruct((N, HIDE), jnp.float32),
        ],
    )(x, w1t, b1, wcatt, bcat)


def _leaky(x):
    return jnp.where(x >= 0, x, 0.01 * x)


NC = 2
NS = 16
NW = NC * NS
KCH = 128
NCHUNKS = EM // KCH
KPW = -(-NCHUNKS // NW)
NSP_S = 10240
SSTR = NSP_S // NS

_sc_mesh = plsc.VectorSubcoreMesh(core_axis_name="c", subcore_axis_name="s")


KA = 64
NCHA = EM // KA
KPA = -(-NCHA // NW)
KPA4 = ((KPA + 3) // 4) * 4


@functools.partial(
    pl.kernel,
    out_type=[
        jax.ShapeDtypeStruct((2, EM), jnp.float32),
        jax.ShapeDtypeStruct((NC, 2, NSP_S), jnp.float32),
    ],
    mesh=_sc_mesh,
    compiler_params=pltpu.CompilerParams(
        use_tc_tiling_on_sc=False, needs_layout_passes=False),
    scratch_types=[
        pltpu.VMEM((2, KA), jnp.int32),
        pltpu.VMEM((2, KA), jnp.int32),
        pltpu.VMEM((2, KA, HIDE), jnp.float32),
        pltpu.VMEM((2, KA, HIDE), jnp.float32),
        pltpu.VMEM((2, KA), jnp.float32),
        pltpu.VMEM((2, KA), jnp.float32),
        pltpu.VMEM((SSTR,), jnp.float32),
        pltpu.VMEM_SHARED((NSP_S,), jnp.float32),
        pltpu.VMEM_SHARED((NSP_S,), jnp.float32),
        [pltpu.SemaphoreType.DMA] * 2,
        [pltpu.SemaphoreType.DMA] * 2,
    ],
)
def _edge_a(qcat_hbm, src_hbm, dst_hbm, e_hbm, s_hbm,
            src_v, dst_v, qsrc_v, qdst_v, e0_v, e1_v, buf_v,
            s0_sh, s1_sh, si, sg):
    cid = lax.axis_index("c")
    sid = lax.axis_index("s")
    wid = cid * NS + sid

    def _zero(i, _):
        buf_v[pl.ds(i * 16, 16)] = jnp.zeros((16,), jnp.float32)
        return 0
    lax.fori_loop(0, SSTR // 16, _zero, 0)
    pltpu.sync_copy(buf_v, s0_sh.at[pl.ds(sid * SSTR, SSTR)])
    pltpu.sync_copy(buf_v, s1_sh.at[pl.ds(sid * SSTR, SSTR)])
    plsc.subcore_barrier()

    def _issue_idx(c, b4):
        @pl.when(c < NCHA)
        def _():
            base = c * KA
            pltpu.async_copy(src_hbm.at[pl.ds(base, KA)], src_v.at[b4], si[b4])
            pltpu.async_copy(dst_hbm.at[pl.ds(base, KA)], dst_v.at[b4], si[b4])

    def _issue_gather(c, b4, b2):
        @pl.when(c < NCHA)
        def _():
            base = c * KA
            pltpu.make_async_copy(src_hbm.at[pl.ds(base, KA)], src_v.at[b4], si[b4]).wait()
            pltpu.make_async_copy(dst_hbm.at[pl.ds(base, KA)], dst_v.at[b4], si[b4]).wait()
            pltpu.async_copy(qcat_hbm.at[src_v.at[b4]], qsrc_v.at[b2], sg[b2])
            pltpu.async_copy(qcat_hbm.at[dst_v.at[b4]], qdst_v.at[b2], sg[b2])

    def _compute(c, b4, b2):
        @pl.when(c < NCHA)
        def _():
            base = c * KA
            pltpu.make_async_copy(qcat_hbm.at[src_v.at[b4]], qsrc_v.at[b2], sg[b2]).wait()
            pltpu.make_async_copy(qcat_hbm.at[dst_v.at[b4]], qdst_v.at[b2], sg[b2]).wait()
            lastlane = lax.iota(jnp.int32, 16) == 15

            def _edge(ei, _):
                z = jnp.zeros((16,), jnp.float32)
                a0 = z
                a1 = z
                for cc in range(DH // 16):
                    a0 = a0 + qsrc_v[b2, ei, pl.ds(cc * 16, 16)] * qdst_v[b2, ei, pl.ds(cc * 16, 16)]
                    a1 = a1 + qsrc_v[b2, ei, pl.ds(DH + cc * 16, 16)] * qdst_v[b2, ei, pl.ds(DH + cc * 16, 16)]
                eidx = jnp.full((16,), ei, jnp.int32)
                plsc.store_scatter(e0_v.at[b4], [eidx], plsc.cumsum(a0), mask=lastlane)
                plsc.store_scatter(e1_v.at[b4], [eidx], plsc.cumsum(a1), mask=lastlane)
                return 0

            lax.fori_loop(0, KA, _edge, 0, unroll=2)

            def _vexp(g, _):
                sl = pl.ds(g * 16, 16)
                e0_v[b4, sl] = jnp.exp(e0_v[b4, sl] * (1.0 / HIDE))
                e1_v[b4, sl] = jnp.exp(e1_v[b4, sl] * (1.0 / HIDE))
                return 0

            lax.fori_loop(0, KA // 16, _vexp, 0)
            pltpu.sync_copy(e0_v.at[b4], e_hbm.at[0, pl.ds(base, KA)])
            pltpu.sync_copy(e1_v.at[b4], e_hbm.at[1, pl.ds(base, KA)])
            pltpu.sync_copy(e0_v.at[b4], s0_sh.at[src_v.at[b4]], add=True)
            pltpu.sync_copy(e1_v.at[b4], s1_sh.at[src_v.at[b4]], add=True)

    _issue_idx(wid, 0)
    _issue_idx(wid + NW, 1)
    _issue_gather(wid, 0, 0)

    def _pair(k2, _):
        for b in range(2):
            k = 2 * k2 + b
            c = wid + NW * k
            _issue_gather(c + NW, 1 - b, 1 - b)
            _compute(c, b, b)
            _issue_idx(c + 2 * NW, b)
        return 0

    lax.fori_loop(0, KPA4 // 2, _pair, 0)
    plsc.subcore_barrier()

    pltpu.sync_copy(s0_sh.at[pl.ds(sid * SSTR, SSTR)], buf_v)
    pltpu.sync_copy(buf_v, s_hbm.at[cid, 0, pl.ds(sid * SSTR, SSTR)])
    pltpu.sync_copy(s1_sh.at[pl.ds(sid * SSTR, SSTR)], buf_v)
    pltpu.sync_copy(buf_v, s_hbm.at[cid, 1, pl.ds(sid * SSTR, SSTR)])


NSP_P = 512


def _mid_body(o_ref, g_ref, b_ref, wcat_ref, bcat_ref, qcat_ref, vcat_ref):
    o = o_ref[...]
    h = jnp.concatenate([o[0] + o[2], o[1] + o[3]], axis=1)
    h = h * g_ref[...] + b_ref[...]
    h = jnp.where(h >= 0, h, 0.01 * h)
    cat = jnp.dot(h, wcat_ref[...], preferred_element_type=jnp.float32) + bcat_ref[...]
    q = cat[:, :HIDE]
    q2 = q.reshape(q.shape[0], HEAD, DH)
    m = jnp.mean(q2, axis=-1, keepdims=True)
    var = jnp.mean((q2 - m) ** 2, axis=-1, keepdims=True)
    qcat_ref[...] = ((q2 - m) / jnp.sqrt(var + 1e-5)).reshape(q.shape)
    vcat_ref[...] = cat[:, HIDE:]


def _mid(outPr, g, b, wcatt, bcat):
    return pl.pallas_call(
        _mid_body,
        grid=(N // ROWS_BLK,),
        in_specs=[
            pl.BlockSpec((4, ROWS_BLK, DH), lambda i: (0, i, 0)),
            pl.BlockSpec((1, HIDE), lambda i: (0, 0)),
            pl.BlockSpec((1, HIDE), lambda i: (0, 0)),
            pl.BlockSpec((HIDE, 2 * HIDE), lambda i: (0, 0)),
            pl.BlockSpec((1, 2 * HIDE), lambda i: (0, 0)),
        ],
        out_specs=[
            pl.BlockSpec((ROWS_BLK, HIDE), lambda i: (i, 0)),
            pl.BlockSpec((ROWS_BLK, HIDE), lambda i: (i, 0)),
        ],
        out_shape=[
            jax.ShapeDtypeStruct((N, HIDE), jnp.float32),
            jax.ShapeDtypeStruct((N, HIDE), jnp.float32),
        ],
    )(outPr, g, b, wcatt, bcat)


def _sf_body(hpP_ref, degP_ref, AP_ref, wt_ref, b_ref, g_ref, be_ref, out_ref):
    degP = degP_ref[...]
    deg = degP[0] + degP[1]
    inv = jnp.where(deg > 0, 1.0 / deg, 0.0)
    hp = (hpP_ref[0] + hpP_ref[1]) * inv[:, None]
    A = AP_ref[0] + AP_ref[1]
    for i in range(NSF):
        hlin = jnp.dot(hp, wt_ref[i], preferred_element_type=jnp.float32) + b_ref[i]
        x_start = jnp.dot(A, hlin, preferred_element_type=jnp.float32)
        xc = hp
        for _ in range(RHP):
            xc = (jnp.dot(A, xc, preferred_element_type=jnp.float32) + x_start) * (1.0 / (2.0 + GAMA))
        hp = xc * g_ref[i] + be_ref[i]
        hp = jnp.where(hp >= 0, hp, 0.01 * hp)
    out_ref[...] = hp


def _sf_dense(hpP, degP, AP, wt, b, g, be):
    return pl.pallas_call(
        _sf_body,
        out_shape=jax.ShapeDtypeStruct((NSP_P, HIDE), jnp.float32),
    )(hpP, degP, AP, wt, b, g, be)


def _fin_body(o_ref, g_ref, b_ref, qa_ref, hp_ref, wout_ref, bout_ref, y_ref):
    o = o_ref[...]
    h = jnp.concatenate([o[0] + o[2], o[1] + o[3]], axis=1)
    h = h * g_ref[...] + b_ref[...]
    pf = jnp.where(h >= 0, h, 0.01 * h)
    qa = qa_ref[...]
    oh = (qa == jax.lax.broadcasted_iota(jnp.int32, (qa.shape[0], NSP_P), 1)
          ).astype(jnp.float32)
    sf = jnp.dot(oh, hp_ref[...], preferred_element_type=jnp.float32)
    H1 = pf + sf
    logits = jnp.dot(H1, wout_ref[...], preferred_element_type=jnp.float32) + bout_ref[...]
    mx = jnp.max(logits, axis=-1, keepdims=True)
    e = jnp.exp(logits - mx)
    y_ref[...] = e / jnp.sum(e, axis=-1, keepdims=True)


def _final(outPr, g, b, qa2, hp, wout, bout):
    return pl.pallas_call(
        _fin_body,
        grid=(N // ROWS_BLK,),
        in_specs=[
            pl.BlockSpec((4, ROWS_BLK, DH), lambda i: (0, i, 0)),
            pl.BlockSpec((1, HIDE), lambda i: (0, 0)),
            pl.BlockSpec((1, HIDE), lambda i: (0, 0)),
            pl.BlockSpec((ROWS_BLK, 1), lambda i: (i, 0)),
            pl.BlockSpec((NSP_P, HIDE), lambda i: (0, 0)),
            pl.BlockSpec((HIDE, 128), lambda i: (0, 0)),
            pl.BlockSpec((1, 128), lambda i: (0, 0)),
        ],
        out_specs=pl.BlockSpec((ROWS_BLK, 128), lambda i: (i, 0)),
        out_shape=jax.ShapeDtypeStruct((N, 128), jnp.float32),
    )(outPr, g, b, qa2, hp, wout, bout)


RPW = N // NS
RPP = 125
KB = 32
NCHB = EM // KB
KPB = -(-NCHB // NW)


@functools.partial(
    pl.kernel,
    out_type=jax.ShapeDtypeStruct((NC, HEAD, N, DH), jnp.float32),
    mesh=_sc_mesh,
    compiler_params=pltpu.CompilerParams(
        use_tc_tiling_on_sc=False, needs_layout_passes=False),
    scratch_types=[
        pltpu.VMEM((2, KB), jnp.int32),
        pltpu.VMEM((2, KB), jnp.int32),
        pltpu.VMEM((2, KB, DH), jnp.float32),
        pltpu.VMEM((2, KB), jnp.float32),
        pltpu.VMEM((KB,), jnp.float32),
        pltpu.VMEM((RPP, DH), jnp.float32),
        pltpu.VMEM((NSP_S,), jnp.float32),
        pltpu.VMEM((NSP_S,), jnp.float32),
        pltpu.VMEM_SHARED((N, DH), jnp.float32),
        [pltpu.SemaphoreType.DMA] * 2,
        [pltpu.SemaphoreType.DMA] * 2,
    ],
)
def _edge_b(v0_hbm, v1_hbm, e_hbm, s_hbm, src_hbm, dst_hbm, out_hbm,
            src_v, dst_v, vrow_v, e_v, w_v, zbuf_v, sA_v, sB_v, out_sh, si, sg):
    cid = lax.axis_index("c")
    sid = lax.axis_index("s")
    wid = cid * NS + sid

    for hh in range(HEAD):
        v_hbm = v0_hbm if hh == 0 else v1_hbm
        pltpu.sync_copy(s_hbm.at[0, hh], sA_v)
        pltpu.sync_copy(s_hbm.at[1, hh], sB_v)

        def _sadd(i, _):
            sA_v[pl.ds(i * 16, 16)] = sA_v[pl.ds(i * 16, 16)] + sB_v[pl.ds(i * 16, 16)]
            return 0
        lax.fori_loop(0, NSP_S // 16, _sadd, 0)

        def _zrow2(i, _):
            def _zc(c, _):
                zbuf_v[i, pl.ds(c * 16, 16)] = jnp.zeros((16,), jnp.float32)
                return 0
            lax.fori_loop(0, DH // 16, _zc, 0)
            return 0
        lax.fori_loop(0, RPP, _zrow2, 0)
        for p in range(RPW // RPP):
            pltpu.sync_copy(zbuf_v,
                            out_sh.at[pl.ds(sid * RPW + p * RPP, RPP), :])
        plsc.subcore_barrier()

        def _issue_idx(c, b):
            @pl.when(c < NCHB)
            def _():
                base = c * KB
                pltpu.async_copy(src_hbm.at[pl.ds(base, KB)], src_v.at[b], si[b])
                pltpu.async_copy(dst_hbm.at[pl.ds(base, KB)], dst_v.at[b], si[b])
                pltpu.async_copy(e_hbm.at[hh, pl.ds(base, KB)], e_v.at[b], si[b])

        def _issue_gather(c, b):
            @pl.when(c < NCHB)
            def _():
                base = c * KB
                pltpu.make_async_copy(src_hbm.at[pl.ds(base, KB)], src_v.at[b], si[b]).wait()
                pltpu.make_async_copy(dst_hbm.at[pl.ds(base, KB)], dst_v.at[b], si[b]).wait()
                pltpu.make_async_copy(e_hbm.at[hh, pl.ds(base, KB)], e_v.at[b], si[b]).wait()
                pltpu.async_copy(v_hbm.at[dst_v.at[b]], vrow_v.at[b], sg[b])

        def _compute(c, b):
            @pl.when(c < NCHB)
            def _():
                pltpu.make_async_copy(v_hbm.at[dst_v.at[b]], vrow_v.at[b], sg[b]).wait()

                def _wgrp(g, _):
                    sl = pl.ds(g * 16, 16)
                    sv = plsc.load_gather(sA_v, [src_v[b, sl]])
                    w_v[sl] = e_v[b, sl] / sv
                    return 0
                lax.fori_loop(0, KB // 16, _wgrp, 0)

                def _scale(ei, _):
                    wb = plsc.load_gather(w_v, [jnp.full((16,), ei, jnp.int32)])
                    for c2 in range(DH // 16):
                        vrow_v[b, ei, pl.ds(c2 * 16, 16)] = vrow_v[b, ei, pl.ds(c2 * 16, 16)] * wb
                    return 0
                lax.fori_loop(0, KB, _scale, 0, unroll=2)

                pltpu.sync_copy(vrow_v.at[b], out_sh.at[src_v.at[b]], add=True)

        _issue_idx(wid, 0)
        _issue_idx(wid + NW, 1)
        _issue_gather(wid, 0)

        def _pair(k2, _):
            for b in range(2):
                k = 2 * k2 + b
                c = wid + NW * k
                _issue_gather(c + NW, 1 - b)
                _compute(c, b)
                _issue_idx(c + 2 * NW, b)
            return 0

        lax.fori_loop(0, (KPB + 1) // 2, _pair, 0)
        plsc.subcore_barrier()

        for p in range(RPW // RPP):
            r0 = sid * RPW + p * RPP
            pltpu.sync_copy(out_sh.at[pl.ds(r0, RPP), :], zbuf_v)
            pltpu.sync_copy(zbuf_v,
                            out_hbm.at[cid, hh, pl.ds(r0, RPP), :])
        if hh == 0:
            plsc.subcore_barrier()


NCH_P = N // KCH
PTAIL = N - NCH_P * KCH
KPB_P = -(-NCH_P // NW)
SPSTR = NSP_P // NS


@functools.partial(
    pl.kernel,
    out_type=[
        jax.ShapeDtypeStruct((NC, NSP_P, HIDE), jnp.float32),
        jax.ShapeDtypeStruct((NC, NSP_P), jnp.float32),
    ],
    mesh=_sc_mesh,
    compiler_params=pltpu.CompilerParams(
        use_tc_tiling_on_sc=False, needs_layout_passes=False),
    scratch_types=[
        pltpu.VMEM((KCH,), jnp.int32),
        pltpu.VMEM((KCH, HIDE), jnp.float32),
        pltpu.VMEM((KCH,), jnp.float32),
        pltpu.VMEM((PTAIL,), jnp.int32),
        pltpu.VMEM((PTAIL, HIDE), jnp.float32),
        pltpu.VMEM((PTAIL,), jnp.float32),
        pltpu.VMEM((SPSTR,), jnp.float32),
        pltpu.VMEM_SHARED((NSP_P, HIDE), jnp.float32),
        pltpu.VMEM_SHARED((NSP_P,), jnp.float32),
    ],
)
def _pool(x_hbm, qa_hbm, hp_hbm, deg_hbm,
          qa_v, rows_v, ones_v, qat_v, rowst_v, onest_v, buf_v,
          hp_sh, deg_sh):
    cid = lax.axis_index("c")
    sid = lax.axis_index("s")
    wid = cid * NS + sid

    def _fill(ref, n, val):
        def _f(i, _):
            ref[pl.ds(i * 16, 16)] = jnp.full((16,), val, jnp.float32)
            return 0
        lax.fori_loop(0, n // 16, _f, 0)

    _fill(ones_v, KCH, 1.0)
    _fill(onest_v, PTAIL, 1.0)
    _fill(buf_v, SPSTR, 0.0)

    def _zrow(i, _):
        def _zc(c, _):
            rows_v[i, pl.ds(c * 16, 16)] = jnp.zeros((16,), jnp.float32)
            return 0
        lax.fori_loop(0, HIDE // 16, _zc, 0)
        return 0
    lax.fori_loop(0, SPSTR, _zrow, 0)
    pltpu.sync_copy(rows_v.at[pl.ds(0, SPSTR), :],
                    hp_sh.at[pl.ds(sid * SPSTR, SPSTR), :])
    pltpu.sync_copy(buf_v, deg_sh.at[pl.ds(sid * SPSTR, SPSTR)])
    plsc.subcore_barrier()

    def _chunk(k, _):
        cidx = wid + NW * k

        @pl.when(cidx < NCH_P)
        def _():
            base = cidx * KCH
            pltpu.sync_copy(qa_hbm.at[pl.ds(base, KCH)], qa_v)
            pltpu.sync_copy(x_hbm.at[pl.ds(base, KCH), :], rows_v)
            pltpu.sync_copy(rows_v, hp_sh.at[qa_v], add=True)
            pltpu.sync_copy(ones_v, deg_sh.at[qa_v], add=True)
        return 0

    lax.fori_loop(0, KPB_P, _chunk, 0)

    @pl.when(wid == 0)
    def _tail():
        base = NCH_P * KCH
        pltpu.sync_copy(qa_hbm.at[pl.ds(base, PTAIL)], qat_v)
        pltpu.sync_copy(x_hbm.at[pl.ds(base, PTAIL), :], rowst_v)
        pltpu.sync_copy(rowst_v, hp_sh.at[qat_v], add=True)
        pltpu.sync_copy(onest_v, deg_sh.at[qat_v], add=True)

    plsc.subcore_barrier()
    r0 = sid * SPSTR
    pltpu.sync_copy(hp_sh.at[pl.ds(r0, SPSTR), :], rows_v.at[pl.ds(0, SPSTR), :])
    pltpu.sync_copy(rows_v.at[pl.ds(0, SPSTR), :], hp_hbm.at[cid, pl.ds(r0, SPSTR), :])
    pltpu.sync_copy(deg_sh.at[pl.ds(r0, SPSTR)], buf_v)
    pltpu.sync_copy(buf_v, deg_hbm.at[cid, pl.ds(r0, SPSTR)])


NCH_A = EA // KCH
ATAIL = EA - NCH_A * KCH
KPB_A = -(-NCH_A // NW)
AFL = NSP_P * NSP_P
ASTR = AFL // NS


@functools.partial(
    pl.kernel,
    out_type=jax.ShapeDtypeStruct((NC, AFL), jnp.float32),
    mesh=_sc_mesh,
    compiler_params=pltpu.CompilerParams(
        use_tc_tiling_on_sc=False, needs_layout_passes=False),
    scratch_types=[
        pltpu.VMEM((KCH,), jnp.int32),
        pltpu.VMEM((KCH,), jnp.int32),
        pltpu.VMEM((KCH,), jnp.float32),
        pltpu.VMEM((KCH,), jnp.int32),
        pltpu.VMEM((ATAIL,), jnp.int32),
        pltpu.VMEM((ATAIL,), jnp.int32),
        pltpu.VMEM((ATAIL,), jnp.float32),
        pltpu.VMEM((ATAIL,), jnp.int32),
        pltpu.VMEM((ASTR,), jnp.float32),
        pltpu.VMEM_SHARED((AFL,), jnp.float32),
    ],
)
def _abuild(asrc_hbm, adst_hbm, aval_hbm, A_hbm,
            s_v, d_v, v_v, fidx_v, st_v, dt_v, vt_v, fidxt_v, zbuf_v, A_sh):
    cid = lax.axis_index("c")
    sid = lax.axis_index("s")
    wid = cid * NS + sid

    def _f(i, _):
        zbuf_v[pl.ds(i * 16, 16)] = jnp.zeros((16,), jnp.float32)
        return 0
    lax.fori_loop(0, ASTR // 16, _f, 0)
    pltpu.sync_copy(zbuf_v, A_sh.at[pl.ds(sid * ASTR, ASTR)])
    plsc.subcore_barrier()

    def _chunk(k, _):
        cidx = wid + NW * k

        @pl.when(cidx < NCH_A)
        def _():
            base = cidx * KCH
            pltpu.sync_copy(asrc_hbm.at[pl.ds(base, KCH)], s_v)
            pltpu.sync_copy(adst_hbm.at[pl.ds(base, KCH)], d_v)
            pltpu.sync_copy(aval_hbm.at[pl.ds(base, KCH)], v_v)

            def _fi(g, _):
                sl = pl.ds(g * 16, 16)
                fidx_v[sl] = d_v[sl] * NSP_P + s_v[sl]
                return 0
            lax.fori_loop(0, KCH // 16, _fi, 0)
            pltpu.sync_copy(v_v, A_sh.at[fidx_v], add=True)
        return 0

    lax.fori_loop(0, KPB_A, _chunk, 0)

    @pl.when(wid == 0)
    def _tail():
        base = NCH_A * KCH
        pltpu.sync_copy(asrc_hbm.at[pl.ds(base, ATAIL)], st_v)
        pltpu.sync_copy(adst_hbm.at[pl.ds(base, ATAIL)], dt_v)
        pltpu.sync_copy(aval_hbm.at[pl.ds(base, ATAIL)], vt_v)

        def _fi(g, _):
            sl = pl.ds(g * 16, 16)
            fidxt_v[sl] = dt_v[sl] * NSP_P + st_v[sl]
            return 0
        lax.fori_loop(0, ATAIL // 16, _fi, 0)
        pltpu.sync_copy(vt_v, A_sh.at[fidxt_v], add=True)

    plsc.subcore_barrier()
    pltpu.sync_copy(A_sh.at[pl.ds(sid * ASTR, ASTR)], zbuf_v)
    pltpu.sync_copy(zbuf_v, A_hbm.at[cid, pl.ds(sid * ASTR, ASTR)])


def kernel(x, mask_index, mask_value, q_assign, a_index, a_value,
           prelin_W, prelin_b, bn0_g, bn0_b,
           sf_W, sf_b, sf_g, sf_be,
           pf_Wv, pf_bv, pf_Wq, pf_bq, pf_g, pf_be,
           out_W, out_b):
    bnscale = 1.0 / jnp.sqrt(1.0 + 1e-5)
    w1 = prelin_W * (bn0_g * bnscale)[:, None]
    b1 = prelin_b * (bn0_g * bnscale) + bn0_b

    src = mask_index[0]
    dst = mask_index[1]

    outPr = None
    for j in range(NPF):
        wcat = jnp.concatenate([pf_Wq[j, 0], pf_Wq[j, 1], pf_Wv[j, 0], pf_Wv[j, 1]], axis=0)
        bcat = jnp.concatenate([pf_bq[j, 0], pf_bq[j, 1], pf_bv[j, 0], pf_bv[j, 1]], axis=0)
        if j == 0:
            h0, qcat, vcat = _prelin_proj(x, w1.T, b1[None, :], wcat.T, bcat[None, :])
            x_pre = h0
        else:
            g0 = (pf_g[0] * bnscale)[None, :]
            b0 = pf_be[0][None, :]
            qcat, vcat = _mid(outPr, g0, b0, wcat.T, bcat[None, :])
        e2, sP = _edge_a(qcat, src, dst)
        outP = _edge_b(vcat[:, :DH], vcat[:, DH:], e2, sP, src, dst)
        outPr = outP.reshape(NC * HEAD, N, DH)

    hpP, degP = _pool(x_pre, q_assign)
    APf = _abuild(a_index[0], a_index[1], a_value)
    AP = APf.reshape(NC, NSP_P, NSP_P)
    hp_fin = _sf_dense(hpP, degP, AP, jnp.transpose(sf_W, (0, 2, 1)),
                       sf_b, sf_g * bnscale, sf_be)

    g1 = (pf_g[1] * bnscale)[None, :]
    b1f = pf_be[1][None, :]
    woutp = jnp.zeros((HIDE, 128), jnp.float32).at[:, :NCLS].set(out_W.T)
    boutp = jnp.full((128,), -1e30, jnp.float32).at[:NCLS].set(out_b)
    ypad = _final(outPr, g1, b1f, q_assign.reshape(N, 1), hp_fin,
                  woutp, boutp[None, :])
    return ypad[:, :NCLS]


# speedup vs baseline: 11.7811x; 1.0475x over previous
import functools

import jax
import jax.numpy as jnp
from jax import lax
from jax.experimental import pallas as pl
from jax.experimental.pallas import tpu as pltpu
from jax.experimental.pallas import tpu_sc as plsc

N = 10000
NSP = 500
C = 256
HIDE = 256
NCLS = 16
EM = 160000
EA = 8000
HEAD = 2
DH = HIDE // HEAD
NSF = 5
NPF = 2
RHP = 5
GAMA = 0.9

ROWS_BLK = 2000


def _prelin_proj_body(x_ref, w1_ref, b1_ref, wcat_ref, bcat_ref,
                      h0_ref, qcat_ref, vcat_ref):
    x = x_ref[...]
    h0 = jnp.dot(x, w1_ref[...], preferred_element_type=jnp.float32) + b1_ref[...]
    h0_ref[...] = h0
    cat = jnp.dot(h0, wcat_ref[...], preferred_element_type=jnp.float32) + bcat_ref[...]
    q = cat[:, :HIDE]
    v = cat[:, HIDE:]
    q2 = q.reshape(q.shape[0], HEAD, DH)
    m = jnp.mean(q2, axis=-1, keepdims=True)
    var = jnp.mean((q2 - m) ** 2, axis=-1, keepdims=True)
    qn = ((q2 - m) / jnp.sqrt(var + 1e-5)).reshape(q.shape)
    qcat_ref[...] = qn
    vcat_ref[...] = v


def _prelin_proj(x, w1t, b1, wcatt, bcat):
    grid = (N // ROWS_BLK,)
    return pl.pallas_call(
        _prelin_proj_body,
        grid=grid,
        in_specs=[
            pl.BlockSpec((ROWS_BLK, C), lambda i: (i, 0)),
            pl.BlockSpec((C, HIDE), lambda i: (0, 0)),
            pl.BlockSpec((1, HIDE), lambda i: (0, 0)),
            pl.BlockSpec((HIDE, 2 * HIDE), lambda i: (0, 0)),
            pl.BlockSpec((1, 2 * HIDE), lambda i: (0, 0)),
        ],
        out_specs=[
            pl.BlockSpec((ROWS_BLK, HIDE), lambda i: (i, 0)),
            pl.BlockSpec((ROWS_BLK, HIDE), lambda i: (i, 0)),
            pl.BlockSpec((ROWS_BLK, HIDE), lambda i: (i, 0)),
        ],
        out_shape=[
            jax.ShapeDtypeStruct((N, HIDE), jnp.float32),
            jax.ShapeDtypeStruct((N, HIDE), jnp.float32),
            jax.ShapeDtypeStruct((N, HIDE), jnp.float32),
        ],
    )(x, w1t, b1, wcatt, bcat)


def _leaky(x):
    return jnp.where(x >= 0, x, 0.01 * x)


NC = 2
NS = 16
NW = NC * NS
KCH = 128
NCHUNKS = EM // KCH
KPW = -(-NCHUNKS // NW)
NSP_S = 10240
SSTR = NSP_S // NS

_sc_mesh = plsc.VectorSubcoreMesh(core_axis_name="c", subcore_axis_name="s")


KA = 64
NCHA = EM // KA
KPA = -(-NCHA // NW)
KPA4 = ((KPA + 3) // 4) * 4


@functools.partial(
    pl.kernel,
    out_type=[
        jax.ShapeDtypeStruct((2, EM), jnp.float32),
        jax.ShapeDtypeStruct((NC, 2, NSP_S), jnp.float32),
    ],
    mesh=_sc_mesh,
    compiler_params=pltpu.CompilerParams(
        use_tc_tiling_on_sc=False, needs_layout_passes=False),
    scratch_types=[
        pltpu.VMEM((2, KA), jnp.int32),
        pltpu.VMEM((2, KA), jnp.int32),
        pltpu.VMEM((2, KA, HIDE), jnp.float32),
        pltpu.VMEM((2, KA, HIDE), jnp.float32),
        pltpu.VMEM((2, KA), jnp.float32),
        pltpu.VMEM((2, KA), jnp.float32),
        pltpu.VMEM((SSTR,), jnp.float32),
        pltpu.VMEM_SHARED((NSP_S,), jnp.float32),
        pltpu.VMEM_SHARED((NSP_S,), jnp.float32),
        [pltpu.SemaphoreType.DMA] * 2,
        [pltpu.SemaphoreType.DMA] * 2,
    ],
)
def _edge_a(qcat_hbm, src_hbm, dst_hbm, e_hbm, s_hbm,
            src_v, dst_v, qsrc_v, qdst_v, e0_v, e1_v, buf_v,
            s0_sh, s1_sh, si, sg):
    cid = lax.axis_index("c")
    sid = lax.axis_index("s")
    wid = cid * NS + sid

    def _zero(i, _):
        buf_v[pl.ds(i * 16, 16)] = jnp.zeros((16,), jnp.float32)
        return 0
    lax.fori_loop(0, SSTR // 16, _zero, 0)
    pltpu.sync_copy(buf_v, s0_sh.at[pl.ds(sid * SSTR, SSTR)])
    pltpu.sync_copy(buf_v, s1_sh.at[pl.ds(sid * SSTR, SSTR)])
    plsc.subcore_barrier()

    def _issue_idx(c, b4):
        @pl.when(c < NCHA)
        def _():
            base = c * KA
            pltpu.async_copy(src_hbm.at[pl.ds(base, KA)], src_v.at[b4], si[b4])
            pltpu.async_copy(dst_hbm.at[pl.ds(base, KA)], dst_v.at[b4], si[b4])

    def _issue_gather(c, b4, b2):
        @pl.when(c < NCHA)
        def _():
            base = c * KA
            pltpu.make_async_copy(src_hbm.at[pl.ds(base, KA)], src_v.at[b4], si[b4]).wait()
            pltpu.make_async_copy(dst_hbm.at[pl.ds(base, KA)], dst_v.at[b4], si[b4]).wait()
            pltpu.async_copy(qcat_hbm.at[src_v.at[b4]], qsrc_v.at[b2], sg[b2])
            pltpu.async_copy(qcat_hbm.at[dst_v.at[b4]], qdst_v.at[b2], sg[b2])

    def _compute(c, b4, b2):
        @pl.when(c < NCHA)
        def _():
            base = c * KA
            pltpu.make_async_copy(qcat_hbm.at[src_v.at[b4]], qsrc_v.at[b2], sg[b2]).wait()
            pltpu.make_async_copy(qcat_hbm.at[dst_v.at[b4]], qdst_v.at[b2], sg[b2]).wait()
            lastlane = lax.iota(jnp.int32, 16) == 15

            def _edge(ei, _):
                z = jnp.zeros((16,), jnp.float32)
                a0 = z
                a1 = z
                for cc in range(DH // 16):
                    a0 = a0 + qsrc_v[b2, ei, pl.ds(cc * 16, 16)] * qdst_v[b2, ei, pl.ds(cc * 16, 16)]
                    a1 = a1 + qsrc_v[b2, ei, pl.ds(DH + cc * 16, 16)] * qdst_v[b2, ei, pl.ds(DH + cc * 16, 16)]
                eidx = jnp.full((16,), ei, jnp.int32)
                plsc.store_scatter(e0_v.at[b4], [eidx], plsc.cumsum(a0), mask=lastlane)
                plsc.store_scatter(e1_v.at[b4], [eidx], plsc.cumsum(a1), mask=lastlane)
                return 0

            lax.fori_loop(0, KA, _edge, 0, unroll=2)

            def _vexp(g, _):
                sl = pl.ds(g * 16, 16)
                e0_v[b4, sl] = jnp.exp(e0_v[b4, sl] * (1.0 / HIDE))
                e1_v[b4, sl] = jnp.exp(e1_v[b4, sl] * (1.0 / HIDE))
                return 0

            lax.fori_loop(0, KA // 16, _vexp, 0)
            pltpu.sync_copy(e0_v.at[b4], e_hbm.at[0, pl.ds(base, KA)])
            pltpu.sync_copy(e1_v.at[b4], e_hbm.at[1, pl.ds(base, KA)])
            pltpu.sync_copy(e0_v.at[b4], s0_sh.at[src_v.at[b4]], add=True)
            pltpu.sync_copy(e1_v.at[b4], s1_sh.at[src_v.at[b4]], add=True)

    _issue_idx(wid, 0)
    _issue_idx(wid + NW, 1)
    _issue_gather(wid, 0, 0)

    def _pair(k2, _):
        for b in range(2):
            k = 2 * k2 + b
            c = wid + NW * k
            _issue_gather(c + NW, 1 - b, 1 - b)
            _compute(c, b, b)
            _issue_idx(c + 2 * NW, b)
        return 0

    lax.fori_loop(0, KPA4 // 2, _pair, 0)
    plsc.subcore_barrier()

    pltpu.sync_copy(s0_sh.at[pl.ds(sid * SSTR, SSTR)], buf_v)
    pltpu.sync_copy(buf_v, s_hbm.at[cid, 0, pl.ds(sid * SSTR, SSTR)])
    pltpu.sync_copy(s1_sh.at[pl.ds(sid * SSTR, SSTR)], buf_v)
    pltpu.sync_copy(buf_v, s_hbm.at[cid, 1, pl.ds(sid * SSTR, SSTR)])


NSP_P = 512


def _mid_body(o_ref, g_ref, b_ref, wcat_ref, bcat_ref, qcat_ref, vcat_ref):
    o = o_ref[...]
    h = jnp.concatenate([o[0] + o[2], o[1] + o[3]], axis=1)
    h = h * g_ref[...] + b_ref[...]
    h = jnp.where(h >= 0, h, 0.01 * h)
    cat = jnp.dot(h, wcat_ref[...], preferred_element_type=jnp.float32) + bcat_ref[...]
    q = cat[:, :HIDE]
    q2 = q.reshape(q.shape[0], HEAD, DH)
    m = jnp.mean(q2, axis=-1, keepdims=True)
    var = jnp.mean((q2 - m) ** 2, axis=-1, keepdims=True)
    qcat_ref[...] = ((q2 - m) / jnp.sqrt(var + 1e-5)).reshape(q.shape)
    vcat_ref[...] = cat[:, HIDE:]


def _mid(outPr, g, b, wcatt, bcat):
    return pl.pallas_call(
        _mid_body,
        grid=(N // ROWS_BLK,),
        in_specs=[
            pl.BlockSpec((4, ROWS_BLK, DH), lambda i: (0, i, 0)),
            pl.BlockSpec((1, HIDE), lambda i: (0, 0)),
            pl.BlockSpec((1, HIDE), lambda i: (0, 0)),
            pl.BlockSpec((HIDE, 2 * HIDE), lambda i: (0, 0)),
            pl.BlockSpec((1, 2 * HIDE), lambda i: (0, 0)),
        ],
        out_specs=[
            pl.BlockSpec((ROWS_BLK, HIDE), lambda i: (i, 0)),
            pl.BlockSpec((ROWS_BLK, HIDE), lambda i: (i, 0)),
        ],
        out_shape=[
            jax.ShapeDtypeStruct((N, HIDE), jnp.float32),
            jax.ShapeDtypeStruct((N, HIDE), jnp.float32),
        ],
    )(outPr, g, b, wcatt, bcat)


def _sf_body(hpP_ref, degP_ref, AP_ref, wt_ref, b_ref, g_ref, be_ref, out_ref):
    degP = degP_ref[...]
    deg = degP[0] + degP[1]
    inv = jnp.where(deg > 0, 1.0 / deg, 0.0)
    hp = (hpP_ref[0] + hpP_ref[1]) * inv[:, None]
    A = AP_ref[0] + AP_ref[1]
    for i in range(NSF):
        hlin = jnp.dot(hp, wt_ref[i], preferred_element_type=jnp.float32) + b_ref[i]
        x_start = jnp.dot(A, hlin, preferred_element_type=jnp.float32)
        xc = hp
        for _ in range(RHP):
            xc = (jnp.dot(A, xc, preferred_element_type=jnp.float32) + x_start) * (1.0 / (2.0 + GAMA))
        hp = xc * g_ref[i] + be_ref[i]
        hp = jnp.where(hp >= 0, hp, 0.01 * hp)
    out_ref[...] = hp


def _sf_dense(hpP, degP, AP, wt, b, g, be):
    return pl.pallas_call(
        _sf_body,
        out_shape=jax.ShapeDtypeStruct((NSP_P, HIDE), jnp.float32),
    )(hpP, degP, AP, wt, b, g, be)


def _fin_body(o_ref, g_ref, b_ref, qa_ref, hp_ref, wout_ref, bout_ref, y_ref):
    o = o_ref[...]
    h = jnp.concatenate([o[0] + o[2], o[1] + o[3]], axis=1)
    h = h * g_ref[...] + b_ref[...]
    pf = jnp.where(h >= 0, h, 0.01 * h)
    qa = qa_ref[...]
    oh = (qa == jax.lax.broadcasted_iota(jnp.int32, (qa.shape[0], NSP_P), 1)
          ).astype(jnp.float32)
    sf = jnp.dot(oh, hp_ref[...], preferred_element_type=jnp.float32)
    H1 = pf + sf
    logits = jnp.dot(H1, wout_ref[...], preferred_element_type=jnp.float32) + bout_ref[...]
    mx = jnp.max(logits, axis=-1, keepdims=True)
    e = jnp.exp(logits - mx)
    y_ref[...] = e / jnp.sum(e, axis=-1, keepdims=True)


def _final(outPr, g, b, qa2, hp, wout, bout):
    return pl.pallas_call(
        _fin_body,
        grid=(N // ROWS_BLK,),
        in_specs=[
            pl.BlockSpec((4, ROWS_BLK, DH), lambda i: (0, i, 0)),
            pl.BlockSpec((1, HIDE), lambda i: (0, 0)),
            pl.BlockSpec((1, HIDE), lambda i: (0, 0)),
            pl.BlockSpec((ROWS_BLK, 1), lambda i: (i, 0)),
            pl.BlockSpec((NSP_P, HIDE), lambda i: (0, 0)),
            pl.BlockSpec((HIDE, 128), lambda i: (0, 0)),
            pl.BlockSpec((1, 128), lambda i: (0, 0)),
        ],
        out_specs=pl.BlockSpec((ROWS_BLK, 128), lambda i: (i, 0)),
        out_shape=jax.ShapeDtypeStruct((N, 128), jnp.float32),
    )(outPr, g, b, qa2, hp, wout, bout)


RPW = N // NS
RPP = 125
KB = 40
NCHB = EM // KB
KPB = -(-NCHB // NW)


@functools.partial(
    pl.kernel,
    out_type=jax.ShapeDtypeStruct((NC, HEAD, N, DH), jnp.float32),
    mesh=_sc_mesh,
    compiler_params=pltpu.CompilerParams(
        use_tc_tiling_on_sc=False, needs_layout_passes=False),
    scratch_types=[
        pltpu.VMEM((2, KB), jnp.int32),
        pltpu.VMEM((2, KB), jnp.int32),
        pltpu.VMEM((2, KB, DH), jnp.float32),
        pltpu.VMEM((2, KB), jnp.float32),
        pltpu.VMEM((KB,), jnp.float32),
        pltpu.VMEM((RPP, DH), jnp.float32),
        pltpu.VMEM((NSP_S,), jnp.float32),
        pltpu.VMEM((NSP_S,), jnp.float32),
        pltpu.VMEM_SHARED((N, DH), jnp.float32),
        [pltpu.SemaphoreType.DMA] * 2,
        [pltpu.SemaphoreType.DMA] * 2,
    ],
)
def _edge_b(v0_hbm, v1_hbm, e_hbm, s_hbm, src_hbm, dst_hbm, out_hbm,
            src_v, dst_v, vrow_v, e_v, w_v, zbuf_v, sA_v, sB_v, out_sh, si, sg):
    cid = lax.axis_index("c")
    sid = lax.axis_index("s")
    wid = cid * NS + sid

    for hh in range(HEAD):
        v_hbm = v0_hbm if hh == 0 else v1_hbm
        pltpu.sync_copy(s_hbm.at[0, hh], sA_v)
        pltpu.sync_copy(s_hbm.at[1, hh], sB_v)

        def _sadd(i, _):
            sA_v[pl.ds(i * 16, 16)] = sA_v[pl.ds(i * 16, 16)] + sB_v[pl.ds(i * 16, 16)]
            return 0
        lax.fori_loop(0, NSP_S // 16, _sadd, 0)

        def _zrow2(i, _):
            def _zc(c, _):
                zbuf_v[i, pl.ds(c * 16, 16)] = jnp.zeros((16,), jnp.float32)
                return 0
            lax.fori_loop(0, DH // 16, _zc, 0)
            return 0
        lax.fori_loop(0, RPP, _zrow2, 0)
        for p in range(RPW // RPP):
            pltpu.sync_copy(zbuf_v,
                            out_sh.at[pl.ds(sid * RPW + p * RPP, RPP), :])
        plsc.subcore_barrier()

        def _issue_idx(c, b):
            @pl.when(c < NCHB)
            def _():
                base = c * KB
                pltpu.async_copy(src_hbm.at[pl.ds(base, KB)], src_v.at[b], si[b])
                pltpu.async_copy(dst_hbm.at[pl.ds(base, KB)], dst_v.at[b], si[b])
                pltpu.async_copy(e_hbm.at[hh, pl.ds(base, KB)], e_v.at[b], si[b])

        def _issue_gather(c, b):
            @pl.when(c < NCHB)
            def _():
                base = c * KB
                pltpu.make_async_copy(src_hbm.at[pl.ds(base, KB)], src_v.at[b], si[b]).wait()
                pltpu.make_async_copy(dst_hbm.at[pl.ds(base, KB)], dst_v.at[b], si[b]).wait()
                pltpu.make_async_copy(e_hbm.at[hh, pl.ds(base, KB)], e_v.at[b], si[b]).wait()
                pltpu.async_copy(v_hbm.at[dst_v.at[b]], vrow_v.at[b], sg[b])

        def _compute(c, b):
            @pl.when(c < NCHB)
            def _():
                pltpu.make_async_copy(v_hbm.at[dst_v.at[b]], vrow_v.at[b], sg[b]).wait()

                def _wgrp(g, _):
                    sl = pl.ds(g * 16, 16)
                    sv = plsc.load_gather(sA_v, [src_v[b, sl]])
                    w_v[sl] = e_v[b, sl] / sv
                    return 0
                lax.fori_loop(0, KB // 16, _wgrp, 0)

                def _scale(ei, _):
                    wb = plsc.load_gather(w_v, [jnp.full((16,), ei, jnp.int32)])
                    for c2 in range(DH // 16):
                        vrow_v[b, ei, pl.ds(c2 * 16, 16)] = vrow_v[b, ei, pl.ds(c2 * 16, 16)] * wb
                    return 0
                lax.fori_loop(0, KB, _scale, 0, unroll=2)

                pltpu.sync_copy(vrow_v.at[b], out_sh.at[src_v.at[b]], add=True)

        _issue_idx(wid, 0)
        _issue_idx(wid + NW, 1)
        _issue_gather(wid, 0)

        def _pair(k2, _):
            for b in range(2):
                k = 2 * k2 + b
                c = wid + NW * k
                _issue_gather(c + NW, 1 - b)
                _compute(c, b)
                _issue_idx(c + 2 * NW, b)
            return 0

        lax.fori_loop(0, (KPB + 1) // 2, _pair, 0)
        plsc.subcore_barrier()

        for p in range(RPW // RPP):
            r0 = sid * RPW + p * RPP
            pltpu.sync_copy(out_sh.at[pl.ds(r0, RPP), :], zbuf_v)
            pltpu.sync_copy(zbuf_v,
                            out_hbm.at[cid, hh, pl.ds(r0, RPP), :])
        if hh == 0:
            plsc.subcore_barrier()


NCH_P = N // KCH
PTAIL = N - NCH_P * KCH
KPB_P = -(-NCH_P // NW)
SPSTR = NSP_P // NS


@functools.partial(
    pl.kernel,
    out_type=[
        jax.ShapeDtypeStruct((NC, NSP_P, HIDE), jnp.float32),
        jax.ShapeDtypeStruct((NC, NSP_P), jnp.float32),
    ],
    mesh=_sc_mesh,
    compiler_params=pltpu.CompilerParams(
        use_tc_tiling_on_sc=False, needs_layout_passes=False),
    scratch_types=[
        pltpu.VMEM((KCH,), jnp.int32),
        pltpu.VMEM((KCH, HIDE), jnp.float32),
        pltpu.VMEM((KCH,), jnp.float32),
        pltpu.VMEM((PTAIL,), jnp.int32),
        pltpu.VMEM((PTAIL, HIDE), jnp.float32),
        pltpu.VMEM((PTAIL,), jnp.float32),
        pltpu.VMEM((SPSTR,), jnp.float32),
        pltpu.VMEM_SHARED((NSP_P, HIDE), jnp.float32),
        pltpu.VMEM_SHARED((NSP_P,), jnp.float32),
    ],
)
def _pool(x_hbm, qa_hbm, hp_hbm, deg_hbm,
          qa_v, rows_v, ones_v, qat_v, rowst_v, onest_v, buf_v,
          hp_sh, deg_sh):
    cid = lax.axis_index("c")
    sid = lax.axis_index("s")
    wid = cid * NS + sid

    def _fill(ref, n, val):
        def _f(i, _):
            ref[pl.ds(i * 16, 16)] = jnp.full((16,), val, jnp.float32)
            return 0
        lax.fori_loop(0, n // 16, _f, 0)

    _fill(ones_v, KCH, 1.0)
    _fill(onest_v, PTAIL, 1.0)
    _fill(buf_v, SPSTR, 0.0)

    def _zrow(i, _):
        def _zc(c, _):
            rows_v[i, pl.ds(c * 16, 16)] = jnp.zeros((16,), jnp.float32)
            return 0
        lax.fori_loop(0, HIDE // 16, _zc, 0)
        return 0
    lax.fori_loop(0, SPSTR, _zrow, 0)
    pltpu.sync_copy(rows_v.at[pl.ds(0, SPSTR), :],
                    hp_sh.at[pl.ds(sid * SPSTR, SPSTR), :])
    pltpu.sync_copy(buf_v, deg_sh.at[pl.ds(sid * SPSTR, SPSTR)])
    plsc.subcore_barrier()

    def _chunk(k, _):
        cidx = wid + NW * k

        @pl.when(cidx < NCH_P)
        def _():
            base = cidx * KCH
            pltpu.sync_copy(qa_hbm.at[pl.ds(base, KCH)], qa_v)
            pltpu.sync_copy(x_hbm.at[pl.ds(base, KCH), :], rows_v)
            pltpu.sync_copy(rows_v, hp_sh.at[qa_v], add=True)
            pltpu.sync_copy(ones_v, deg_sh.at[qa_v], add=True)
        return 0

    lax.fori_loop(0, KPB_P, _chunk, 0)

    @pl.when(wid == 0)
    def _tail():
        base = NCH_P * KCH
        pltpu.sync_copy(qa_hbm.at[pl.ds(base, PTAIL)], qat_v)
        pltpu.sync_copy(x_hbm.at[pl.ds(base, PTAIL), :], rowst_v)
        pltpu.sync_copy(rowst_v, hp_sh.at[qat_v], add=True)
        pltpu.sync_copy(onest_v, deg_sh.at[qat_v], add=True)

    plsc.subcore_barrier()
    r0 = sid * SPSTR
    pltpu.sync_copy(hp_sh.at[pl.ds(r0, SPSTR), :], rows_v.at[pl.ds(0, SPSTR), :])
    pltpu.sync_copy(rows_v.at[pl.ds(0, SPSTR), :], hp_hbm.at[cid, pl.ds(r0, SPSTR), :])
    pltpu.sync_copy(deg_sh.at[pl.ds(r0, SPSTR)], buf_v)
    pltpu.sync_copy(buf_v, deg_hbm.at[cid, pl.ds(r0, SPSTR)])


NCH_A = EA // KCH
ATAIL = EA - NCH_A * KCH
KPB_A = -(-NCH_A // NW)
AFL = NSP_P * NSP_P
ASTR = AFL // NS


@functools.partial(
    pl.kernel,
    out_type=jax.ShapeDtypeStruct((NC, AFL), jnp.float32),
    mesh=_sc_mesh,
    compiler_params=pltpu.CompilerParams(
        use_tc_tiling_on_sc=False, needs_layout_passes=False),
    scratch_types=[
        pltpu.VMEM((KCH,), jnp.int32),
        pltpu.VMEM((KCH,), jnp.int32),
        pltpu.VMEM((KCH,), jnp.float32),
        pltpu.VMEM((KCH,), jnp.int32),
        pltpu.VMEM((ATAIL,), jnp.int32),
        pltpu.VMEM((ATAIL,), jnp.int32),
        pltpu.VMEM((ATAIL,), jnp.float32),
        pltpu.VMEM((ATAIL,), jnp.int32),
        pltpu.VMEM((ASTR,), jnp.float32),
        pltpu.VMEM_SHARED((AFL,), jnp.float32),
    ],
)
def _abuild(asrc_hbm, adst_hbm, aval_hbm, A_hbm,
            s_v, d_v, v_v, fidx_v, st_v, dt_v, vt_v, fidxt_v, zbuf_v, A_sh):
    cid = lax.axis_index("c")
    sid = lax.axis_index("s")
    wid = cid * NS + sid

    def _f(i, _):
        zbuf_v[pl.ds(i * 16, 16)] = jnp.zeros((16,), jnp.float32)
        return 0
    lax.fori_loop(0, ASTR // 16, _f, 0)
    pltpu.sync_copy(zbuf_v, A_sh.at[pl.ds(sid * ASTR, ASTR)])
    plsc.subcore_barrier()

    def _chunk(k, _):
        cidx = wid + NW * k

        @pl.when(cidx < NCH_A)
        def _():
            base = cidx * KCH
            pltpu.sync_copy(asrc_hbm.at[pl.ds(base, KCH)], s_v)
            pltpu.sync_copy(adst_hbm.at[pl.ds(base, KCH)], d_v)
            pltpu.sync_copy(aval_hbm.at[pl.ds(base, KCH)], v_v)

            def _fi(g, _):
                sl = pl.ds(g * 16, 16)
                fidx_v[sl] = d_v[sl] * NSP_P + s_v[sl]
                return 0
            lax.fori_loop(0, KCH // 16, _fi, 0)
            pltpu.sync_copy(v_v, A_sh.at[fidx_v], add=True)
        return 0

    lax.fori_loop(0, KPB_A, _chunk, 0)

    @pl.when(wid == 0)
    def _tail():
        base = NCH_A * KCH
        pltpu.sync_copy(asrc_hbm.at[pl.ds(base, ATAIL)], st_v)
        pltpu.sync_copy(adst_hbm.at[pl.ds(base, ATAIL)], dt_v)
        pltpu.sync_copy(aval_hbm.at[pl.ds(base, ATAIL)], vt_v)

        def _fi(g, _):
            sl = pl.ds(g * 16, 16)
            fidxt_v[sl] = dt_v[sl] * NSP_P + st_v[sl]
            return 0
        lax.fori_loop(0, ATAIL // 16, _fi, 0)
        pltpu.sync_copy(vt_v, A_sh.at[fidxt_v], add=True)

    plsc.subcore_barrier()
    pltpu.sync_copy(A_sh.at[pl.ds(sid * ASTR, ASTR)], zbuf_v)
    pltpu.sync_copy(zbuf_v, A_hbm.at[cid, pl.ds(sid * ASTR, ASTR)])


def kernel(x, mask_index, mask_value, q_assign, a_index, a_value,
           prelin_W, prelin_b, bn0_g, bn0_b,
           sf_W, sf_b, sf_g, sf_be,
           pf_Wv, pf_bv, pf_Wq, pf_bq, pf_g, pf_be,
           out_W, out_b):
    bnscale = 1.0 / jnp.sqrt(1.0 + 1e-5)
    w1 = prelin_W * (bn0_g * bnscale)[:, None]
    b1 = prelin_b * (bn0_g * bnscale) + bn0_b

    src = mask_index[0]
    dst = mask_index[1]

    outPr = None
    for j in range(NPF):
        wcat = jnp.concatenate([pf_Wq[j, 0], pf_Wq[j, 1], pf_Wv[j, 0], pf_Wv[j, 1]], axis=0)
        bcat = jnp.concatenate([pf_bq[j, 0], pf_bq[j, 1], pf_bv[j, 0], pf_bv[j, 1]], axis=0)
        if j == 0:
            h0, qcat, vcat = _prelin_proj(x, w1.T, b1[None, :], wcat.T, bcat[None, :])
            x_pre = h0
        else:
            g0 = (pf_g[0] * bnscale)[None, :]
            b0 = pf_be[0][None, :]
            qcat, vcat = _mid(outPr, g0, b0, wcat.T, bcat[None, :])
        e2, sP = _edge_a(qcat, src, dst)
        outP = _edge_b(vcat[:, :DH], vcat[:, DH:], e2, sP, src, dst)
        outPr = outP.reshape(NC * HEAD, N, DH)

    hpP, degP = _pool(x_pre, q_assign)
    APf = _abuild(a_index[0], a_index[1], a_value)
    AP = APf.reshape(NC, NSP_P, NSP_P)
    hp_fin = _sf_dense(hpP, degP, AP, jnp.transpose(sf_W, (0, 2, 1)),
                       sf_b, sf_g * bnscale, sf_be)

    g1 = (pf_g[1] * bnscale)[None, :]
    b1f = pf_be[1][None, :]
    woutp = jnp.zeros((HIDE, 128), jnp.float32).at[:, :NCLS].set(out_W.T)
    boutp = jnp.full((128,), -1e30, jnp.float32).at[:NCLS].set(out_b)
    ypad = _final(outPr, g1, b1f, q_assign.reshape(N, 1), hp_fin,
                  woutp, boutp[None, :])
    return ypad[:, :NCLS]


# speedup vs baseline: 11.9716x; 1.0162x over previous
import functools

import jax
import jax.numpy as jnp
from jax import lax
from jax.experimental import pallas as pl
from jax.experimental.pallas import tpu as pltpu
from jax.experimental.pallas import tpu_sc as plsc

N = 10000
NSP = 500
C = 256
HIDE = 256
NCLS = 16
EM = 160000
EA = 8000
HEAD = 2
DH = HIDE // HEAD
NSF = 5
NPF = 2
RHP = 5
GAMA = 0.9

ROWS_BLK = 2000


def _prelin_proj_body(x_ref, w1_ref, b1_ref, wcat_ref, bcat_ref,
                      h0_ref, qcat_ref, vcat_ref):
    x = x_ref[...]
    h0 = jnp.dot(x, w1_ref[...], preferred_element_type=jnp.float32) + b1_ref[...]
    h0_ref[...] = h0
    cat = jnp.dot(h0, wcat_ref[...], preferred_element_type=jnp.float32) + bcat_ref[...]
    q = cat[:, :HIDE]
    v = cat[:, HIDE:]
    q2 = q.reshape(q.shape[0], HEAD, DH)
    m = jnp.mean(q2, axis=-1, keepdims=True)
    var = jnp.mean((q2 - m) ** 2, axis=-1, keepdims=True)
    qn = ((q2 - m) / jnp.sqrt(var + 1e-5)).reshape(q.shape)
    qcat_ref[...] = qn
    vcat_ref[...] = v


def _prelin_proj(x, w1t, b1, wcatt, bcat):
    grid = (N // ROWS_BLK,)
    return pl.pallas_call(
        _prelin_proj_body,
        grid=grid,
        in_specs=[
            pl.BlockSpec((ROWS_BLK, C), lambda i: (i, 0)),
            pl.BlockSpec((C, HIDE), lambda i: (0, 0)),
            pl.BlockSpec((1, HIDE), lambda i: (0, 0)),
            pl.BlockSpec((HIDE, 2 * HIDE), lambda i: (0, 0)),
            pl.BlockSpec((1, 2 * HIDE), lambda i: (0, 0)),
        ],
        out_specs=[
            pl.BlockSpec((ROWS_BLK, HIDE), lambda i: (i, 0)),
            pl.BlockSpec((ROWS_BLK, HIDE), lambda i: (i, 0)),
            pl.BlockSpec((ROWS_BLK, HIDE), lambda i: (i, 0)),
        ],
        out_shape=[
            jax.ShapeDtypeStruct((N, HIDE), jnp.float32),
            jax.ShapeDtypeStruct((N, HIDE), jnp.float32),
            jax.ShapeDtypeStruct((N, HIDE), jnp.float32),
        ],
    )(x, w1t, b1, wcatt, bcat)


def _leaky(x):
    return jnp.where(x >= 0, x, 0.01 * x)


NC = 2
NS = 16
NW = NC * NS
KCH = 128
NCHUNKS = EM // KCH
KPW = -(-NCHUNKS // NW)
NSP_S = 10240
SSTR = NSP_S // NS

_sc_mesh = plsc.VectorSubcoreMesh(core_axis_name="c", subcore_axis_name="s")


KA = 80
NCHA = EM // KA
KPA = -(-NCHA // NW)
KPA4 = ((KPA + 3) // 4) * 4


@functools.partial(
    pl.kernel,
    out_type=[
        jax.ShapeDtypeStruct((2, EM), jnp.float32),
        jax.ShapeDtypeStruct((NC, 2, NSP_S), jnp.float32),
    ],
    mesh=_sc_mesh,
    compiler_params=pltpu.CompilerParams(
        use_tc_tiling_on_sc=False, needs_layout_passes=False),
    scratch_types=[
        pltpu.VMEM((2, KA), jnp.int32),
        pltpu.VMEM((2, KA), jnp.int32),
        pltpu.VMEM((2, KA, HIDE), jnp.float32),
        pltpu.VMEM((2, KA, HIDE), jnp.float32),
        pltpu.VMEM((2, KA), jnp.float32),
        pltpu.VMEM((2, KA), jnp.float32),
        pltpu.VMEM((SSTR,), jnp.float32),
        pltpu.VMEM_SHARED((NSP_S,), jnp.float32),
        pltpu.VMEM_SHARED((NSP_S,), jnp.float32),
        [pltpu.SemaphoreType.DMA] * 2,
        [pltpu.SemaphoreType.DMA] * 2,
    ],
)
def _edge_a(qcat_hbm, src_hbm, dst_hbm, e_hbm, s_hbm,
            src_v, dst_v, qsrc_v, qdst_v, e0_v, e1_v, buf_v,
            s0_sh, s1_sh, si, sg):
    cid = lax.axis_index("c")
    sid = lax.axis_index("s")
    wid = cid * NS + sid

    def _zero(i, _):
        buf_v[pl.ds(i * 16, 16)] = jnp.zeros((16,), jnp.float32)
        return 0
    lax.fori_loop(0, SSTR // 16, _zero, 0)
    pltpu.sync_copy(buf_v, s0_sh.at[pl.ds(sid * SSTR, SSTR)])
    pltpu.sync_copy(buf_v, s1_sh.at[pl.ds(sid * SSTR, SSTR)])
    plsc.subcore_barrier()

    def _issue_idx(c, b4):
        @pl.when(c < NCHA)
        def _():
            base = c * KA
            pltpu.async_copy(src_hbm.at[pl.ds(base, KA)], src_v.at[b4], si[b4])
            pltpu.async_copy(dst_hbm.at[pl.ds(base, KA)], dst_v.at[b4], si[b4])

    def _issue_gather(c, b4, b2):
        @pl.when(c < NCHA)
        def _():
            base = c * KA
            pltpu.make_async_copy(src_hbm.at[pl.ds(base, KA)], src_v.at[b4], si[b4]).wait()
            pltpu.make_async_copy(dst_hbm.at[pl.ds(base, KA)], dst_v.at[b4], si[b4]).wait()
            pltpu.async_copy(qcat_hbm.at[src_v.at[b4]], qsrc_v.at[b2], sg[b2])
            pltpu.async_copy(qcat_hbm.at[dst_v.at[b4]], qdst_v.at[b2], sg[b2])

    def _compute(c, b4, b2):
        @pl.when(c < NCHA)
        def _():
            base = c * KA
            pltpu.make_async_copy(qcat_hbm.at[src_v.at[b4]], qsrc_v.at[b2], sg[b2]).wait()
            pltpu.make_async_copy(qcat_hbm.at[dst_v.at[b4]], qdst_v.at[b2], sg[b2]).wait()
            lastlane = lax.iota(jnp.int32, 16) == 15

            def _edge(ei, _):
                z = jnp.zeros((16,), jnp.float32)
                a0 = z
                a1 = z
                for cc in range(DH // 16):
                    a0 = a0 + qsrc_v[b2, ei, pl.ds(cc * 16, 16)] * qdst_v[b2, ei, pl.ds(cc * 16, 16)]
                    a1 = a1 + qsrc_v[b2, ei, pl.ds(DH + cc * 16, 16)] * qdst_v[b2, ei, pl.ds(DH + cc * 16, 16)]
                eidx = jnp.full((16,), ei, jnp.int32)
                plsc.store_scatter(e0_v.at[b4], [eidx], plsc.cumsum(a0), mask=lastlane)
                plsc.store_scatter(e1_v.at[b4], [eidx], plsc.cumsum(a1), mask=lastlane)
                return 0

            lax.fori_loop(0, KA, _edge, 0, unroll=2)

            def _vexp(g, _):
                sl = pl.ds(g * 16, 16)
                e0_v[b4, sl] = jnp.exp(e0_v[b4, sl] * (1.0 / HIDE))
                e1_v[b4, sl] = jnp.exp(e1_v[b4, sl] * (1.0 / HIDE))
                return 0

            lax.fori_loop(0, KA // 16, _vexp, 0)
            pltpu.sync_copy(e0_v.at[b4], e_hbm.at[0, pl.ds(base, KA)])
            pltpu.sync_copy(e1_v.at[b4], e_hbm.at[1, pl.ds(base, KA)])
            pltpu.sync_copy(e0_v.at[b4], s0_sh.at[src_v.at[b4]], add=True)
            pltpu.sync_copy(e1_v.at[b4], s1_sh.at[src_v.at[b4]], add=True)

    _issue_idx(wid, 0)
    _issue_idx(wid + NW, 1)
    _issue_gather(wid, 0, 0)

    def _pair(k2, _):
        for b in range(2):
            k = 2 * k2 + b
            c = wid + NW * k
            _issue_gather(c + NW, 1 - b, 1 - b)
            _compute(c, b, b)
            _issue_idx(c + 2 * NW, b)
        return 0

    lax.fori_loop(0, KPA4 // 2, _pair, 0)
    plsc.subcore_barrier()

    pltpu.sync_copy(s0_sh.at[pl.ds(sid * SSTR, SSTR)], buf_v)
    pltpu.sync_copy(buf_v, s_hbm.at[cid, 0, pl.ds(sid * SSTR, SSTR)])
    pltpu.sync_copy(s1_sh.at[pl.ds(sid * SSTR, SSTR)], buf_v)
    pltpu.sync_copy(buf_v, s_hbm.at[cid, 1, pl.ds(sid * SSTR, SSTR)])


NSP_P = 512


def _mid_body(o_ref, g_ref, b_ref, wcat_ref, bcat_ref, qcat_ref, vcat_ref):
    o = o_ref[...]
    h = jnp.concatenate([o[0] + o[2], o[1] + o[3]], axis=1)
    h = h * g_ref[...] + b_ref[...]
    h = jnp.where(h >= 0, h, 0.01 * h)
    cat = jnp.dot(h, wcat_ref[...], preferred_element_type=jnp.float32) + bcat_ref[...]
    q = cat[:, :HIDE]
    q2 = q.reshape(q.shape[0], HEAD, DH)
    m = jnp.mean(q2, axis=-1, keepdims=True)
    var = jnp.mean((q2 - m) ** 2, axis=-1, keepdims=True)
    qcat_ref[...] = ((q2 - m) / jnp.sqrt(var + 1e-5)).reshape(q.shape)
    vcat_ref[...] = cat[:, HIDE:]


def _mid(outPr, g, b, wcatt, bcat):
    return pl.pallas_call(
        _mid_body,
        grid=(N // ROWS_BLK,),
        in_specs=[
            pl.BlockSpec((4, ROWS_BLK, DH), lambda i: (0, i, 0)),
            pl.BlockSpec((1, HIDE), lambda i: (0, 0)),
            pl.BlockSpec((1, HIDE), lambda i: (0, 0)),
            pl.BlockSpec((HIDE, 2 * HIDE), lambda i: (0, 0)),
            pl.BlockSpec((1, 2 * HIDE), lambda i: (0, 0)),
        ],
        out_specs=[
            pl.BlockSpec((ROWS_BLK, HIDE), lambda i: (i, 0)),
            pl.BlockSpec((ROWS_BLK, HIDE), lambda i: (i, 0)),
        ],
        out_shape=[
            jax.ShapeDtypeStruct((N, HIDE), jnp.float32),
            jax.ShapeDtypeStruct((N, HIDE), jnp.float32),
        ],
    )(outPr, g, b, wcatt, bcat)


def _sf_body(hpP_ref, degP_ref, AP_ref, wt_ref, b_ref, g_ref, be_ref, out_ref):
    degP = degP_ref[...]
    deg = degP[0] + degP[1]
    inv = jnp.where(deg > 0, 1.0 / deg, 0.0)
    hp = (hpP_ref[0] + hpP_ref[1]) * inv[:, None]
    A = AP_ref[0] + AP_ref[1]
    for i in range(NSF):
        hlin = jnp.dot(hp, wt_ref[i], preferred_element_type=jnp.float32) + b_ref[i]
        x_start = jnp.dot(A, hlin, preferred_element_type=jnp.float32)
        xc = hp
        for _ in range(RHP):
            xc = (jnp.dot(A, xc, preferred_element_type=jnp.float32) + x_start) * (1.0 / (2.0 + GAMA))
        hp = xc * g_ref[i] + be_ref[i]
        hp = jnp.where(hp >= 0, hp, 0.01 * hp)
    out_ref[...] = hp


def _sf_dense(hpP, degP, AP, wt, b, g, be):
    return pl.pallas_call(
        _sf_body,
        out_shape=jax.ShapeDtypeStruct((NSP_P, HIDE), jnp.float32),
    )(hpP, degP, AP, wt, b, g, be)


def _fin_body(o_ref, g_ref, b_ref, qa_ref, hp_ref, wout_ref, bout_ref, y_ref):
    o = o_ref[...]
    h = jnp.concatenate([o[0] + o[2], o[1] + o[3]], axis=1)
    h = h * g_ref[...] + b_ref[...]
    pf = jnp.where(h >= 0, h, 0.01 * h)
    qa = qa_ref[...]
    oh = (qa == jax.lax.broadcasted_iota(jnp.int32, (qa.shape[0], NSP_P), 1)
          ).astype(jnp.float32)
    sf = jnp.dot(oh, hp_ref[...], preferred_element_type=jnp.float32)
    H1 = pf + sf
    logits = jnp.dot(H1, wout_ref[...], preferred_element_type=jnp.float32) + bout_ref[...]
    mx = jnp.max(logits, axis=-1, keepdims=True)
    e = jnp.exp(logits - mx)
    y_ref[...] = e / jnp.sum(e, axis=-1, keepdims=True)


def _final(outPr, g, b, qa2, hp, wout, bout):
    return pl.pallas_call(
        _fin_body,
        grid=(N // ROWS_BLK,),
        in_specs=[
            pl.BlockSpec((4, ROWS_BLK, DH), lambda i: (0, i, 0)),
            pl.BlockSpec((1, HIDE), lambda i: (0, 0)),
            pl.BlockSpec((1, HIDE), lambda i: (0, 0)),
            pl.BlockSpec((ROWS_BLK, 1), lambda i: (i, 0)),
            pl.BlockSpec((NSP_P, HIDE), lambda i: (0, 0)),
            pl.BlockSpec((HIDE, 128), lambda i: (0, 0)),
            pl.BlockSpec((1, 128), lambda i: (0, 0)),
        ],
        out_specs=pl.BlockSpec((ROWS_BLK, 128), lambda i: (i, 0)),
        out_shape=jax.ShapeDtypeStruct((N, 128), jnp.float32),
    )(outPr, g, b, qa2, hp, wout, bout)


RPW = N // NS
RPP = 125
KB = 40
NCHB = EM // KB
KPB = -(-NCHB // NW)


@functools.partial(
    pl.kernel,
    out_type=jax.ShapeDtypeStruct((NC, HEAD, N, DH), jnp.float32),
    mesh=_sc_mesh,
    compiler_params=pltpu.CompilerParams(
        use_tc_tiling_on_sc=False, needs_layout_passes=False),
    scratch_types=[
        pltpu.VMEM((2, KB), jnp.int32),
        pltpu.VMEM((2, KB), jnp.int32),
        pltpu.VMEM((2, KB, DH), jnp.float32),
        pltpu.VMEM((2, KB), jnp.float32),
        pltpu.VMEM((KB,), jnp.float32),
        pltpu.VMEM((RPP, DH), jnp.float32),
        pltpu.VMEM((NSP_S,), jnp.float32),
        pltpu.VMEM((NSP_S,), jnp.float32),
        pltpu.VMEM_SHARED((N, DH), jnp.float32),
        [pltpu.SemaphoreType.DMA] * 2,
        [pltpu.SemaphoreType.DMA] * 2,
    ],
)
def _edge_b(v0_hbm, v1_hbm, e_hbm, s_hbm, src_hbm, dst_hbm, out_hbm,
            src_v, dst_v, vrow_v, e_v, w_v, zbuf_v, sA_v, sB_v, out_sh, si, sg):
    cid = lax.axis_index("c")
    sid = lax.axis_index("s")
    wid = cid * NS + sid

    for hh in range(HEAD):
        v_hbm = v0_hbm if hh == 0 else v1_hbm
        pltpu.sync_copy(s_hbm.at[0, hh], sA_v)
        pltpu.sync_copy(s_hbm.at[1, hh], sB_v)

        def _sadd(i, _):
            sA_v[pl.ds(i * 16, 16)] = sA_v[pl.ds(i * 16, 16)] + sB_v[pl.ds(i * 16, 16)]
            return 0
        lax.fori_loop(0, NSP_S // 16, _sadd, 0)

        def _zrow2(i, _):
            def _zc(c, _):
                zbuf_v[i, pl.ds(c * 16, 16)] = jnp.zeros((16,), jnp.float32)
                return 0
            lax.fori_loop(0, DH // 16, _zc, 0)
            return 0
        lax.fori_loop(0, RPP, _zrow2, 0)
        for p in range(RPW // RPP):
            pltpu.sync_copy(zbuf_v,
                            out_sh.at[pl.ds(sid * RPW + p * RPP, RPP), :])
        plsc.subcore_barrier()

        def _issue_idx(c, b):
            @pl.when(c < NCHB)
            def _():
                base = c * KB
                pltpu.async_copy(src_hbm.at[pl.ds(base, KB)], src_v.at[b], si[b])
                pltpu.async_copy(dst_hbm.at[pl.ds(base, KB)], dst_v.at[b], si[b])
                pltpu.async_copy(e_hbm.at[hh, pl.ds(base, KB)], e_v.at[b], si[b])

        def _issue_gather(c, b):
            @pl.when(c < NCHB)
            def _():
                base = c * KB
                pltpu.make_async_copy(src_hbm.at[pl.ds(base, KB)], src_v.at[b], si[b]).wait()
                pltpu.make_async_copy(dst_hbm.at[pl.ds(base, KB)], dst_v.at[b], si[b]).wait()
                pltpu.make_async_copy(e_hbm.at[hh, pl.ds(base, KB)], e_v.at[b], si[b]).wait()
                pltpu.async_copy(v_hbm.at[dst_v.at[b]], vrow_v.at[b], sg[b])

        def _compute(c, b):
            @pl.when(c < NCHB)
            def _():
                pltpu.make_async_copy(v_hbm.at[dst_v.at[b]], vrow_v.at[b], sg[b]).wait()

                def _wgrp(g, _):
                    sl = pl.ds(g * 16, 16)
                    sv = plsc.load_gather(sA_v, [src_v[b, sl]])
                    w_v[sl] = e_v[b, sl] / sv
                    return 0
                lax.fori_loop(0, KB // 16, _wgrp, 0)

                def _scale(ei, _):
                    wb = plsc.load_gather(w_v, [jnp.full((16,), ei, jnp.int32)])
                    for c2 in range(DH // 16):
                        vrow_v[b, ei, pl.ds(c2 * 16, 16)] = vrow_v[b, ei, pl.ds(c2 * 16, 16)] * wb
                    return 0
                lax.fori_loop(0, KB, _scale, 0, unroll=2)

                pltpu.sync_copy(vrow_v.at[b], out_sh.at[src_v.at[b]], add=True)

        _issue_idx(wid, 0)
        _issue_idx(wid + NW, 1)
        _issue_gather(wid, 0)

        def _pair(k2, _):
            for b in range(2):
                k = 2 * k2 + b
                c = wid + NW * k
                _issue_gather(c + NW, 1 - b)
                _compute(c, b)
                _issue_idx(c + 2 * NW, b)
            return 0

        lax.fori_loop(0, (KPB + 1) // 2, _pair, 0)
        plsc.subcore_barrier()

        for p in range(RPW // RPP):
            r0 = sid * RPW + p * RPP
            pltpu.sync_copy(out_sh.at[pl.ds(r0, RPP), :], zbuf_v)
            pltpu.sync_copy(zbuf_v,
                            out_hbm.at[cid, hh, pl.ds(r0, RPP), :])
        if hh == 0:
            plsc.subcore_barrier()


NCH_P = N // KCH
PTAIL = N - NCH_P * KCH
KPB_P = -(-NCH_P // NW)
SPSTR = NSP_P // NS


@functools.partial(
    pl.kernel,
    out_type=[
        jax.ShapeDtypeStruct((NC, NSP_P, HIDE), jnp.float32),
        jax.ShapeDtypeStruct((NC, NSP_P), jnp.float32),
    ],
    mesh=_sc_mesh,
    compiler_params=pltpu.CompilerParams(
        use_tc_tiling_on_sc=False, needs_layout_passes=False),
    scratch_types=[
        pltpu.VMEM((KCH,), jnp.int32),
        pltpu.VMEM((KCH, HIDE), jnp.float32),
        pltpu.VMEM((KCH,), jnp.float32),
        pltpu.VMEM((PTAIL,), jnp.int32),
        pltpu.VMEM((PTAIL, HIDE), jnp.float32),
        pltpu.VMEM((PTAIL,), jnp.float32),
        pltpu.VMEM((SPSTR,), jnp.float32),
        pltpu.VMEM_SHARED((NSP_P, HIDE), jnp.float32),
        pltpu.VMEM_SHARED((NSP_P,), jnp.float32),
    ],
)
def _pool(x_hbm, qa_hbm, hp_hbm, deg_hbm,
          qa_v, rows_v, ones_v, qat_v, rowst_v, onest_v, buf_v,
          hp_sh, deg_sh):
    cid = lax.axis_index("c")
    sid = lax.axis_index("s")
    wid = cid * NS + sid

    def _fill(ref, n, val):
        def _f(i, _):
            ref[pl.ds(i * 16, 16)] = jnp.full((16,), val, jnp.float32)
            return 0
        lax.fori_loop(0, n // 16, _f, 0)

    _fill(ones_v, KCH, 1.0)
    _fill(onest_v, PTAIL, 1.0)
    _fill(buf_v, SPSTR, 0.0)

    def _zrow(i, _):
        def _zc(c, _):
            rows_v[i, pl.ds(c * 16, 16)] = jnp.zeros((16,), jnp.float32)
            return 0
        lax.fori_loop(0, HIDE // 16, _zc, 0)
        return 0
    lax.fori_loop(0, SPSTR, _zrow, 0)
    pltpu.sync_copy(rows_v.at[pl.ds(0, SPSTR), :],
                    hp_sh.at[pl.ds(sid * SPSTR, SPSTR), :])
    pltpu.sync_copy(buf_v, deg_sh.at[pl.ds(sid * SPSTR, SPSTR)])
    plsc.subcore_barrier()

    def _chunk(k, _):
        cidx = wid + NW * k

        @pl.when(cidx < NCH_P)
        def _():
            base = cidx * KCH
            pltpu.sync_copy(qa_hbm.at[pl.ds(base, KCH)], qa_v)
            pltpu.sync_copy(x_hbm.at[pl.ds(base, KCH), :], rows_v)
            pltpu.sync_copy(rows_v, hp_sh.at[qa_v], add=True)
            pltpu.sync_copy(ones_v, deg_sh.at[qa_v], add=True)
        return 0

    lax.fori_loop(0, KPB_P, _chunk, 0)

    @pl.when(wid == 0)
    def _tail():
        base = NCH_P * KCH
        pltpu.sync_copy(qa_hbm.at[pl.ds(base, PTAIL)], qat_v)
        pltpu.sync_copy(x_hbm.at[pl.ds(base, PTAIL), :], rowst_v)
        pltpu.sync_copy(rowst_v, hp_sh.at[qat_v], add=True)
        pltpu.sync_copy(onest_v, deg_sh.at[qat_v], add=True)

    plsc.subcore_barrier()
    r0 = sid * SPSTR
    pltpu.sync_copy(hp_sh.at[pl.ds(r0, SPSTR), :], rows_v.at[pl.ds(0, SPSTR), :])
    pltpu.sync_copy(rows_v.at[pl.ds(0, SPSTR), :], hp_hbm.at[cid, pl.ds(r0, SPSTR), :])
    pltpu.sync_copy(deg_sh.at[pl.ds(r0, SPSTR)], buf_v)
    pltpu.sync_copy(buf_v, deg_hbm.at[cid, pl.ds(r0, SPSTR)])


NCH_A = EA // KCH
ATAIL = EA - NCH_A * KCH
KPB_A = -(-NCH_A // NW)
AFL = NSP_P * NSP_P
ASTR = AFL // NS


@functools.partial(
    pl.kernel,
    out_type=jax.ShapeDtypeStruct((NC, AFL), jnp.float32),
    mesh=_sc_mesh,
    compiler_params=pltpu.CompilerParams(
        use_tc_tiling_on_sc=False, needs_layout_passes=False),
    scratch_types=[
        pltpu.VMEM((KCH,), jnp.int32),
        pltpu.VMEM((KCH,), jnp.int32),
        pltpu.VMEM((KCH,), jnp.float32),
        pltpu.VMEM((KCH,), jnp.int32),
        pltpu.VMEM((ATAIL,), jnp.int32),
        pltpu.VMEM((ATAIL,), jnp.int32),
        pltpu.VMEM((ATAIL,), jnp.float32),
        pltpu.VMEM((ATAIL,), jnp.int32),
        pltpu.VMEM((ASTR,), jnp.float32),
        pltpu.VMEM_SHARED((AFL,), jnp.float32),
    ],
)
def _abuild(asrc_hbm, adst_hbm, aval_hbm, A_hbm,
            s_v, d_v, v_v, fidx_v, st_v, dt_v, vt_v, fidxt_v, zbuf_v, A_sh):
    cid = lax.axis_index("c")
    sid = lax.axis_index("s")
    wid = cid * NS + sid

    def _f(i, _):
        zbuf_v[pl.ds(i * 16, 16)] = jnp.zeros((16,), jnp.float32)
        return 0
    lax.fori_loop(0, ASTR // 16, _f, 0)
    pltpu.sync_copy(zbuf_v, A_sh.at[pl.ds(sid * ASTR, ASTR)])
    plsc.subcore_barrier()

    def _chunk(k, _):
        cidx = wid + NW * k

        @pl.when(cidx < NCH_A)
        def _():
            base = cidx * KCH
            pltpu.sync_copy(asrc_hbm.at[pl.ds(base, KCH)], s_v)
            pltpu.sync_copy(adst_hbm.at[pl.ds(base, KCH)], d_v)
            pltpu.sync_copy(aval_hbm.at[pl.ds(base, KCH)], v_v)

            def _fi(g, _):
                sl = pl.ds(g * 16, 16)
                fidx_v[sl] = d_v[sl] * NSP_P + s_v[sl]
                return 0
            lax.fori_loop(0, KCH // 16, _fi, 0)
            pltpu.sync_copy(v_v, A_sh.at[fidx_v], add=True)
        return 0

    lax.fori_loop(0, KPB_A, _chunk, 0)

    @pl.when(wid == 0)
    def _tail():
        base = NCH_A * KCH
        pltpu.sync_copy(asrc_hbm.at[pl.ds(base, ATAIL)], st_v)
        pltpu.sync_copy(adst_hbm.at[pl.ds(base, ATAIL)], dt_v)
        pltpu.sync_copy(aval_hbm.at[pl.ds(base, ATAIL)], vt_v)

        def _fi(g, _):
            sl = pl.ds(g * 16, 16)
            fidxt_v[sl] = dt_v[sl] * NSP_P + st_v[sl]
            return 0
        lax.fori_loop(0, ATAIL // 16, _fi, 0)
        pltpu.sync_copy(vt_v, A_sh.at[fidxt_v], add=True)

    plsc.subcore_barrier()
    pltpu.sync_copy(A_sh.at[pl.ds(sid * ASTR, ASTR)], zbuf_v)
    pltpu.sync_copy(zbuf_v, A_hbm.at[cid, pl.ds(sid * ASTR, ASTR)])


def kernel(x, mask_index, mask_value, q_assign, a_index, a_value,
           prelin_W, prelin_b, bn0_g, bn0_b,
           sf_W, sf_b, sf_g, sf_be,
           pf_Wv, pf_bv, pf_Wq, pf_bq, pf_g, pf_be,
           out_W, out_b):
    bnscale = 1.0 / jnp.sqrt(1.0 + 1e-5)
    w1 = prelin_W * (bn0_g * bnscale)[:, None]
    b1 = prelin_b * (bn0_g * bnscale) + bn0_b

    src = mask_index[0]
    dst = mask_index[1]

    outPr = None
    for j in range(NPF):
        wcat = jnp.concatenate([pf_Wq[j, 0], pf_Wq[j, 1], pf_Wv[j, 0], pf_Wv[j, 1]], axis=0)
        bcat = jnp.concatenate([pf_bq[j, 0], pf_bq[j, 1], pf_bv[j, 0], pf_bv[j, 1]], axis=0)
        if j == 0:
            h0, qcat, vcat = _prelin_proj(x, w1.T, b1[None, :], wcat.T, bcat[None, :])
            x_pre = h0
        else:
            g0 = (pf_g[0] * bnscale)[None, :]
            b0 = pf_be[0][None, :]
            qcat, vcat = _mid(outPr, g0, b0, wcat.T, bcat[None, :])
        e2, sP = _edge_a(qcat, src, dst)
        outP = _edge_b(vcat[:, :DH], vcat[:, DH:], e2, sP, src, dst)
        outPr = outP.reshape(NC * HEAD, N, DH)

    hpP, degP = _pool(x_pre, q_assign)
    APf = _abuild(a_index[0], a_index[1], a_value)
    AP = APf.reshape(NC, NSP_P, NSP_P)
    hp_fin = _sf_dense(hpP, degP, AP, jnp.transpose(sf_W, (0, 2, 1)),
                       sf_b, sf_g * bnscale, sf_be)

    g1 = (pf_g[1] * bnscale)[None, :]
    b1f = pf_be[1][None, :]
    woutp = jnp.zeros((HIDE, 128), jnp.float32).at[:, :NCLS].set(out_W.T)
    boutp = jnp.full((128,), -1e30, jnp.float32).at[:NCLS].set(out_b)
    ypad = _final(outPr, g1, b1f, q_assign.reshape(N, 1), hp_fin,
                  woutp, boutp[None, :])
    return ypad[:, :NCLS]


# speedup vs baseline: 12.0530x; 1.0068x over previous
import functools

import jax
import jax.numpy as jnp
from jax import lax
from jax.experimental import pallas as pl
from jax.experimental.pallas import tpu as pltpu
from jax.experimental.pallas import tpu_sc as plsc

N = 10000
NSP = 500
C = 256
HIDE = 256
NCLS = 16
EM = 160000
EA = 8000
HEAD = 2
DH = HIDE // HEAD
NSF = 5
NPF = 2
RHP = 5
GAMA = 0.9

ROWS_BLK = 2000


def _prelin_proj_body(x_ref, w1_ref, b1_ref, wcat_ref, bcat_ref,
                      h0_ref, qcat_ref, vcat_ref):
    x = x_ref[...]
    h0 = jnp.dot(x, w1_ref[...], preferred_element_type=jnp.float32) + b1_ref[...]
    h0_ref[...] = h0
    cat = jnp.dot(h0, wcat_ref[...], preferred_element_type=jnp.float32) + bcat_ref[...]
    q = cat[:, :HIDE]
    v = cat[:, HIDE:]
    q2 = q.reshape(q.shape[0], HEAD, DH)
    m = jnp.mean(q2, axis=-1, keepdims=True)
    var = jnp.mean((q2 - m) ** 2, axis=-1, keepdims=True)
    qn = ((q2 - m) / jnp.sqrt(var + 1e-5)).reshape(q.shape)
    qcat_ref[...] = qn
    vcat_ref[...] = v


def _prelin_proj(x, w1t, b1, wcatt, bcat):
    grid = (N // ROWS_BLK,)
    return pl.pallas_call(
        _prelin_proj_body,
        grid=grid,
        in_specs=[
            pl.BlockSpec((ROWS_BLK, C), lambda i: (i, 0)),
            pl.BlockSpec((C, HIDE), lambda i: (0, 0)),
            pl.BlockSpec((1, HIDE), lambda i: (0, 0)),
            pl.BlockSpec((HIDE, 2 * HIDE), lambda i: (0, 0)),
            pl.BlockSpec((1, 2 * HIDE), lambda i: (0, 0)),
        ],
        out_specs=[
            pl.BlockSpec((ROWS_BLK, HIDE), lambda i: (i, 0)),
            pl.BlockSpec((ROWS_BLK, HIDE), lambda i: (i, 0)),
            pl.BlockSpec((ROWS_BLK, HIDE), lambda i: (i, 0)),
        ],
        out_shape=[
            jax.ShapeDtypeStruct((N, HIDE), jnp.float32),
            jax.ShapeDtypeStruct((N, HIDE), jnp.float32),
            jax.ShapeDtypeStruct((N, HIDE), jnp.float32),
        ],
    )(x, w1t, b1, wcatt, bcat)


def _leaky(x):
    return jnp.where(x >= 0, x, 0.01 * x)


NC = 2
NS = 16
NW = NC * NS
KCH = 128
NCHUNKS = EM // KCH
KPW = -(-NCHUNKS // NW)
NSP_S = 10240
SSTR = NSP_S // NS

_sc_mesh = plsc.VectorSubcoreMesh(core_axis_name="c", subcore_axis_name="s")


KA = 80
NCHA = EM // KA
KPA = -(-NCHA // NW)
KPA4 = ((KPA + 3) // 4) * 4


@functools.partial(
    pl.kernel,
    out_type=[
        jax.ShapeDtypeStruct((2, EM), jnp.float32),
        jax.ShapeDtypeStruct((NC, 2, NSP_S), jnp.float32),
    ],
    mesh=_sc_mesh,
    compiler_params=pltpu.CompilerParams(
        use_tc_tiling_on_sc=False, needs_layout_passes=False),
    scratch_types=[
        pltpu.VMEM((2, KA), jnp.int32),
        pltpu.VMEM((2, KA), jnp.int32),
        pltpu.VMEM((2, KA, HIDE), jnp.float32),
        pltpu.VMEM((2, KA, HIDE), jnp.float32),
        pltpu.VMEM((2, KA), jnp.float32),
        pltpu.VMEM((2, KA), jnp.float32),
        pltpu.VMEM((SSTR,), jnp.float32),
        pltpu.VMEM_SHARED((NSP_S,), jnp.float32),
        pltpu.VMEM_SHARED((NSP_S,), jnp.float32),
        [pltpu.SemaphoreType.DMA] * 2,
        [pltpu.SemaphoreType.DMA] * 2,
    ],
)
def _edge_a(qcat_hbm, src_hbm, dst_hbm, e_hbm, s_hbm,
            src_v, dst_v, qsrc_v, qdst_v, e0_v, e1_v, buf_v,
            s0_sh, s1_sh, si, sg):
    cid = lax.axis_index("c")
    sid = lax.axis_index("s")
    wid = cid * NS + sid

    def _zero(i, _):
        buf_v[pl.ds(i * 16, 16)] = jnp.zeros((16,), jnp.float32)
        return 0
    lax.fori_loop(0, SSTR // 16, _zero, 0)
    pltpu.sync_copy(buf_v, s0_sh.at[pl.ds(sid * SSTR, SSTR)])
    pltpu.sync_copy(buf_v, s1_sh.at[pl.ds(sid * SSTR, SSTR)])
    plsc.subcore_barrier()

    def _issue_idx(c, b4):
        @pl.when(c < NCHA)
        def _():
            base = c * KA
            pltpu.async_copy(src_hbm.at[pl.ds(base, KA)], src_v.at[b4], si[b4])
            pltpu.async_copy(dst_hbm.at[pl.ds(base, KA)], dst_v.at[b4], si[b4])

    def _issue_gather(c, b4, b2):
        @pl.when(c < NCHA)
        def _():
            base = c * KA
            pltpu.make_async_copy(src_hbm.at[pl.ds(base, KA)], src_v.at[b4], si[b4]).wait()
            pltpu.make_async_copy(dst_hbm.at[pl.ds(base, KA)], dst_v.at[b4], si[b4]).wait()
            pltpu.async_copy(qcat_hbm.at[src_v.at[b4]], qsrc_v.at[b2], sg[b2])
            pltpu.async_copy(qcat_hbm.at[dst_v.at[b4]], qdst_v.at[b2], sg[b2])

    def _compute(c, b4, b2):
        @pl.when(c < NCHA)
        def _():
            base = c * KA
            pltpu.make_async_copy(qcat_hbm.at[src_v.at[b4]], qsrc_v.at[b2], sg[b2]).wait()
            pltpu.make_async_copy(qcat_hbm.at[dst_v.at[b4]], qdst_v.at[b2], sg[b2]).wait()
            lastlane = lax.iota(jnp.int32, 16) == 15

            def _edge(ei, _):
                z = jnp.zeros((16,), jnp.float32)
                a0 = z
                a1 = z
                for cc in range(DH // 16):
                    a0 = a0 + qsrc_v[b2, ei, pl.ds(cc * 16, 16)] * qdst_v[b2, ei, pl.ds(cc * 16, 16)]
                    a1 = a1 + qsrc_v[b2, ei, pl.ds(DH + cc * 16, 16)] * qdst_v[b2, ei, pl.ds(DH + cc * 16, 16)]
                eidx = jnp.full((16,), ei, jnp.int32)
                plsc.store_scatter(e0_v.at[b4], [eidx], plsc.cumsum(a0), mask=lastlane)
                plsc.store_scatter(e1_v.at[b4], [eidx], plsc.cumsum(a1), mask=lastlane)
                return 0

            lax.fori_loop(0, KA, _edge, 0, unroll=4)

            def _vexp(g, _):
                sl = pl.ds(g * 16, 16)
                e0_v[b4, sl] = jnp.exp(e0_v[b4, sl] * (1.0 / HIDE))
                e1_v[b4, sl] = jnp.exp(e1_v[b4, sl] * (1.0 / HIDE))
                return 0

            lax.fori_loop(0, KA // 16, _vexp, 0)
            pltpu.sync_copy(e0_v.at[b4], e_hbm.at[0, pl.ds(base, KA)])
            pltpu.sync_copy(e1_v.at[b4], e_hbm.at[1, pl.ds(base, KA)])
            pltpu.sync_copy(e0_v.at[b4], s0_sh.at[src_v.at[b4]], add=True)
            pltpu.sync_copy(e1_v.at[b4], s1_sh.at[src_v.at[b4]], add=True)

    _issue_idx(wid, 0)
    _issue_idx(wid + NW, 1)
    _issue_gather(wid, 0, 0)

    def _pair(k2, _):
        for b in range(2):
            k = 2 * k2 + b
            c = wid + NW * k
            _issue_gather(c + NW, 1 - b, 1 - b)
            _compute(c, b, b)
            _issue_idx(c + 2 * NW, b)
        return 0

    lax.fori_loop(0, KPA4 // 2, _pair, 0)
    plsc.subcore_barrier()

    pltpu.sync_copy(s0_sh.at[pl.ds(sid * SSTR, SSTR)], buf_v)
    pltpu.sync_copy(buf_v, s_hbm.at[cid, 0, pl.ds(sid * SSTR, SSTR)])
    pltpu.sync_copy(s1_sh.at[pl.ds(sid * SSTR, SSTR)], buf_v)
    pltpu.sync_copy(buf_v, s_hbm.at[cid, 1, pl.ds(sid * SSTR, SSTR)])


NSP_P = 512


def _mid_body(o_ref, g_ref, b_ref, wcat_ref, bcat_ref, qcat_ref, vcat_ref):
    o = o_ref[...]
    h = jnp.concatenate([o[0] + o[2], o[1] + o[3]], axis=1)
    h = h * g_ref[...] + b_ref[...]
    h = jnp.where(h >= 0, h, 0.01 * h)
    cat = jnp.dot(h, wcat_ref[...], preferred_element_type=jnp.float32) + bcat_ref[...]
    q = cat[:, :HIDE]
    q2 = q.reshape(q.shape[0], HEAD, DH)
    m = jnp.mean(q2, axis=-1, keepdims=True)
    var = jnp.mean((q2 - m) ** 2, axis=-1, keepdims=True)
    qcat_ref[...] = ((q2 - m) / jnp.sqrt(var + 1e-5)).reshape(q.shape)
    vcat_ref[...] = cat[:, HIDE:]


def _mid(outPr, g, b, wcatt, bcat):
    return pl.pallas_call(
        _mid_body,
        grid=(N // ROWS_BLK,),
        in_specs=[
            pl.BlockSpec((4, ROWS_BLK, DH), lambda i: (0, i, 0)),
            pl.BlockSpec((1, HIDE), lambda i: (0, 0)),
            pl.BlockSpec((1, HIDE), lambda i: (0, 0)),
            pl.BlockSpec((HIDE, 2 * HIDE), lambda i: (0, 0)),
            pl.BlockSpec((1, 2 * HIDE), lambda i: (0, 0)),
        ],
        out_specs=[
            pl.BlockSpec((ROWS_BLK, HIDE), lambda i: (i, 0)),
            pl.BlockSpec((ROWS_BLK, HIDE), lambda i: (i, 0)),
        ],
        out_shape=[
            jax.ShapeDtypeStruct((N, HIDE), jnp.float32),
            jax.ShapeDtypeStruct((N, HIDE), jnp.float32),
        ],
    )(outPr, g, b, wcatt, bcat)


def _sf_body(hpP_ref, degP_ref, AP_ref, wt_ref, b_ref, g_ref, be_ref, out_ref):
    degP = degP_ref[...]
    deg = degP[0] + degP[1]
    inv = jnp.where(deg > 0, 1.0 / deg, 0.0)
    hp = (hpP_ref[0] + hpP_ref[1]) * inv[:, None]
    A = AP_ref[0] + AP_ref[1]
    for i in range(NSF):
        hlin = jnp.dot(hp, wt_ref[i], preferred_element_type=jnp.float32) + b_ref[i]
        x_start = jnp.dot(A, hlin, preferred_element_type=jnp.float32)
        xc = hp
        for _ in range(RHP):
            xc = (jnp.dot(A, xc, preferred_element_type=jnp.float32) + x_start) * (1.0 / (2.0 + GAMA))
        hp = xc * g_ref[i] + be_ref[i]
        hp = jnp.where(hp >= 0, hp, 0.01 * hp)
    out_ref[...] = hp


def _sf_dense(hpP, degP, AP, wt, b, g, be):
    return pl.pallas_call(
        _sf_body,
        out_shape=jax.ShapeDtypeStruct((NSP_P, HIDE), jnp.float32),
    )(hpP, degP, AP, wt, b, g, be)


def _fin_body(o_ref, g_ref, b_ref, qa_ref, hp_ref, wout_ref, bout_ref, y_ref):
    o = o_ref[...]
    h = jnp.concatenate([o[0] + o[2], o[1] + o[3]], axis=1)
    h = h * g_ref[...] + b_ref[...]
    pf = jnp.where(h >= 0, h, 0.01 * h)
    qa = qa_ref[...]
    oh = (qa == jax.lax.broadcasted_iota(jnp.int32, (qa.shape[0], NSP_P), 1)
          ).astype(jnp.float32)
    sf = jnp.dot(oh, hp_ref[...], preferred_element_type=jnp.float32)
    H1 = pf + sf
    logits = jnp.dot(H1, wout_ref[...], preferred_element_type=jnp.float32) + bout_ref[...]
    mx = jnp.max(logits, axis=-1, keepdims=True)
    e = jnp.exp(logits - mx)
    y_ref[...] = e / jnp.sum(e, axis=-1, keepdims=True)


def _final(outPr, g, b, qa2, hp, wout, bout):
    return pl.pallas_call(
        _fin_body,
        grid=(N // ROWS_BLK,),
        in_specs=[
            pl.BlockSpec((4, ROWS_BLK, DH), lambda i: (0, i, 0)),
            pl.BlockSpec((1, HIDE), lambda i: (0, 0)),
            pl.BlockSpec((1, HIDE), lambda i: (0, 0)),
            pl.BlockSpec((ROWS_BLK, 1), lambda i: (i, 0)),
            pl.BlockSpec((NSP_P, HIDE), lambda i: (0, 0)),
            pl.BlockSpec((HIDE, 128), lambda i: (0, 0)),
            pl.BlockSpec((1, 128), lambda i: (0, 0)),
        ],
        out_specs=pl.BlockSpec((ROWS_BLK, 128), lambda i: (i, 0)),
        out_shape=jax.ShapeDtypeStruct((N, 128), jnp.float32),
    )(outPr, g, b, qa2, hp, wout, bout)


RPW = N // NS
RPP = 125
KB = 40
NCHB = EM // KB
KPB = -(-NCHB // NW)


@functools.partial(
    pl.kernel,
    out_type=jax.ShapeDtypeStruct((NC, HEAD, N, DH), jnp.float32),
    mesh=_sc_mesh,
    compiler_params=pltpu.CompilerParams(
        use_tc_tiling_on_sc=False, needs_layout_passes=False),
    scratch_types=[
        pltpu.VMEM((2, KB), jnp.int32),
        pltpu.VMEM((2, KB), jnp.int32),
        pltpu.VMEM((2, KB, DH), jnp.float32),
        pltpu.VMEM((2, KB), jnp.float32),
        pltpu.VMEM((KB,), jnp.float32),
        pltpu.VMEM((RPP, DH), jnp.float32),
        pltpu.VMEM((NSP_S,), jnp.float32),
        pltpu.VMEM((NSP_S,), jnp.float32),
        pltpu.VMEM_SHARED((N, DH), jnp.float32),
        [pltpu.SemaphoreType.DMA] * 2,
        [pltpu.SemaphoreType.DMA] * 2,
    ],
)
def _edge_b(v0_hbm, v1_hbm, e_hbm, s_hbm, src_hbm, dst_hbm, out_hbm,
            src_v, dst_v, vrow_v, e_v, w_v, zbuf_v, sA_v, sB_v, out_sh, si, sg):
    cid = lax.axis_index("c")
    sid = lax.axis_index("s")
    wid = cid * NS + sid

    for hh in range(HEAD):
        v_hbm = v0_hbm if hh == 0 else v1_hbm
        pltpu.sync_copy(s_hbm.at[0, hh], sA_v)
        pltpu.sync_copy(s_hbm.at[1, hh], sB_v)

        def _sadd(i, _):
            sA_v[pl.ds(i * 16, 16)] = sA_v[pl.ds(i * 16, 16)] + sB_v[pl.ds(i * 16, 16)]
            return 0
        lax.fori_loop(0, NSP_S // 16, _sadd, 0)

        def _zrow2(i, _):
            def _zc(c, _):
                zbuf_v[i, pl.ds(c * 16, 16)] = jnp.zeros((16,), jnp.float32)
                return 0
            lax.fori_loop(0, DH // 16, _zc, 0)
            return 0
        lax.fori_loop(0, RPP, _zrow2, 0)
        for p in range(RPW // RPP):
            pltpu.sync_copy(zbuf_v,
                            out_sh.at[pl.ds(sid * RPW + p * RPP, RPP), :])
        plsc.subcore_barrier()

        def _issue_idx(c, b):
            @pl.when(c < NCHB)
            def _():
                base = c * KB
                pltpu.async_copy(src_hbm.at[pl.ds(base, KB)], src_v.at[b], si[b])
                pltpu.async_copy(dst_hbm.at[pl.ds(base, KB)], dst_v.at[b], si[b])
                pltpu.async_copy(e_hbm.at[hh, pl.ds(base, KB)], e_v.at[b], si[b])

        def _issue_gather(c, b):
            @pl.when(c < NCHB)
            def _():
                base = c * KB
                pltpu.make_async_copy(src_hbm.at[pl.ds(base, KB)], src_v.at[b], si[b]).wait()
                pltpu.make_async_copy(dst_hbm.at[pl.ds(base, KB)], dst_v.at[b], si[b]).wait()
                pltpu.make_async_copy(e_hbm.at[hh, pl.ds(base, KB)], e_v.at[b], si[b]).wait()
                pltpu.async_copy(v_hbm.at[dst_v.at[b]], vrow_v.at[b], sg[b])

        def _compute(c, b):
            @pl.when(c < NCHB)
            def _():
                pltpu.make_async_copy(v_hbm.at[dst_v.at[b]], vrow_v.at[b], sg[b]).wait()

                def _wgrp(g, _):
                    sl = pl.ds(g * 16, 16)
                    sv = plsc.load_gather(sA_v, [src_v[b, sl]])
                    w_v[sl] = e_v[b, sl] / sv
                    return 0
                lax.fori_loop(0, KB // 16, _wgrp, 0)

                def _scale(ei, _):
                    wb = plsc.load_gather(w_v, [jnp.full((16,), ei, jnp.int32)])
                    for c2 in range(DH // 16):
                        vrow_v[b, ei, pl.ds(c2 * 16, 16)] = vrow_v[b, ei, pl.ds(c2 * 16, 16)] * wb
                    return 0
                lax.fori_loop(0, KB, _scale, 0, unroll=4)

                pltpu.sync_copy(vrow_v.at[b], out_sh.at[src_v.at[b]], add=True)

        _issue_idx(wid, 0)
        _issue_idx(wid + NW, 1)
        _issue_gather(wid, 0)

        def _pair(k2, _):
            for b in range(2):
                k = 2 * k2 + b
                c = wid + NW * k
                _issue_gather(c + NW, 1 - b)
                _compute(c, b)
                _issue_idx(c + 2 * NW, b)
            return 0

        lax.fori_loop(0, (KPB + 1) // 2, _pair, 0)
        plsc.subcore_barrier()

        for p in range(RPW // RPP):
            r0 = sid * RPW + p * RPP
            pltpu.sync_copy(out_sh.at[pl.ds(r0, RPP), :], zbuf_v)
            pltpu.sync_copy(zbuf_v,
                            out_hbm.at[cid, hh, pl.ds(r0, RPP), :])
        if hh == 0:
            plsc.subcore_barrier()


NCH_P = N // KCH
PTAIL = N - NCH_P * KCH
KPB_P = -(-NCH_P // NW)
SPSTR = NSP_P // NS


@functools.partial(
    pl.kernel,
    out_type=[
        jax.ShapeDtypeStruct((NC, NSP_P, HIDE), jnp.float32),
        jax.ShapeDtypeStruct((NC, NSP_P), jnp.float32),
    ],
    mesh=_sc_mesh,
    compiler_params=pltpu.CompilerParams(
        use_tc_tiling_on_sc=False, needs_layout_passes=False),
    scratch_types=[
        pltpu.VMEM((KCH,), jnp.int32),
        pltpu.VMEM((KCH, HIDE), jnp.float32),
        pltpu.VMEM((KCH,), jnp.float32),
        pltpu.VMEM((PTAIL,), jnp.int32),
        pltpu.VMEM((PTAIL, HIDE), jnp.float32),
        pltpu.VMEM((PTAIL,), jnp.float32),
        pltpu.VMEM((SPSTR,), jnp.float32),
        pltpu.VMEM_SHARED((NSP_P, HIDE), jnp.float32),
        pltpu.VMEM_SHARED((NSP_P,), jnp.float32),
    ],
)
def _pool(x_hbm, qa_hbm, hp_hbm, deg_hbm,
          qa_v, rows_v, ones_v, qat_v, rowst_v, onest_v, buf_v,
          hp_sh, deg_sh):
    cid = lax.axis_index("c")
    sid = lax.axis_index("s")
    wid = cid * NS + sid

    def _fill(ref, n, val):
        def _f(i, _):
            ref[pl.ds(i * 16, 16)] = jnp.full((16,), val, jnp.float32)
            return 0
        lax.fori_loop(0, n // 16, _f, 0)

    _fill(ones_v, KCH, 1.0)
    _fill(onest_v, PTAIL, 1.0)
    _fill(buf_v, SPSTR, 0.0)

    def _zrow(i, _):
        def _zc(c, _):
            rows_v[i, pl.ds(c * 16, 16)] = jnp.zeros((16,), jnp.float32)
            return 0
        lax.fori_loop(0, HIDE // 16, _zc, 0)
        return 0
    lax.fori_loop(0, SPSTR, _zrow, 0)
    pltpu.sync_copy(rows_v.at[pl.ds(0, SPSTR), :],
                    hp_sh.at[pl.ds(sid * SPSTR, SPSTR), :])
    pltpu.sync_copy(buf_v, deg_sh.at[pl.ds(sid * SPSTR, SPSTR)])
    plsc.subcore_barrier()

    def _chunk(k, _):
        cidx = wid + NW * k

        @pl.when(cidx < NCH_P)
        def _():
            base = cidx * KCH
            pltpu.sync_copy(qa_hbm.at[pl.ds(base, KCH)], qa_v)
            pltpu.sync_copy(x_hbm.at[pl.ds(base, KCH), :], rows_v)
            pltpu.sync_copy(rows_v, hp_sh.at[qa_v], add=True)
            pltpu.sync_copy(ones_v, deg_sh.at[qa_v], add=True)
        return 0

    lax.fori_loop(0, KPB_P, _chunk, 0)

    @pl.when(wid == 0)
    def _tail():
        base = NCH_P * KCH
        pltpu.sync_copy(qa_hbm.at[pl.ds(base, PTAIL)], qat_v)
        pltpu.sync_copy(x_hbm.at[pl.ds(base, PTAIL), :], rowst_v)
        pltpu.sync_copy(rowst_v, hp_sh.at[qat_v], add=True)
        pltpu.sync_copy(onest_v, deg_sh.at[qat_v], add=True)

    plsc.subcore_barrier()
    r0 = sid * SPSTR
    pltpu.sync_copy(hp_sh.at[pl.ds(r0, SPSTR), :], rows_v.at[pl.ds(0, SPSTR), :])
    pltpu.sync_copy(rows_v.at[pl.ds(0, SPSTR), :], hp_hbm.at[cid, pl.ds(r0, SPSTR), :])
    pltpu.sync_copy(deg_sh.at[pl.ds(r0, SPSTR)], buf_v)
    pltpu.sync_copy(buf_v, deg_hbm.at[cid, pl.ds(r0, SPSTR)])


NCH_A = EA // KCH
ATAIL = EA - NCH_A * KCH
KPB_A = -(-NCH_A // NW)
AFL = NSP_P * NSP_P
ASTR = AFL // NS


@functools.partial(
    pl.kernel,
    out_type=jax.ShapeDtypeStruct((NC, AFL), jnp.float32),
    mesh=_sc_mesh,
    compiler_params=pltpu.CompilerParams(
        use_tc_tiling_on_sc=False, needs_layout_passes=False),
    scratch_types=[
        pltpu.VMEM((KCH,), jnp.int32),
        pltpu.VMEM((KCH,), jnp.int32),
        pltpu.VMEM((KCH,), jnp.float32),
        pltpu.VMEM((KCH,), jnp.int32),
        pltpu.VMEM((ATAIL,), jnp.int32),
        pltpu.VMEM((ATAIL,), jnp.int32),
        pltpu.VMEM((ATAIL,), jnp.float32),
        pltpu.VMEM((ATAIL,), jnp.int32),
        pltpu.VMEM((ASTR,), jnp.float32),
        pltpu.VMEM_SHARED((AFL,), jnp.float32),
    ],
)
def _abuild(asrc_hbm, adst_hbm, aval_hbm, A_hbm,
            s_v, d_v, v_v, fidx_v, st_v, dt_v, vt_v, fidxt_v, zbuf_v, A_sh):
    cid = lax.axis_index("c")
    sid = lax.axis_index("s")
    wid = cid * NS + sid

    def _f(i, _):
        zbuf_v[pl.ds(i * 16, 16)] = jnp.zeros((16,), jnp.float32)
        return 0
    lax.fori_loop(0, ASTR // 16, _f, 0)
    pltpu.sync_copy(zbuf_v, A_sh.at[pl.ds(sid * ASTR, ASTR)])
    plsc.subcore_barrier()

    def _chunk(k, _):
        cidx = wid + NW * k

        @pl.when(cidx < NCH_A)
        def _():
            base = cidx * KCH
            pltpu.sync_copy(asrc_hbm.at[pl.ds(base, KCH)], s_v)
            pltpu.sync_copy(adst_hbm.at[pl.ds(base, KCH)], d_v)
            pltpu.sync_copy(aval_hbm.at[pl.ds(base, KCH)], v_v)

            def _fi(g, _):
                sl = pl.ds(g * 16, 16)
                fidx_v[sl] = d_v[sl] * NSP_P + s_v[sl]
                return 0
            lax.fori_loop(0, KCH // 16, _fi, 0)
            pltpu.sync_copy(v_v, A_sh.at[fidx_v], add=True)
        return 0

    lax.fori_loop(0, KPB_A, _chunk, 0)

    @pl.when(wid == 0)
    def _tail():
        base = NCH_A * KCH
        pltpu.sync_copy(asrc_hbm.at[pl.ds(base, ATAIL)], st_v)
        pltpu.sync_copy(adst_hbm.at[pl.ds(base, ATAIL)], dt_v)
        pltpu.sync_copy(aval_hbm.at[pl.ds(base, ATAIL)], vt_v)

        def _fi(g, _):
            sl = pl.ds(g * 16, 16)
            fidxt_v[sl] = dt_v[sl] * NSP_P + st_v[sl]
            return 0
        lax.fori_loop(0, ATAIL // 16, _fi, 0)
        pltpu.sync_copy(vt_v, A_sh.at[fidxt_v], add=True)

    plsc.subcore_barrier()
    pltpu.sync_copy(A_sh.at[pl.ds(sid * ASTR, ASTR)], zbuf_v)
    pltpu.sync_copy(zbuf_v, A_hbm.at[cid, pl.ds(sid * ASTR, ASTR)])


def kernel(x, mask_index, mask_value, q_assign, a_index, a_value,
           prelin_W, prelin_b, bn0_g, bn0_b,
           sf_W, sf_b, sf_g, sf_be,
           pf_Wv, pf_bv, pf_Wq, pf_bq, pf_g, pf_be,
           out_W, out_b):
    bnscale = 1.0 / jnp.sqrt(1.0 + 1e-5)
    w1 = prelin_W * (bn0_g * bnscale)[:, None]
    b1 = prelin_b * (bn0_g * bnscale) + bn0_b

    src = mask_index[0]
    dst = mask_index[1]

    outPr = None
    for j in range(NPF):
        wcat = jnp.concatenate([pf_Wq[j, 0], pf_Wq[j, 1], pf_Wv[j, 0], pf_Wv[j, 1]], axis=0)
        bcat = jnp.concatenate([pf_bq[j, 0], pf_bq[j, 1], pf_bv[j, 0], pf_bv[j, 1]], axis=0)
        if j == 0:
            h0, qcat, vcat = _prelin_proj(x, w1.T, b1[None, :], wcat.T, bcat[None, :])
            x_pre = h0
        else:
            g0 = (pf_g[0] * bnscale)[None, :]
            b0 = pf_be[0][None, :]
            qcat, vcat = _mid(outPr, g0, b0, wcat.T, bcat[None, :])
        e2, sP = _edge_a(qcat, src, dst)
        outP = _edge_b(vcat[:, :DH], vcat[:, DH:], e2, sP, src, dst)
        outPr = outP.reshape(NC * HEAD, N, DH)

    hpP, degP = _pool(x_pre, q_assign)
    APf = _abuild(a_index[0], a_index[1], a_value)
    AP = APf.reshape(NC, NSP_P, NSP_P)
    hp_fin = _sf_dense(hpP, degP, AP, jnp.transpose(sf_W, (0, 2, 1)),
                       sf_b, sf_g * bnscale, sf_be)

    g1 = (pf_g[1] * bnscale)[None, :]
    b1f = pf_be[1][None, :]
    woutp = jnp.zeros((HIDE, 128), jnp.float32).at[:, :NCLS].set(out_W.T)
    boutp = jnp.full((128,), -1e30, jnp.float32).at[:NCLS].set(out_b)
    ypad = _final(outPr, g1, b1f, q_assign.reshape(N, 1), hp_fin,
                  woutp, boutp[None, :])
    return ypad[:, :NCLS]
